# baseline (speedup 1.0000x reference)
import functools

import jax
import jax.numpy as jnp
from jax import lax
from jax.experimental import pallas as pl
from jax.experimental.pallas import tpu as pltpu
from jax.experimental.pallas import tpu_sc as plsc

_INTERPRET = False

_NROWS = 6016
_STRIPE = _NROWS // 16
_CHUNK = 128


def _wid():
    return lax.axis_index("s") * 2 + lax.axis_index("c")


def _zero_vmem2d(buf, ncols):
    z = jnp.zeros((16,), jnp.float32)

    def body(i, _):
        for k in range(ncols // 16):
            buf[i, pl.ds(k * 16, 16)] = z
        return 0

    lax.fori_loop(0, buf.shape[0], body, 0)


def _zero_acc_stripe(acc, zbuf, s):
    base = s * _STRIPE
    pltpu.sync_copy(zbuf, acc.at[pl.ds(base, _CHUNK)])
    pltpu.sync_copy(zbuf, acc.at[pl.ds(base + _CHUNK, _CHUNK)])
    rem = _STRIPE - 2 * _CHUNK
    pltpu.sync_copy(zbuf.at[pl.ds(0, rem)], acc.at[pl.ds(base + 2 * _CHUNK, rem)])


def _read_scalar(vref):
    return jnp.sum(vref[...]).astype(jnp.int32)


def _make_deg(S):
    nchunks = S // _CHUNK

    def body(d0_hbm, d1_hbm, d2_hbm, out_hbm,
             dv0, dv1, dv2, ones0, ones1, ones2, zbuf, acc):
        c = lax.axis_index("c")
        s = lax.axis_index("s")
        w = _wid()
        lanes = jnp.arange(16, dtype=jnp.int32)

        def fill(i, _):
            ones0[i, :] = jnp.where(lanes == 0, 1.0, 0.0)
            ones1[i, :] = jnp.where(lanes == 1, 1.0, 0.0)
            ones2[i, :] = jnp.where(lanes == 2, 1.0, 0.0)
            zbuf[i, :] = jnp.zeros((16,), jnp.float32)
            return 0

        lax.fori_loop(0, _CHUNK, fill, 0)
        _zero_acc_stripe(acc, zbuf, s)
        plsc.subcore_barrier()

        def chunk(j, _):
            base = (w + 32 * j) * _CHUNK
            pltpu.sync_copy(d0_hbm.at[pl.ds(base, _CHUNK)], dv0)
            pltpu.sync_copy(d1_hbm.at[pl.ds(base, _CHUNK)], dv1)
            pltpu.sync_copy(d2_hbm.at[pl.ds(base, _CHUNK)], dv2)
            pltpu.sync_copy(ones0, acc.at[dv0], add=True)
            pltpu.sync_copy(ones1, acc.at[dv1], add=True)
            pltpu.sync_copy(ones2, acc.at[dv2], add=True)
            return 0

        lax.fori_loop(0, nchunks // 32, chunk, 0)
        plsc.subcore_barrier()
        base = s * _STRIPE
        pltpu.sync_copy(acc.at[pl.ds(base, _STRIPE)],
                        out_hbm.at[c].at[pl.ds(base, _STRIPE)])

    return body


def _make_agg(S, F):
    nchunks = S // _CHUNK

    def body(y0_hbm, y1_hbm, y2_hbm, s_hbm, d0_hbm, d1_hbm, d2_hbm,
             o0_hbm, o1_hbm, o2_hbm,
             sv, dv0, dv1, dv2, r0, r1, r2, sem,
             acc0, acc1, acc2):
        c = lax.axis_index("c")
        s = lax.axis_index("s")
        w = _wid()
        _zero_vmem2d(r0, F)
        _zero_acc_stripe(acc0, r0, s)
        _zero_acc_stripe(acc1, r0, s)
        _zero_acc_stripe(acc2, r0, s)
        plsc.subcore_barrier()

        def chunk(j, _):
            base = (w + 32 * j) * _CHUNK
            pltpu.sync_copy(s_hbm.at[pl.ds(base, _CHUNK)], sv)
            pltpu.sync_copy(d0_hbm.at[pl.ds(base, _CHUNK)], dv0)
            pltpu.sync_copy(d1_hbm.at[pl.ds(base, _CHUNK)], dv1)
            pltpu.sync_copy(d2_hbm.at[pl.ds(base, _CHUNK)], dv2)
            cp0 = pltpu.async_copy(y0_hbm.at[sv], r0, sem)
            cp1 = pltpu.async_copy(y1_hbm.at[sv], r1, sem)
            cp2 = pltpu.async_copy(y2_hbm.at[sv], r2, sem)
            cp0.wait()
            cp1.wait()
            cp2.wait()
            pltpu.sync_copy(r0, acc0.at[dv0], add=True)
            pltpu.sync_copy(r1, acc1.at[dv1], add=True)
            pltpu.sync_copy(r2, acc2.at[dv2], add=True)
            return 0

        lax.fori_loop(0, nchunks // 32, chunk, 0)
        plsc.subcore_barrier()
        base = s * _STRIPE
        pltpu.sync_copy(acc0.at[pl.ds(base, _STRIPE)],
                        o0_hbm.at[c].at[pl.ds(base, _STRIPE)])
        pltpu.sync_copy(acc1.at[pl.ds(base, _STRIPE)],
                        o1_hbm.at[c].at[pl.ds(base, _STRIPE)])
        pltpu.sync_copy(acc2.at[pl.ds(base, _STRIPE)],
                        o2_hbm.at[c].at[pl.ds(base, _STRIPE)])

    return body


def _sc_deg(d0, d1, d2):
    S = d0.shape[0]
    mesh = plsc.VectorSubcoreMesh(core_axis_name="c", subcore_axis_name="s")
    return pl.kernel(
        _make_deg(S),
        out_type=jax.ShapeDtypeStruct((2, _NROWS, 16), jnp.float32),
        mesh=mesh,
        scratch_types=[
            pltpu.VMEM((_CHUNK,), jnp.int32),
            pltpu.VMEM((_CHUNK,), jnp.int32),
            pltpu.VMEM((_CHUNK,), jnp.int32),
            pltpu.VMEM((_CHUNK, 16), jnp.float32),
            pltpu.VMEM((_CHUNK, 16), jnp.float32),
            pltpu.VMEM((_CHUNK, 16), jnp.float32),
            pltpu.VMEM((_CHUNK, 16), jnp.float32),
            pltpu.VMEM_SHARED((_NROWS, 16), jnp.float32),
        ],
        compiler_params=pltpu.CompilerParams(use_tc_tiling_on_sc=False),
        interpret=_INTERPRET,
    )(d0, d1, d2)


def _sc_agg(F, y0, y1, y2, src, d0, d1, d2):
    S = src.shape[0]
    mesh = plsc.VectorSubcoreMesh(core_axis_name="c", subcore_axis_name="s")
    out = jax.ShapeDtypeStruct((2, _NROWS, F), jnp.float32)
    return pl.kernel(
        _make_agg(S, F),
        out_type=(out, out, out),
        mesh=mesh,
        scratch_types=[
            pltpu.VMEM((_CHUNK,), jnp.int32),
            pltpu.VMEM((_CHUNK,), jnp.int32),
            pltpu.VMEM((_CHUNK,), jnp.int32),
            pltpu.VMEM((_CHUNK,), jnp.int32),
            pltpu.VMEM((_CHUNK, F), jnp.float32),
            pltpu.VMEM((_CHUNK, F), jnp.float32),
            pltpu.VMEM((_CHUNK, F), jnp.float32),
            pltpu.SemaphoreType.DMA,
            pltpu.VMEM_SHARED((_NROWS, F), jnp.float32),
            pltpu.VMEM_SHARED((_NROWS, F), jnp.float32),
            pltpu.VMEM_SHARED((_NROWS, F), jnp.float32),
        ],
        compiler_params=pltpu.CompilerParams(use_tc_tiling_on_sc=False),
        interpret=_INTERPRET,
    )(y0, y1, y2, src, d0, d1, d2)


def _lr(x):
    return jnp.where(x >= 0, x, 0.01 * x)


def _k1_body(gene_ref, w_ref, degp_ref, xw_ref, y0_ref, y1_ref, y2_ref,
             dinv_ref):
    xw = jnp.dot(gene_ref[...], w_ref[...], preferred_element_type=jnp.float32)
    degp = degp_ref[...]
    deg = degp[0, :6000, 0:3] + degp[1, :6000, 0:3] + 1.0
    dinv = jax.lax.rsqrt(deg)
    dinv_ref[...] = dinv
    drep = jnp.repeat(dinv, 64, axis=1)
    xw_ref[...] = xw
    y = xw * drep
    y0_ref[...] = y[:, 0:64]
    y1_ref[...] = y[:, 64:128]
    y2_ref[...] = y[:, 128:192]


def _k2_body(s0_ref, s1_ref, s2_ref, xw_ref, dinv_ref, b_ref, w2_ref,
             xw2_ref, y0_ref, y1_ref, y2_ref):
    dinv = dinv_ref[...]
    drep = jnp.repeat(dinv, 64, axis=1)
    scat = jnp.concatenate(
        [s0_ref[0, :6000, :] + s0_ref[1, :6000, :],
         s1_ref[0, :6000, :] + s1_ref[1, :6000, :],
         s2_ref[0, :6000, :] + s2_ref[1, :6000, :]], axis=1)
    h1 = _lr(drep * (scat + drep * xw_ref[...]) + b_ref[...])
    xw2 = jnp.dot(h1, w2_ref[...], preferred_element_type=jnp.float32)
    xw2_ref[...] = xw2
    drep2 = jnp.repeat(dinv, 32, axis=1)
    y2 = xw2 * drep2
    y0_ref[...] = y2[:, 0:32]
    y1_ref[...] = y2[:, 32:64]
    y2_ref[...] = y2[:, 64:96]


def _k3_body(s0_ref, s1_ref, s2_ref, xw_ref, dinv_ref, b_ref, prot_ref,
             pew_ref, peb_ref, mnw_ref, mnb_ref, vrw_ref, vrb_ref,
             decw_ref, decb_ref, om_ref,
             h2_ref, mu_ref, lv_ref, c0_ref, pr_ref):
    dinv = dinv_ref[...]
    drep = jnp.repeat(dinv, 32, axis=1)
    scat = jnp.concatenate(
        [s0_ref[0, :6000, :] + s0_ref[1, :6000, :],
         s1_ref[0, :6000, :] + s1_ref[1, :6000, :],
         s2_ref[0, :6000, :] + s2_ref[1, :6000, :]], axis=1)
    h2 = _lr(drep * (scat + drep * xw_ref[...]) + b_ref[...])
    h2_ref[...] = h2
    enc = _lr(jnp.dot(prot_ref[...], pew_ref[...],
                      preferred_element_type=jnp.float32) + peb_ref[...])
    mu = _lr(jnp.dot(enc, mnw_ref[...],
                     preferred_element_type=jnp.float32) + mnb_ref[...])
    lv = _lr(jnp.dot(enc, vrw_ref[...],
                     preferred_element_type=jnp.float32) + vrb_ref[...])
    mu_ref[...] = mu
    lv_ref[...] = lv
    w0 = om_ref[0, 0]
    w1 = om_ref[0, 1]
    gex = h2[:, 64:96]
    c0 = (w0 * gex + w1 * mu) / (w0 + w1)
    c0_ref[...] = c0
    pr_ref[...] = _lr(jnp.dot(c0, decw_ref[...],
                              preferred_element_type=jnp.float32) + decb_ref[...])


def _k4_body(a_ref, b_ref, o_ref):
    o_ref[...] = jax.lax.dot_general(
        a_ref[...], b_ref[...], (((1,), (1,)), ((), ())),
        preferred_element_type=jnp.float32)


def kernel(gene_matrix, protein_matrix, adjacency_matrix, W1, b1, W2, b2,
           pe_W, pe_b, mn_W, mn_b, vr_W, vr_b, dec_W, dec_b, omega):
    N, G = gene_matrix.shape
    P = protein_matrix.shape[1]
    L = W2.shape[1]
    F1, F2 = 2 * L, L
    S = 32 * N

    src, dst = jnp.nonzero(adjacency_matrix, size=S, fill_value=0)
    src = src.astype(jnp.int32)
    dst = dst.astype(jnp.int32)
    E = jnp.count_nonzero(adjacency_matrix).astype(jnp.int32)
    idx = jnp.arange(S)
    valid = idx < E
    mk = jax.random.key(42)
    if jax.config.jax_threefry_partitionable:
        u1 = jax.random.uniform(jax.random.fold_in(mk, 1), (S,))
        u2 = jax.random.uniform(jax.random.fold_in(mk, 2), (S,))
    else:
        Eu = E.astype(jnp.uint32)
        u1 = _unif_prefix(jax.random.key_data(jax.random.fold_in(mk, 1)), S, Eu)
        u2 = _unif_prefix(jax.random.key_data(jax.random.fold_in(mk, 2)), S, Eu)
    trash = jnp.int32(6000)
    d0 = jnp.where((u1 >= 0.4) & valid, dst, trash)
    d1 = jnp.where((u2 >= 0.5) & valid, dst, trash)
    d2 = jnp.where(valid, dst, trash)
    f1 = (jax.random.uniform(jax.random.fold_in(mk, 3), (G,)) >= 0.3).astype(jnp.float32)
    f2 = (jax.random.uniform(jax.random.fold_in(mk, 4), (G,)) >= 0.2).astype(jnp.float32)

    W1cat = jnp.concatenate([W1 * f1[:, None], W1 * f2[:, None], W1], axis=1)
    z = jnp.zeros((2 * L, L), jnp.float32)
    W2bd = jnp.block([[W2, z, z], [z, W2, z], [z, z, W2]])
    b1t = jnp.tile(b1, 3)[None, :]
    b2t = jnp.tile(b2, 3)[None, :]

    deg_parts = _sc_deg(d0, d1, d2)

    xw1, ya, yb, yc, dinv = pl.pallas_call(
        _k1_body,
        out_shape=(jax.ShapeDtypeStruct((N, 3 * F1), jnp.float32),
                   jax.ShapeDtypeStruct((N, F1), jnp.float32),
                   jax.ShapeDtypeStruct((N, F1), jnp.float32),
                   jax.ShapeDtypeStruct((N, F1), jnp.float32),
                   jax.ShapeDtypeStruct((N, 3), jnp.float32)),
        interpret=_INTERPRET,
    )(gene_matrix, W1cat, deg_parts)

    sa, sb, sc_ = _sc_agg(F1, ya, yb, yc, src, d0, d1, d2)

    xw2, ya2, yb2, yc2 = pl.pallas_call(
        _k2_body,
        out_shape=(jax.ShapeDtypeStruct((N, 3 * F2), jnp.float32),
                   jax.ShapeDtypeStruct((N, F2), jnp.float32),
                   jax.ShapeDtypeStruct((N, F2), jnp.float32),
                   jax.ShapeDtypeStruct((N, F2), jnp.float32)),
        interpret=_INTERPRET,
    )(sa, sb, sc_, xw1, dinv, b1t, W2bd)

    sa2, sb2, sc2 = _sc_agg(F2, ya2, yb2, yc2, src, d0, d1, d2)

    h2, mu, logvar, c0, pex_recons = pl.pallas_call(
        _k3_body,
        out_shape=(jax.ShapeDtypeStruct((N, 3 * F2), jnp.float32),
                   jax.ShapeDtypeStruct((N, L), jnp.float32),
                   jax.ShapeDtypeStruct((N, L), jnp.float32),
                   jax.ShapeDtypeStruct((N, L), jnp.float32),
                   jax.ShapeDtypeStruct((N, P), jnp.float32)),
        interpret=_INTERPRET,
    )(sa2, sb2, sc2, xw2, dinv, b2t, protein_matrix, pe_W, pe_b[None, :],
      mn_W, mn_b[None, :], vr_W, vr_b[None, :], dec_W, dec_b[None, :],
      omega[None, :])

    BM = 600
    adj_recon = pl.pallas_call(
        _k4_body,
        grid=(N // BM,),
        in_specs=[pl.BlockSpec((BM, L), lambda i: (i, 0)),
                  pl.BlockSpec((N, L), lambda i: (0, 0))],
        out_specs=pl.BlockSpec((BM, N), lambda i: (i, 0)),
        out_shape=jax.ShapeDtypeStruct((N, N), jnp.float32),
        interpret=_INTERPRET,
    )(c0, c0)

    z1, z2, gex_z = h2[:, :L], h2[:, L:2 * L], h2[:, 2 * L:]
    return (adj_recon, pex_recons, z1, z2, gex_z, mu, mu, logvar, c0, c0, omega)


def _tf2x32(k0, k1, x0, x1):
    def rotl(x, d):
        return (x << jnp.uint32(d)) | (x >> jnp.uint32(32 - d))
    ks = (k0, k1, k0 ^ k1 ^ jnp.uint32(0x1BD11BDA))
    x0 = x0 + ks[0]
    x1 = x1 + ks[1]
    rotations = ((13, 15, 26, 6), (17, 29, 16, 24))
    for i in range(1, 6):
        for r in rotations[(i - 1) % 2]:
            x0 = x0 + x1
            x1 = rotl(x1, r)
            x1 = x0 ^ x1
        x0 = x0 + ks[i % 3]
        x1 = x1 + ks[(i + 1) % 3] + jnp.uint32(i)
    return x0, x1


def _unif_prefix(kd, S, e):
    k0 = kd[0]
    k1 = kd[1]
    idx = jnp.arange(S, dtype=jnp.uint32)
    half = (e + jnp.uint32(1)) // jnp.uint32(2)
    c1a = jnp.where(half + idx < e, half + idx, jnp.uint32(0))
    a0, _ = _tf2x32(k0, k1, idx, c1a)
    c0b = jnp.where(idx >= half, idx - half, jnp.uint32(0))
    _, b1 = _tf2x32(k0, k1, c0b, idx)
    bits = jnp.where(idx < half, a0, b1)
    f = jax.lax.bitcast_convert_type(
        (bits >> jnp.uint32(9)) | jnp.uint32(0x3F800000), jnp.float32)
    return jnp.maximum(jnp.float32(0.0), f - jnp.float32(1.0))


# speedup vs baseline: 5.3350x; 5.3350x over previous
import jax
import jax.numpy as jnp
from jax import lax
from jax.experimental import pallas as pl
from jax.experimental.pallas import tpu as pltpu
from jax.experimental.pallas import tpu_sc as plsc

_INTERPRET = False

_N = 6000
_NP = 6016
_NROWS = 6016
_STRIPE = _NROWS // 16
_CHUNK = 128
_CAP = 16384
_RPT = 188
_TRASH = 6000


def _wid():
    return lax.axis_index("s") * 2 + lax.axis_index("c")


def _zero_vmem2d(buf, ncols):
    z = jnp.zeros((16,), jnp.float32)

    def body(i, _):
        for k in range(ncols // 16):
            buf[i, pl.ds(k * 16, 16)] = z
        return 0

    lax.fori_loop(0, buf.shape[0], body, 0)


def _zero_acc_stripe(acc, zbuf, s):
    base = s * _STRIPE
    pltpu.sync_copy(zbuf, acc.at[pl.ds(base, _CHUNK)])
    pltpu.sync_copy(zbuf, acc.at[pl.ds(base + _CHUNK, _CHUNK)])
    rem = _STRIPE - 2 * _CHUNK
    pltpu.sync_copy(zbuf.at[pl.ds(0, rem)], acc.at[pl.ds(base + 2 * _CHUNK, rem)])


def _e1_body(a_hbm, src_hbm, col_hbm, cnt_hbm, rowbuf, colb, srcb, cntv):
    w = _wid()
    r0 = w * _RPT
    nr = jnp.where(w == 31, _N - 31 * _RPT, _RPT)
    iota = jnp.arange(16, dtype=jnp.int32)
    nvr = _N // 16

    def block(b, off):
        bb = jnp.minimum(r0 + b * 8, _N - 8)
        pltpu.sync_copy(a_hbm.at[pl.ds(bb, 8)], rowbuf)
        for i in range(8):
            rglob = bb + i

            def do_row(off_i):
                rsplat = jnp.broadcast_to(rglob, (16,)).astype(jnp.int32)

                def vloop(v, o):
                    x = rowbuf[i, pl.ds(v * 16, 16)]
                    m = x != 0.0
                    pc = plsc.all_reduce_population_count(m)[0]

                    def hit(o2):
                        colids = v * 16 + iota
                        plsc.store_compressed(
                            colb.at[pl.ds(o2, 16)], colids, mask=m)
                        plsc.store_compressed(
                            srcb.at[pl.ds(o2, 16)], rsplat, mask=m)
                        return o2 + pc

                    return lax.cond(pc > 0, hit, lambda o2: o2, o)

                return lax.fori_loop(0, nvr, vloop, off_i)

            valid_row = (rglob >= r0 + b * 8) & (rglob < r0 + nr)
            off = lax.cond(valid_row, do_row, lambda o: o, off)
        return off

    nb = (nr + 7) // 8
    off = lax.fori_loop(0, nb, block, jnp.int32(0))
    cntv[...] = jnp.broadcast_to(off, (16,)).astype(jnp.int32)
    pltpu.sync_copy(cntv, cnt_hbm.at[w])
    pltpu.sync_copy(colb.at[pl.ds(0, _CAP)], col_hbm.at[w])
    pltpu.sync_copy(srcb.at[pl.ds(0, _CAP)], src_hbm.at[w])


def _sc_extract(a_u8):
    mesh = plsc.VectorSubcoreMesh(core_axis_name="c", subcore_axis_name="s")
    return pl.kernel(
        _e1_body,
        out_type=(jax.ShapeDtypeStruct((32, _CAP), jnp.int32),
                  jax.ShapeDtypeStruct((32, _CAP), jnp.int32),
                  jax.ShapeDtypeStruct((32, 16), jnp.int32)),
        mesh=mesh,
        scratch_types=[
            pltpu.VMEM((8, _N), jnp.float32),
            pltpu.VMEM((_CAP + 16,), jnp.int32),
            pltpu.VMEM((_CAP + 16,), jnp.int32),
            pltpu.VMEM((16,), jnp.int32),
        ],
        compiler_params=pltpu.CompilerParams(
            use_tc_tiling_on_sc=False, needs_layout_passes=False),
        interpret=_INTERPRET,
    )(a_u8)


def _e2_body(src_hbm, col_hbm, cnt_hbm, t1_hbm, t2_hbm,
             srcO, d0O, d1O, d2O, degO,
             cntall, sv, cv, sv2, d0v, d1v, d2v, tw1, tw2,
             ones0, ones1, ones2, zbuf, acc):
    c = lax.axis_index("c")
    s = lax.axis_index("s")
    w = _wid()
    iota = jnp.arange(16, dtype=jnp.int32)

    pltpu.sync_copy(cnt_hbm, cntall)

    def fill(i, _):
        ones0[i, :] = jnp.where(iota == 0, 1.0, 0.0)
        ones1[i, :] = jnp.where(iota == 1, 1.0, 0.0)
        ones2[i, :] = jnp.where(iota == 2, 1.0, 0.0)
        zbuf[i, :] = jnp.zeros((16,), jnp.float32)
        return 0

    lax.fori_loop(0, _CHUNK, fill, 0)
    _zero_acc_stripe(acc, zbuf, s)
    plsc.subcore_barrier()

    def pf(t, carry):
        ge, ga = carry
        ct = cntall[t, pl.ds(0, 16)][0]
        return (ge + ct, ga + ((ct + 127) // 128) * 128)

    ge, ga = lax.fori_loop(0, w, pf, (jnp.int32(0), jnp.int32(0)))
    cnt = cntall[w, pl.ds(0, 16)][0]
    nch = (cnt + 127) // 128

    def chunk(k, _):
        kb = pl.multiple_of(k * 128, 128)
        pltpu.sync_copy(src_hbm.at[w].at[pl.ds(kb, 128)], sv)
        pltpu.sync_copy(col_hbm.at[w].at[pl.ds(kb, 128)], cv)
        rb = ge + k * 128
        al = pl.multiple_of((rb // 16) * 16, 16)
        sh = rb - al
        pltpu.sync_copy(t1_hbm.at[pl.ds(al, 144)], tw1)
        pltpu.sync_copy(t2_hbm.at[pl.ds(al, 144)], tw2)
        for g in range(8):
            lidx = k * 128 + g * 16 + iota
            vld = lidx < cnt
            colg = cv[pl.ds(g * 16, 16)]
            srcg = sv[pl.ds(g * 16, 16)]
            t1g = tw1[pl.ds(sh + g * 16, 16)]
            t2g = tw2[pl.ds(sh + g * 16, 16)]
            d0v[pl.ds(g * 16, 16)] = jnp.where(
                vld & (t1g > 0), colg, jnp.int32(_TRASH))
            d1v[pl.ds(g * 16, 16)] = jnp.where(
                vld & (t2g > 0), colg, jnp.int32(_TRASH))
            d2v[pl.ds(g * 16, 16)] = jnp.where(vld, colg, jnp.int32(_TRASH))
            sv2[pl.ds(g * 16, 16)] = jnp.where(vld, srcg, jnp.int32(0))
        pltpu.sync_copy(ones0, acc.at[d0v], add=True)
        pltpu.sync_copy(ones1, acc.at[d1v], add=True)
        pltpu.sync_copy(ones2, acc.at[d2v], add=True)
        ob = pl.multiple_of(ga + k * 128, 128)
        pltpu.sync_copy(sv2, srcO.at[pl.ds(ob, 128)])
        pltpu.sync_copy(d0v, d0O.at[pl.ds(ob, 128)])
        pltpu.sync_copy(d1v, d1O.at[pl.ds(ob, 128)])
        pltpu.sync_copy(d2v, d2O.at[pl.ds(ob, 128)])
        return 0

    lax.fori_loop(0, nch, chunk, 0)
    plsc.subcore_barrier()
    base = s * _STRIPE
    pltpu.sync_copy(acc.at[pl.ds(base, _STRIPE)],
                    degO.at[c].at[pl.ds(base, _STRIPE)])


def _sc_rank_deg(srcB, colB, cntB, t1, t2, S):
    mesh = plsc.VectorSubcoreMesh(core_axis_name="c", subcore_axis_name="s")
    return pl.kernel(
        _e2_body,
        out_type=(jax.ShapeDtypeStruct((S,), jnp.int32),
                  jax.ShapeDtypeStruct((S,), jnp.int32),
                  jax.ShapeDtypeStruct((S,), jnp.int32),
                  jax.ShapeDtypeStruct((S,), jnp.int32),
                  jax.ShapeDtypeStruct((2, _NROWS, 16), jnp.float32)),
        mesh=mesh,
        scratch_types=[
            pltpu.VMEM((32, 16), jnp.int32),
            pltpu.VMEM((_CHUNK,), jnp.int32),
            pltpu.VMEM((_CHUNK,), jnp.int32),
            pltpu.VMEM((_CHUNK,), jnp.int32),
            pltpu.VMEM((_CHUNK,), jnp.int32),
            pltpu.VMEM((_CHUNK,), jnp.int32),
            pltpu.VMEM((_CHUNK,), jnp.int32),
            pltpu.VMEM((144,), jnp.float32),
            pltpu.VMEM((144,), jnp.float32),
            pltpu.VMEM((_CHUNK, 16), jnp.float32),
            pltpu.VMEM((_CHUNK, 16), jnp.float32),
            pltpu.VMEM((_CHUNK, 16), jnp.float32),
            pltpu.VMEM((_CHUNK, 16), jnp.float32),
            pltpu.VMEM_SHARED((_NROWS, 16), jnp.float32),
        ],
        compiler_params=pltpu.CompilerParams(
            use_tc_tiling_on_sc=False, needs_layout_passes=False),
        interpret=_INTERPRET,
    )(srcB, colB, cntB, t1, t2)


def _make_agg(F):
    def body(y0_hbm, y1_hbm, y2_hbm, s_hbm, d0_hbm, d1_hbm, d2_hbm, cnt_hbm,
             o0_hbm, o1_hbm, o2_hbm,
             cntall, sv, dv0, dv1, dv2, r0, r1, r2, sem,
             acc0, acc1, acc2):
        c = lax.axis_index("c")
        s = lax.axis_index("s")
        w = _wid()
        pltpu.sync_copy(cnt_hbm, cntall)
        _zero_vmem2d(r0, F)
        _zero_acc_stripe(acc0, r0, s)
        _zero_acc_stripe(acc1, r0, s)
        _zero_acc_stripe(acc2, r0, s)
        plsc.subcore_barrier()

        def pf(t, ga):
            ct = cntall[t, pl.ds(0, 16)][0]
            return ga + ((ct + 127) // 128) * 128

        ga = lax.fori_loop(0, w, pf, jnp.int32(0))
        cnt = cntall[w, pl.ds(0, 16)][0]
        nch = (cnt + 127) // 128

        def chunk(k, _):
            base = pl.multiple_of(ga + k * 128, 128)
            pltpu.sync_copy(s_hbm.at[pl.ds(base, _CHUNK)], sv)
            pltpu.sync_copy(d0_hbm.at[pl.ds(base, _CHUNK)], dv0)
            pltpu.sync_copy(d1_hbm.at[pl.ds(base, _CHUNK)], dv1)
            pltpu.sync_copy(d2_hbm.at[pl.ds(base, _CHUNK)], dv2)
            cp0 = pltpu.async_copy(y0_hbm.at[sv], r0, sem)
            cp1 = pltpu.async_copy(y1_hbm.at[sv], r1, sem)
            cp2 = pltpu.async_copy(y2_hbm.at[sv], r2, sem)
            cp0.wait()
            cp1.wait()
            cp2.wait()
            pltpu.sync_copy(r0, acc0.at[dv0], add=True)
            pltpu.sync_copy(r1, acc1.at[dv1], add=True)
            pltpu.sync_copy(r2, acc2.at[dv2], add=True)
            return 0

        lax.fori_loop(0, nch, chunk, 0)
        plsc.subcore_barrier()
        base = s * _STRIPE
        pltpu.sync_copy(acc0.at[pl.ds(base, _STRIPE)],
                        o0_hbm.at[c].at[pl.ds(base, _STRIPE)])
        pltpu.sync_copy(acc1.at[pl.ds(base, _STRIPE)],
                        o1_hbm.at[c].at[pl.ds(base, _STRIPE)])
        pltpu.sync_copy(acc2.at[pl.ds(base, _STRIPE)],
                        o2_hbm.at[c].at[pl.ds(base, _STRIPE)])

    return body


def _sc_agg(F, y0, y1, y2, src, d0, d1, d2, cntB):
    mesh = plsc.VectorSubcoreMesh(core_axis_name="c", subcore_axis_name="s")
    out = jax.ShapeDtypeStruct((2, _NROWS, F), jnp.float32)
    return pl.kernel(
        _make_agg(F),
        out_type=(out, out, out),
        mesh=mesh,
        scratch_types=[
            pltpu.VMEM((32, 16), jnp.int32),
            pltpu.VMEM((_CHUNK,), jnp.int32),
            pltpu.VMEM((_CHUNK,), jnp.int32),
            pltpu.VMEM((_CHUNK,), jnp.int32),
            pltpu.VMEM((_CHUNK,), jnp.int32),
            pltpu.VMEM((_CHUNK, F), jnp.float32),
            pltpu.VMEM((_CHUNK, F), jnp.float32),
            pltpu.VMEM((_CHUNK, F), jnp.float32),
            pltpu.SemaphoreType.DMA,
            pltpu.VMEM_SHARED((_NROWS, F), jnp.float32),
            pltpu.VMEM_SHARED((_NROWS, F), jnp.float32),
            pltpu.VMEM_SHARED((_NROWS, F), jnp.float32),
        ],
        compiler_params=pltpu.CompilerParams(
            use_tc_tiling_on_sc=False, needs_layout_passes=False),
        interpret=_INTERPRET,
    )(y0, y1, y2, src, d0, d1, d2, cntB)


def _lr(x):
    return jnp.where(x >= 0, x, 0.01 * x)


def _k1_body(gene_ref, w_ref, degp_ref, xw_ref, y0_ref, y1_ref, y2_ref,
             dinv_ref):
    xw = jnp.dot(gene_ref[...], w_ref[...], preferred_element_type=jnp.float32)
    degp = degp_ref[...]
    deg = degp[0, :_N, 0:3] + degp[1, :_N, 0:3] + 1.0
    dinv = jax.lax.rsqrt(deg)
    dinv_ref[...] = dinv
    drep = jnp.repeat(dinv, 64, axis=1)
    xw_ref[...] = xw
    y = xw * drep
    y0_ref[...] = y[:, 0:64]
    y1_ref[...] = y[:, 64:128]
    y2_ref[...] = y[:, 128:192]


def _k2_body(s0_ref, s1_ref, s2_ref, xw_ref, dinv_ref, b_ref, w2_ref,
             xw2_ref, y0_ref, y1_ref, y2_ref):
    dinv = dinv_ref[...]
    drep = jnp.repeat(dinv, 64, axis=1)
    scat = jnp.concatenate(
        [s0_ref[0, :_N, :] + s0_ref[1, :_N, :],
         s1_ref[0, :_N, :] + s1_ref[1, :_N, :],
         s2_ref[0, :_N, :] + s2_ref[1, :_N, :]], axis=1)
    h1 = _lr(drep * (scat + drep * xw_ref[...]) + b_ref[...])
    xw2 = jnp.dot(h1, w2_ref[...], preferred_element_type=jnp.float32)
    xw2_ref[...] = xw2
    drep2 = jnp.repeat(dinv, 32, axis=1)
    y2 = xw2 * drep2
    y0_ref[...] = y2[:, 0:32]
    y1_ref[...] = y2[:, 32:64]
    y2_ref[...] = y2[:, 64:96]


def _k3_body(s0_ref, s1_ref, s2_ref, xw_ref, dinv_ref, b_ref, prot_ref,
             pew_ref, peb_ref, mnw_ref, mnb_ref, vrw_ref, vrb_ref,
             decw_ref, decb_ref, om_ref,
             h2_ref, mu_ref, lv_ref, c0_ref, pr_ref):
    dinv = dinv_ref[...]
    drep = jnp.repeat(dinv, 32, axis=1)
    scat = jnp.concatenate(
        [s0_ref[0, :_N, :] + s0_ref[1, :_N, :],
         s1_ref[0, :_N, :] + s1_ref[1, :_N, :],
         s2_ref[0, :_N, :] + s2_ref[1, :_N, :]], axis=1)
    h2 = _lr(drep * (scat + drep * xw_ref[...]) + b_ref[...])
    h2_ref[...] = h2
    enc = _lr(jnp.dot(prot_ref[...], pew_ref[...],
                      preferred_element_type=jnp.float32) + peb_ref[...])
    mu = _lr(jnp.dot(enc, mnw_ref[...],
                     preferred_element_type=jnp.float32) + mnb_ref[...])
    lv = _lr(jnp.dot(enc, vrw_ref[...],
                     preferred_element_type=jnp.float32) + vrb_ref[...])
    mu_ref[...] = mu
    lv_ref[...] = lv
    w0 = om_ref[0, 0]
    w1 = om_ref[0, 1]
    gex = h2[:, 64:96]
    c0 = (w0 * gex + w1 * mu) / (w0 + w1)
    c0_ref[...] = c0
    pr_ref[...] = _lr(jnp.dot(c0, decw_ref[...],
                              preferred_element_type=jnp.float32) + decb_ref[...])


def _k4_body(a_ref, b_ref, o_ref):
    o_ref[...] = jax.lax.dot_general(
        a_ref[...], b_ref[...], (((1,), (1,)), ((), ())),
        preferred_element_type=jnp.float32)


def kernel(gene_matrix, protein_matrix, adjacency_matrix, W1, b1, W2, b2,
           pe_W, pe_b, mn_W, mn_b, vr_W, vr_b, dec_W, dec_b, omega):
    N, G = gene_matrix.shape
    P = protein_matrix.shape[1]
    L = W2.shape[1]
    F1, F2 = 2 * L, L
    S = 32 * N

    srcB, colB, cntB = _sc_extract(adjacency_matrix)

    mk = jax.random.key(42)
    if jax.config.jax_threefry_partitionable:
        u1 = jax.random.uniform(jax.random.fold_in(mk, 1), (S,))
        u2 = jax.random.uniform(jax.random.fold_in(mk, 2), (S,))
    else:
        E = jnp.sum(cntB[:, 0]).astype(jnp.uint32)
        u1 = _unif_prefix(jax.random.key_data(jax.random.fold_in(mk, 1)), S, E)
        u2 = _unif_prefix(jax.random.key_data(jax.random.fold_in(mk, 2)), S, E)
    t1 = (u1 >= 0.4).astype(jnp.float32)
    t2 = (u2 >= 0.5).astype(jnp.float32)
    f1 = (jax.random.uniform(jax.random.fold_in(mk, 3), (G,)) >= 0.3).astype(jnp.float32)
    f2 = (jax.random.uniform(jax.random.fold_in(mk, 4), (G,)) >= 0.2).astype(jnp.float32)

    src, d0, d1, d2, deg_parts = _sc_rank_deg(srcB, colB, cntB, t1, t2, S)

    W1cat = jnp.concatenate([W1 * f1[:, None], W1 * f2[:, None], W1], axis=1)
    z = jnp.zeros((2 * L, L), jnp.float32)
    W2bd = jnp.block([[W2, z, z], [z, W2, z], [z, z, W2]])
    b1t = jnp.tile(b1, 3)[None, :]
    b2t = jnp.tile(b2, 3)[None, :]

    xw1, ya, yb, yc, dinv = pl.pallas_call(
        _k1_body,
        out_shape=(jax.ShapeDtypeStruct((N, 3 * F1), jnp.float32),
                   jax.ShapeDtypeStruct((N, F1), jnp.float32),
                   jax.ShapeDtypeStruct((N, F1), jnp.float32),
                   jax.ShapeDtypeStruct((N, F1), jnp.float32),
                   jax.ShapeDtypeStruct((N, 3), jnp.float32)),
        interpret=_INTERPRET,
    )(gene_matrix, W1cat, deg_parts)

    sa, sb, sc_ = _sc_agg(F1, ya, yb, yc, src, d0, d1, d2, cntB)

    xw2, ya2, yb2, yc2 = pl.pallas_call(
        _k2_body,
        out_shape=(jax.ShapeDtypeStruct((N, 3 * F2), jnp.float32),
                   jax.ShapeDtypeStruct((N, F2), jnp.float32),
                   jax.ShapeDtypeStruct((N, F2), jnp.float32),
                   jax.ShapeDtypeStruct((N, F2), jnp.float32)),
        interpret=_INTERPRET,
    )(sa, sb, sc_, xw1, dinv, b1t, W2bd)

    sa2, sb2, sc2 = _sc_agg(F2, ya2, yb2, yc2, src, d0, d1, d2, cntB)

    h2, mu, logvar, c0, pex_recons = pl.pallas_call(
        _k3_body,
        out_shape=(jax.ShapeDtypeStruct((N, 3 * F2), jnp.float32),
                   jax.ShapeDtypeStruct((N, L), jnp.float32),
                   jax.ShapeDtypeStruct((N, L), jnp.float32),
                   jax.ShapeDtypeStruct((N, L), jnp.float32),
                   jax.ShapeDtypeStruct((N, P), jnp.float32)),
        interpret=_INTERPRET,
    )(sa2, sb2, sc2, xw2, dinv, b2t, protein_matrix, pe_W, pe_b[None, :],
      mn_W, mn_b[None, :], vr_W, vr_b[None, :], dec_W, dec_b[None, :],
      omega[None, :])

    BM = 600
    adj_recon = pl.pallas_call(
        _k4_body,
        grid=(N // BM,),
        in_specs=[pl.BlockSpec((BM, L), lambda i: (i, 0)),
                  pl.BlockSpec((N, L), lambda i: (0, 0))],
        out_specs=pl.BlockSpec((BM, N), lambda i: (i, 0)),
        out_shape=jax.ShapeDtypeStruct((N, N), jnp.float32),
        interpret=_INTERPRET,
    )(c0, c0)

    z1, z2, gex_z = h2[:, :L], h2[:, L:2 * L], h2[:, 2 * L:]
    return (adj_recon, pex_recons, z1, z2, gex_z, mu, mu, logvar, c0, c0, omega)


def _tf2x32(k0, k1, x0, x1):
    def rotl(x, d):
        return (x << jnp.uint32(d)) | (x >> jnp.uint32(32 - d))
    ks = (k0, k1, k0 ^ k1 ^ jnp.uint32(0x1BD11BDA))
    x0 = x0 + ks[0]
    x1 = x1 + ks[1]
    rotations = ((13, 15, 26, 6), (17, 29, 16, 24))
    for i in range(1, 6):
        for r in rotations[(i - 1) % 2]:
            x0 = x0 + x1
            x1 = rotl(x1, r)
            x1 = x0 ^ x1
        x0 = x0 + ks[i % 3]
        x1 = x1 + ks[(i + 1) % 3] + jnp.uint32(i)
    return x0, x1


def _unif_prefix(kd, S, e):
    k0 = kd[0]
    k1 = kd[1]
    idx = jnp.arange(S, dtype=jnp.uint32)
    half = (e + jnp.uint32(1)) // jnp.uint32(2)
    c1a = jnp.where(half + idx < e, half + idx, jnp.uint32(0))
    a0, _ = _tf2x32(k0, k1, idx, c1a)
    c0b = jnp.where(idx >= half, idx - half, jnp.uint32(0))
    _, b1 = _tf2x32(k0, k1, c0b, idx)
    bits = jnp.where(idx < half, a0, b1)
    f = jax.lax.bitcast_convert_type(
        (bits >> jnp.uint32(9)) | jnp.uint32(0x3F800000), jnp.float32)
    return jnp.maximum(jnp.float32(0.0), f - jnp.float32(1.0))


# speedup vs baseline: 7.4879x; 1.4035x over previous
import jax
import jax.numpy as jnp
from jax import lax
from jax.experimental import pallas as pl
from jax.experimental.pallas import tpu as pltpu
from jax.experimental.pallas import tpu_sc as plsc

_INTERPRET = False

_N = 6000
_NP = 6016
_NROWS = 6016
_STRIPE = _NROWS // 16
_CHUNK = 128
_CAP = 16384
_RPT = 188
_TRASH = 6000


def _wid():
    return lax.axis_index("s") * 2 + lax.axis_index("c")


def _zero_vmem2d(buf, ncols):
    z = jnp.zeros((16,), jnp.float32)

    def body(i, _):
        for k in range(ncols // 16):
            buf[i, pl.ds(k * 16, 16)] = z
        return 0

    lax.fori_loop(0, buf.shape[0], body, 0)


def _zero_acc_stripe(acc, zbuf, s):
    base = s * _STRIPE
    pltpu.sync_copy(zbuf, acc.at[pl.ds(base, _CHUNK)])
    pltpu.sync_copy(zbuf, acc.at[pl.ds(base + _CHUNK, _CHUNK)])
    rem = _STRIPE - 2 * _CHUNK
    pltpu.sync_copy(zbuf.at[pl.ds(0, rem)], acc.at[pl.ds(base + 2 * _CHUNK, rem)])


_RBLK = 6
_NBLK = 32


def _e1_process_block(rowbuf, colb, srcb, r0, nr, bb, blk, off):
    iota = jnp.arange(16, dtype=jnp.int32)
    nq = _N // 64

    def do_row(i, off_i):
        rglob = bb + i
        rsplat = jnp.broadcast_to(rglob, (16,)).astype(jnp.int32)

        def emit(o2, colids, m, pc):
            plsc.store_compressed(colb.at[pl.ds(o2, 16)], colids, mask=m)
            plsc.store_compressed(srcb.at[pl.ds(o2, 16)], rsplat, mask=m)
            return o2 + pc

        def quad(q, o):
            xs = [rowbuf[i, pl.ds(q * 64 + 16 * k, 16)] for k in range(4)]
            ssum = (xs[0] + xs[1]) + (xs[2] + xs[3])
            anyc = plsc.all_reduce_population_count(ssum != 0.0)[0]

            def slow(o2):
                for k in range(4):
                    mk_ = xs[k] != 0.0
                    pck = plsc.all_reduce_population_count(mk_)[0]
                    o2 = lax.cond(
                        pck > 0,
                        lambda o3, m=mk_, k=k, pc=pck: emit(
                            o3, (q * 64 + k * 16 + iota).astype(jnp.int32),
                            m, pc),
                        lambda o3: o3, o2)
                return o2

            return lax.cond(anyc > 0, slow, lambda o2: o2, o)

        off_i = lax.fori_loop(0, nq, quad, off_i)
        for t in range(3):
            v = nq * 4 + t
            x = rowbuf[i, pl.ds(v * 16, 16)]
            m = x != 0.0
            pc = plsc.all_reduce_population_count(m)[0]
            off_i = lax.cond(
                pc > 0,
                lambda o3, m=m, v=v, pc=pc: emit(
                    o3, (v * 16 + iota).astype(jnp.int32), m, pc),
                lambda o3: o3, off_i)
        return off_i

    def row_iter(i, off_i):
        rglob = bb + i
        valid = (rglob >= r0 + blk * _RBLK) & (rglob < r0 + nr)
        return lax.cond(valid, lambda o: do_row(i, o), lambda o: o, off_i)

    for i in range(_RBLK):
        off = row_iter(i, off)
    return off


def _e1_body(a_hbm, src_hbm, col_hbm, cnt_hbm,
             rb0, rb1, colb, srcb, cntv, sem0, sem1):
    w = _wid()
    r0 = w * _RPT
    nr = jnp.where(w == 31, _N - 31 * _RPT, _RPT)

    def bbase(b):
        return jnp.minimum(r0 + b * _RBLK, _N - _RBLK)

    pltpu.async_copy(a_hbm.at[pl.ds(bbase(0), _RBLK)], rb0, sem0)

    def pair(p, off):
        b0 = 2 * p
        b1 = 2 * p + 1
        pltpu.async_copy(a_hbm.at[pl.ds(bbase(b1), _RBLK)], rb1, sem1)
        pltpu.make_async_copy(a_hbm.at[pl.ds(0, _RBLK)], rb0, sem0).wait()
        off = _e1_process_block(rb0, colb, srcb, r0, nr, bbase(b0), b0, off)

        def prefetch(_):
            pltpu.async_copy(a_hbm.at[pl.ds(bbase(b0 + 2), _RBLK)], rb0, sem0)
            return 0

        lax.cond(p < _NBLK // 2 - 1, prefetch, lambda _: 0, 0)
        pltpu.make_async_copy(a_hbm.at[pl.ds(0, _RBLK)], rb1, sem1).wait()
        off = _e1_process_block(rb1, colb, srcb, r0, nr, bbase(b1), b1, off)
        return off

    off = lax.fori_loop(0, _NBLK // 2, pair, jnp.int32(0))
    cntv[...] = jnp.broadcast_to(off, (16,)).astype(jnp.int32)
    pltpu.sync_copy(cntv, cnt_hbm.at[w])
    pltpu.sync_copy(colb.at[pl.ds(0, _CAP)], col_hbm.at[w])
    pltpu.sync_copy(srcb.at[pl.ds(0, _CAP)], src_hbm.at[w])


def _sc_extract(a_u8):
    mesh = plsc.VectorSubcoreMesh(core_axis_name="c", subcore_axis_name="s")
    return pl.kernel(
        _e1_body,
        out_type=(jax.ShapeDtypeStruct((32, _CAP), jnp.int32),
                  jax.ShapeDtypeStruct((32, _CAP), jnp.int32),
                  jax.ShapeDtypeStruct((32, 16), jnp.int32)),
        mesh=mesh,
        scratch_types=[
            pltpu.VMEM((_RBLK, _N), jnp.float32),
            pltpu.VMEM((_RBLK, _N), jnp.float32),
            pltpu.VMEM((_CAP + 16,), jnp.int32),
            pltpu.VMEM((_CAP + 16,), jnp.int32),
            pltpu.VMEM((16,), jnp.int32),
            pltpu.SemaphoreType.DMA,
            pltpu.SemaphoreType.DMA,
        ],
        compiler_params=pltpu.CompilerParams(
            use_tc_tiling_on_sc=False, needs_layout_passes=False),
        interpret=_INTERPRET,
    )(a_u8)


def _e2_body(src_hbm, col_hbm, cnt_hbm, t1_hbm, t2_hbm,
             srcO, d0O, d1O, d2O, degO,
             cntall, sv, cv, sv2, d0v, d1v, d2v, tw1, tw2,
             ones0, ones1, ones2, zbuf, acc):
    c = lax.axis_index("c")
    s = lax.axis_index("s")
    w = _wid()
    iota = jnp.arange(16, dtype=jnp.int32)

    pltpu.sync_copy(cnt_hbm, cntall)

    def fill(i, _):
        ones0[i, :] = jnp.where(iota == 0, 1.0, 0.0)
        ones1[i, :] = jnp.where(iota == 1, 1.0, 0.0)
        ones2[i, :] = jnp.where(iota == 2, 1.0, 0.0)
        zbuf[i, :] = jnp.zeros((16,), jnp.float32)
        return 0

    lax.fori_loop(0, _CHUNK, fill, 0)
    _zero_acc_stripe(acc, zbuf, s)
    plsc.subcore_barrier()

    def pf(t, carry):
        ge, ga = carry
        ct = cntall[t, pl.ds(0, 16)][0]
        return (ge + ct, ga + ((ct + 127) // 128) * 128)

    ge, ga = lax.fori_loop(0, w, pf, (jnp.int32(0), jnp.int32(0)))
    cnt = cntall[w, pl.ds(0, 16)][0]
    nch = (cnt + 127) // 128

    def chunk(k, _):
        kb = pl.multiple_of(k * 128, 128)
        pltpu.sync_copy(src_hbm.at[w].at[pl.ds(kb, 128)], sv)
        pltpu.sync_copy(col_hbm.at[w].at[pl.ds(kb, 128)], cv)
        rb = ge + k * 128
        al = pl.multiple_of((rb // 16) * 16, 16)
        sh = rb - al
        pltpu.sync_copy(t1_hbm.at[pl.ds(al, 144)], tw1)
        pltpu.sync_copy(t2_hbm.at[pl.ds(al, 144)], tw2)
        for g in range(8):
            lidx = k * 128 + g * 16 + iota
            vld = lidx < cnt
            colg = cv[pl.ds(g * 16, 16)]
            srcg = sv[pl.ds(g * 16, 16)]
            t1g = tw1[pl.ds(sh + g * 16, 16)]
            t2g = tw2[pl.ds(sh + g * 16, 16)]
            d0v[pl.ds(g * 16, 16)] = jnp.where(
                vld & (t1g > 0), colg, jnp.int32(_TRASH))
            d1v[pl.ds(g * 16, 16)] = jnp.where(
                vld & (t2g > 0), colg, jnp.int32(_TRASH))
            d2v[pl.ds(g * 16, 16)] = jnp.where(vld, colg, jnp.int32(_TRASH))
            sv2[pl.ds(g * 16, 16)] = jnp.where(vld, srcg, jnp.int32(0))
        pltpu.sync_copy(ones0, acc.at[d0v], add=True)
        pltpu.sync_copy(ones1, acc.at[d1v], add=True)
        pltpu.sync_copy(ones2, acc.at[d2v], add=True)
        ob = pl.multiple_of(ga + k * 128, 128)
        pltpu.sync_copy(sv2, srcO.at[pl.ds(ob, 128)])
        pltpu.sync_copy(d0v, d0O.at[pl.ds(ob, 128)])
        pltpu.sync_copy(d1v, d1O.at[pl.ds(ob, 128)])
        pltpu.sync_copy(d2v, d2O.at[pl.ds(ob, 128)])
        return 0

    lax.fori_loop(0, nch, chunk, 0)
    plsc.subcore_barrier()
    base = s * _STRIPE
    pltpu.sync_copy(acc.at[pl.ds(base, _STRIPE)],
                    degO.at[c].at[pl.ds(base, _STRIPE)])


def _sc_rank_deg(srcB, colB, cntB, t1, t2, S):
    mesh = plsc.VectorSubcoreMesh(core_axis_name="c", subcore_axis_name="s")
    return pl.kernel(
        _e2_body,
        out_type=(jax.ShapeDtypeStruct((S,), jnp.int32),
                  jax.ShapeDtypeStruct((S,), jnp.int32),
                  jax.ShapeDtypeStruct((S,), jnp.int32),
                  jax.ShapeDtypeStruct((S,), jnp.int32),
                  jax.ShapeDtypeStruct((2, _NROWS, 16), jnp.float32)),
        mesh=mesh,
        scratch_types=[
            pltpu.VMEM((32, 16), jnp.int32),
            pltpu.VMEM((_CHUNK,), jnp.int32),
            pltpu.VMEM((_CHUNK,), jnp.int32),
            pltpu.VMEM((_CHUNK,), jnp.int32),
            pltpu.VMEM((_CHUNK,), jnp.int32),
            pltpu.VMEM((_CHUNK,), jnp.int32),
            pltpu.VMEM((_CHUNK,), jnp.int32),
            pltpu.VMEM((144,), jnp.float32),
            pltpu.VMEM((144,), jnp.float32),
            pltpu.VMEM((_CHUNK, 16), jnp.float32),
            pltpu.VMEM((_CHUNK, 16), jnp.float32),
            pltpu.VMEM((_CHUNK, 16), jnp.float32),
            pltpu.VMEM((_CHUNK, 16), jnp.float32),
            pltpu.VMEM_SHARED((_NROWS, 16), jnp.float32),
        ],
        compiler_params=pltpu.CompilerParams(
            use_tc_tiling_on_sc=False, needs_layout_passes=False),
        interpret=_INTERPRET,
    )(srcB, colB, cntB, t1, t2)


def _make_agg(F):
    def body(y0_hbm, y1_hbm, y2_hbm, s_hbm, d0_hbm, d1_hbm, d2_hbm, cnt_hbm,
             o0_hbm, o1_hbm, o2_hbm,
             cntall, sv, dv0, dv1, dv2, r0, r1, r2, sem,
             acc0, acc1, acc2):
        c = lax.axis_index("c")
        s = lax.axis_index("s")
        w = _wid()
        pltpu.sync_copy(cnt_hbm, cntall)
        _zero_vmem2d(r0, F)
        _zero_acc_stripe(acc0, r0, s)
        _zero_acc_stripe(acc1, r0, s)
        _zero_acc_stripe(acc2, r0, s)
        plsc.subcore_barrier()

        def pf(t, ga):
            ct = cntall[t, pl.ds(0, 16)][0]
            return ga + ((ct + 127) // 128) * 128

        ga = lax.fori_loop(0, w, pf, jnp.int32(0))
        cnt = cntall[w, pl.ds(0, 16)][0]
        nch = (cnt + 127) // 128

        def chunk(k, _):
            base = pl.multiple_of(ga + k * 128, 128)
            pltpu.sync_copy(s_hbm.at[pl.ds(base, _CHUNK)], sv)
            pltpu.sync_copy(d0_hbm.at[pl.ds(base, _CHUNK)], dv0)
            pltpu.sync_copy(d1_hbm.at[pl.ds(base, _CHUNK)], dv1)
            pltpu.sync_copy(d2_hbm.at[pl.ds(base, _CHUNK)], dv2)
            cp0 = pltpu.async_copy(y0_hbm.at[sv], r0, sem)
            cp1 = pltpu.async_copy(y1_hbm.at[sv], r1, sem)
            cp2 = pltpu.async_copy(y2_hbm.at[sv], r2, sem)
            cp0.wait()
            cp1.wait()
            cp2.wait()
            pltpu.sync_copy(r0, acc0.at[dv0], add=True)
            pltpu.sync_copy(r1, acc1.at[dv1], add=True)
            pltpu.sync_copy(r2, acc2.at[dv2], add=True)
            return 0

        lax.fori_loop(0, nch, chunk, 0)
        plsc.subcore_barrier()
        base = s * _STRIPE
        pltpu.sync_copy(acc0.at[pl.ds(base, _STRIPE)],
                        o0_hbm.at[c].at[pl.ds(base, _STRIPE)])
        pltpu.sync_copy(acc1.at[pl.ds(base, _STRIPE)],
                        o1_hbm.at[c].at[pl.ds(base, _STRIPE)])
        pltpu.sync_copy(acc2.at[pl.ds(base, _STRIPE)],
                        o2_hbm.at[c].at[pl.ds(base, _STRIPE)])

    return body


def _sc_agg(F, y0, y1, y2, src, d0, d1, d2, cntB):
    mesh = plsc.VectorSubcoreMesh(core_axis_name="c", subcore_axis_name="s")
    out = jax.ShapeDtypeStruct((2, _NROWS, F), jnp.float32)
    return pl.kernel(
        _make_agg(F),
        out_type=(out, out, out),
        mesh=mesh,
        scratch_types=[
            pltpu.VMEM((32, 16), jnp.int32),
            pltpu.VMEM((_CHUNK,), jnp.int32),
            pltpu.VMEM((_CHUNK,), jnp.int32),
            pltpu.VMEM((_CHUNK,), jnp.int32),
            pltpu.VMEM((_CHUNK,), jnp.int32),
            pltpu.VMEM((_CHUNK, F), jnp.float32),
            pltpu.VMEM((_CHUNK, F), jnp.float32),
            pltpu.VMEM((_CHUNK, F), jnp.float32),
            pltpu.SemaphoreType.DMA,
            pltpu.VMEM_SHARED((_NROWS, F), jnp.float32),
            pltpu.VMEM_SHARED((_NROWS, F), jnp.float32),
            pltpu.VMEM_SHARED((_NROWS, F), jnp.float32),
        ],
        compiler_params=pltpu.CompilerParams(
            use_tc_tiling_on_sc=False, needs_layout_passes=False),
        interpret=_INTERPRET,
    )(y0, y1, y2, src, d0, d1, d2, cntB)


def _lr(x):
    return jnp.where(x >= 0, x, 0.01 * x)


def _k1_body(gene_ref, w_ref, degp_ref, xw_ref, y0_ref, y1_ref, y2_ref,
             dinv_ref):
    xw = jnp.dot(gene_ref[...], w_ref[...], preferred_element_type=jnp.float32)
    degp = degp_ref[...]
    deg = degp[0, :_N, 0:3] + degp[1, :_N, 0:3] + 1.0
    dinv = jax.lax.rsqrt(deg)
    dinv_ref[...] = dinv
    drep = jnp.repeat(dinv, 64, axis=1)
    xw_ref[...] = xw
    y = xw * drep
    y0_ref[...] = y[:, 0:64]
    y1_ref[...] = y[:, 64:128]
    y2_ref[...] = y[:, 128:192]


def _k2_body(s0_ref, s1_ref, s2_ref, xw_ref, dinv_ref, b_ref, w2_ref,
             xw2_ref, y0_ref, y1_ref, y2_ref):
    dinv = dinv_ref[...]
    drep = jnp.repeat(dinv, 64, axis=1)
    scat = jnp.concatenate(
        [s0_ref[0, :_N, :] + s0_ref[1, :_N, :],
         s1_ref[0, :_N, :] + s1_ref[1, :_N, :],
         s2_ref[0, :_N, :] + s2_ref[1, :_N, :]], axis=1)
    h1 = _lr(drep * (scat + drep * xw_ref[...]) + b_ref[...])
    xw2 = jnp.dot(h1, w2_ref[...], preferred_element_type=jnp.float32)
    xw2_ref[...] = xw2
    drep2 = jnp.repeat(dinv, 32, axis=1)
    y2 = xw2 * drep2
    y0_ref[...] = y2[:, 0:32]
    y1_ref[...] = y2[:, 32:64]
    y2_ref[...] = y2[:, 64:96]


def _k3_body(s0_ref, s1_ref, s2_ref, xw_ref, dinv_ref, b_ref, prot_ref,
             pew_ref, peb_ref, mnw_ref, mnb_ref, vrw_ref, vrb_ref,
             decw_ref, decb_ref, om_ref,
             h2_ref, mu_ref, lv_ref, c0_ref, pr_ref):
    dinv = dinv_ref[...]
    drep = jnp.repeat(dinv, 32, axis=1)
    scat = jnp.concatenate(
        [s0_ref[0, :_N, :] + s0_ref[1, :_N, :],
         s1_ref[0, :_N, :] + s1_ref[1, :_N, :],
         s2_ref[0, :_N, :] + s2_ref[1, :_N, :]], axis=1)
    h2 = _lr(drep * (scat + drep * xw_ref[...]) + b_ref[...])
    h2_ref[...] = h2
    enc = _lr(jnp.dot(prot_ref[...], pew_ref[...],
                      preferred_element_type=jnp.float32) + peb_ref[...])
    mu = _lr(jnp.dot(enc, mnw_ref[...],
                     preferred_element_type=jnp.float32) + mnb_ref[...])
    lv = _lr(jnp.dot(enc, vrw_ref[...],
                     preferred_element_type=jnp.float32) + vrb_ref[...])
    mu_ref[...] = mu
    lv_ref[...] = lv
    w0 = om_ref[0, 0]
    w1 = om_ref[0, 1]
    gex = h2[:, 64:96]
    c0 = (w0 * gex + w1 * mu) / (w0 + w1)
    c0_ref[...] = c0
    pr_ref[...] = _lr(jnp.dot(c0, decw_ref[...],
                              preferred_element_type=jnp.float32) + decb_ref[...])


def _k4_body(a_ref, b_ref, o_ref):
    o_ref[...] = jax.lax.dot_general(
        a_ref[...], b_ref[...], (((1,), (1,)), ((), ())),
        preferred_element_type=jnp.float32)


def kernel(gene_matrix, protein_matrix, adjacency_matrix, W1, b1, W2, b2,
           pe_W, pe_b, mn_W, mn_b, vr_W, vr_b, dec_W, dec_b, omega):
    N, G = gene_matrix.shape
    P = protein_matrix.shape[1]
    L = W2.shape[1]
    F1, F2 = 2 * L, L
    S = 32 * N

    srcB, colB, cntB = _sc_extract(adjacency_matrix)

    mk = jax.random.key(42)
    if jax.config.jax_threefry_partitionable:
        u1 = jax.random.uniform(jax.random.fold_in(mk, 1), (S,))
        u2 = jax.random.uniform(jax.random.fold_in(mk, 2), (S,))
    else:
        E = jnp.sum(cntB[:, 0]).astype(jnp.uint32)
        u1 = _unif_prefix(jax.random.key_data(jax.random.fold_in(mk, 1)), S, E)
        u2 = _unif_prefix(jax.random.key_data(jax.random.fold_in(mk, 2)), S, E)
    t1 = (u1 >= 0.4).astype(jnp.float32)
    t2 = (u2 >= 0.5).astype(jnp.float32)
    f1 = (jax.random.uniform(jax.random.fold_in(mk, 3), (G,)) >= 0.3).astype(jnp.float32)
    f2 = (jax.random.uniform(jax.random.fold_in(mk, 4), (G,)) >= 0.2).astype(jnp.float32)

    src, d0, d1, d2, deg_parts = _sc_rank_deg(srcB, colB, cntB, t1, t2, S)

    W1cat = jnp.concatenate([W1 * f1[:, None], W1 * f2[:, None], W1], axis=1)
    z = jnp.zeros((2 * L, L), jnp.float32)
    W2bd = jnp.block([[W2, z, z], [z, W2, z], [z, z, W2]])
    b1t = jnp.tile(b1, 3)[None, :]
    b2t = jnp.tile(b2, 3)[None, :]

    xw1, ya, yb, yc, dinv = pl.pallas_call(
        _k1_body,
        out_shape=(jax.ShapeDtypeStruct((N, 3 * F1), jnp.float32),
                   jax.ShapeDtypeStruct((N, F1), jnp.float32),
                   jax.ShapeDtypeStruct((N, F1), jnp.float32),
                   jax.ShapeDtypeStruct((N, F1), jnp.float32),
                   jax.ShapeDtypeStruct((N, 3), jnp.float32)),
        interpret=_INTERPRET,
    )(gene_matrix, W1cat, deg_parts)

    sa, sb, sc_ = _sc_agg(F1, ya, yb, yc, src, d0, d1, d2, cntB)

    xw2, ya2, yb2, yc2 = pl.pallas_call(
        _k2_body,
        out_shape=(jax.ShapeDtypeStruct((N, 3 * F2), jnp.float32),
                   jax.ShapeDtypeStruct((N, F2), jnp.float32),
                   jax.ShapeDtypeStruct((N, F2), jnp.float32),
                   jax.ShapeDtypeStruct((N, F2), jnp.float32)),
        interpret=_INTERPRET,
    )(sa, sb, sc_, xw1, dinv, b1t, W2bd)

    sa2, sb2, sc2 = _sc_agg(F2, ya2, yb2, yc2, src, d0, d1, d2, cntB)

    h2, mu, logvar, c0, pex_recons = pl.pallas_call(
        _k3_body,
        out_shape=(jax.ShapeDtypeStruct((N, 3 * F2), jnp.float32),
                   jax.ShapeDtypeStruct((N, L), jnp.float32),
                   jax.ShapeDtypeStruct((N, L), jnp.float32),
                   jax.ShapeDtypeStruct((N, L), jnp.float32),
                   jax.ShapeDtypeStruct((N, P), jnp.float32)),
        interpret=_INTERPRET,
    )(sa2, sb2, sc2, xw2, dinv, b2t, protein_matrix, pe_W, pe_b[None, :],
      mn_W, mn_b[None, :], vr_W, vr_b[None, :], dec_W, dec_b[None, :],
      omega[None, :])

    BM = 600
    adj_recon = pl.pallas_call(
        _k4_body,
        grid=(N // BM,),
        in_specs=[pl.BlockSpec((BM, L), lambda i: (i, 0)),
                  pl.BlockSpec((N, L), lambda i: (0, 0))],
        out_specs=pl.BlockSpec((BM, N), lambda i: (i, 0)),
        out_shape=jax.ShapeDtypeStruct((N, N), jnp.float32),
        interpret=_INTERPRET,
    )(c0, c0)

    z1, z2, gex_z = h2[:, :L], h2[:, L:2 * L], h2[:, 2 * L:]
    return (adj_recon, pex_recons, z1, z2, gex_z, mu, mu, logvar, c0, c0, omega)


def _tf2x32(k0, k1, x0, x1):
    def rotl(x, d):
        return (x << jnp.uint32(d)) | (x >> jnp.uint32(32 - d))
    ks = (k0, k1, k0 ^ k1 ^ jnp.uint32(0x1BD11BDA))
    x0 = x0 + ks[0]
    x1 = x1 + ks[1]
    rotations = ((13, 15, 26, 6), (17, 29, 16, 24))
    for i in range(1, 6):
        for r in rotations[(i - 1) % 2]:
            x0 = x0 + x1
            x1 = rotl(x1, r)
            x1 = x0 ^ x1
        x0 = x0 + ks[i % 3]
        x1 = x1 + ks[(i + 1) % 3] + jnp.uint32(i)
    return x0, x1


def _unif_prefix(kd, S, e):
    k0 = kd[0]
    k1 = kd[1]
    idx = jnp.arange(S, dtype=jnp.uint32)
    half = (e + jnp.uint32(1)) // jnp.uint32(2)
    c1a = jnp.where(half + idx < e, half + idx, jnp.uint32(0))
    a0, _ = _tf2x32(k0, k1, idx, c1a)
    c0b = jnp.where(idx >= half, idx - half, jnp.uint32(0))
    _, b1 = _tf2x32(k0, k1, c0b, idx)
    bits = jnp.where(idx < half, a0, b1)
    f = jax.lax.bitcast_convert_type(
        (bits >> jnp.uint32(9)) | jnp.uint32(0x3F800000), jnp.float32)
    return jnp.maximum(jnp.float32(0.0), f - jnp.float32(1.0))


# speedup vs baseline: 10.9951x; 1.4684x over previous
import jax
import jax.numpy as jnp
from jax import lax
from jax.experimental import pallas as pl
from jax.experimental.pallas import tpu as pltpu
from jax.experimental.pallas import tpu_sc as plsc

_INTERPRET = False

_N = 6000
_NP = 6016
_NROWS = 6016
_STRIPE = _NROWS // 16
_CHUNK = 128
_CAP = 16384
_RPT = 188
_TRASH = 6000


def _wid():
    return lax.axis_index("s") * 2 + lax.axis_index("c")


def _zero_vmem2d(buf, ncols):
    z = jnp.zeros((16,), jnp.float32)

    def body(i, _):
        for k in range(ncols // 16):
            buf[i, pl.ds(k * 16, 16)] = z
        return 0

    lax.fori_loop(0, buf.shape[0], body, 0)


def _zero_acc_stripe(acc, zbuf, s):
    base = s * _STRIPE
    pltpu.sync_copy(zbuf, acc.at[pl.ds(base, _CHUNK)])
    pltpu.sync_copy(zbuf, acc.at[pl.ds(base + _CHUNK, _CHUNK)])
    rem = _STRIPE - 2 * _CHUNK
    pltpu.sync_copy(zbuf.at[pl.ds(0, rem)], acc.at[pl.ds(base + 2 * _CHUNK, rem)])


_RBLK = 6
_NBLK = 32
_NQ = 93


def _e1_process_block(rowbuf, sumbuf, hitq, colb, srcb, r0, nr, bb, blk, off):
    iota = jnp.arange(16, dtype=jnp.int32)

    def do_row(i, off_i):
        rglob = bb + i
        rsplat = jnp.broadcast_to(rglob, (16,)).astype(jnp.int32)

        def emit(o2, colids, m, pc):
            plsc.store_compressed(colb.at[pl.ds(o2, 16)], colids, mask=m)
            plsc.store_compressed(srcb.at[pl.ds(o2, 16)], rsplat, mask=m)
            return o2 + pc

        hoff = jnp.int32(0)
        for sv in range(6):
            f = sumbuf[i, pl.ds(16 * sv, 16)]
            m = f != 0.0
            pc = plsc.all_reduce_population_count(m)[0]
            plsc.store_compressed(
                hitq.at[pl.ds(hoff, 16)], (16 * sv + iota), mask=m)
            hoff = hoff + pc

        def hit_quad(j, o):
            qid = hitq[pl.ds(j, 16)][0]
            base = qid * 64
            for k in range(4):
                x = rowbuf[i, pl.ds(base + 16 * k, 16)]
                mk_ = x != 0.0
                pck = plsc.all_reduce_population_count(mk_)[0]
                o = emit(o, base + k * 16 + iota, mk_, pck)
            return o

        off_i = lax.fori_loop(0, hoff, hit_quad, off_i)
        for t in range(3):
            v = _NQ * 4 + t
            x = rowbuf[i, pl.ds(v * 16, 16)]
            m = x != 0.0
            pc = plsc.all_reduce_population_count(m)[0]
            off_i = lax.cond(
                pc > 0,
                lambda o3, m=m, v=v, pc=pc: emit(
                    o3, (v * 16 + iota).astype(jnp.int32), m, pc),
                lambda o3: o3, off_i)
        return off_i

    def row_iter(i, off_i):
        rglob = bb + i
        valid = (rglob >= r0 + blk * _RBLK) & (rglob < r0 + nr)
        return lax.cond(valid, lambda o: do_row(i, o), lambda o: o, off_i)

    for i in range(_RBLK):
        off = row_iter(i, off)
    return off


def _e1_body(a_hbm, sum_hbm, src_hbm, col_hbm, cnt_hbm,
             rb0, rb1, sb0, sb1, hitq, colb, srcb, cntv, sem0, sem1):
    w = _wid()
    r0 = w * _RPT
    nr = jnp.where(w == 31, _N - 31 * _RPT, _RPT)

    def bbase(b):
        return jnp.minimum(r0 + b * _RBLK, _N - _RBLK)

    def issue(b, rb, sb, sem):
        pltpu.async_copy(a_hbm.at[pl.ds(bbase(b), _RBLK)], rb, sem)
        pltpu.async_copy(sum_hbm.at[pl.ds(bbase(b), _RBLK)], sb, sem)

    def drain(rb, sb, sem):
        pltpu.make_async_copy(a_hbm.at[pl.ds(0, _RBLK)], rb, sem).wait()
        pltpu.make_async_copy(sum_hbm.at[pl.ds(0, _RBLK)], sb, sem).wait()

    issue(0, rb0, sb0, sem0)

    def pair(p, off):
        b0 = 2 * p
        b1 = 2 * p + 1
        issue(b1, rb1, sb1, sem1)
        drain(rb0, sb0, sem0)
        off = _e1_process_block(rb0, sb0, hitq, colb, srcb, r0, nr,
                                bbase(b0), b0, off)

        def prefetch(_):
            issue(b0 + 2, rb0, sb0, sem0)
            return 0

        lax.cond(p < _NBLK // 2 - 1, prefetch, lambda _: 0, 0)
        drain(rb1, sb1, sem1)
        off = _e1_process_block(rb1, sb1, hitq, colb, srcb, r0, nr,
                                bbase(b1), b1, off)
        return off

    off = lax.fori_loop(0, _NBLK // 2, pair, jnp.int32(0))
    cntv[...] = jnp.broadcast_to(off, (16,)).astype(jnp.int32)
    pltpu.sync_copy(cntv, cnt_hbm.at[w])
    pltpu.sync_copy(colb.at[pl.ds(0, _CAP)], col_hbm.at[w])
    pltpu.sync_copy(srcb.at[pl.ds(0, _CAP)], src_hbm.at[w])


def _sc_extract(a_f32, summary):
    mesh = plsc.VectorSubcoreMesh(core_axis_name="c", subcore_axis_name="s")
    return pl.kernel(
        _e1_body,
        out_type=(jax.ShapeDtypeStruct((32, _CAP), jnp.int32),
                  jax.ShapeDtypeStruct((32, _CAP), jnp.int32),
                  jax.ShapeDtypeStruct((32, 16), jnp.int32)),
        mesh=mesh,
        scratch_types=[
            pltpu.VMEM((_RBLK, _N), jnp.float32),
            pltpu.VMEM((_RBLK, _N), jnp.float32),
            pltpu.VMEM((_RBLK, 96), jnp.float32),
            pltpu.VMEM((_RBLK, 96), jnp.float32),
            pltpu.VMEM((112,), jnp.int32),
            pltpu.VMEM((_CAP + 16,), jnp.int32),
            pltpu.VMEM((_CAP + 16,), jnp.int32),
            pltpu.VMEM((16,), jnp.int32),
            pltpu.SemaphoreType.DMA,
            pltpu.SemaphoreType.DMA,
        ],
        compiler_params=pltpu.CompilerParams(
            use_tc_tiling_on_sc=False, needs_layout_passes=False),
        interpret=_INTERPRET,
    )(a_f32, summary)


def _e2_body(src_hbm, col_hbm, cnt_hbm, t1_hbm, t2_hbm,
             srcO, d0O, d1O, d2O, degO,
             cntall, sv, cv, sv2, d0v, d1v, d2v, tw1, tw2,
             ones0, ones1, ones2, zbuf, acc):
    c = lax.axis_index("c")
    s = lax.axis_index("s")
    w = _wid()
    iota = jnp.arange(16, dtype=jnp.int32)

    pltpu.sync_copy(cnt_hbm, cntall)

    def fill(i, _):
        ones0[i, :] = jnp.where(iota == 0, 1.0, 0.0)
        ones1[i, :] = jnp.where(iota == 1, 1.0, 0.0)
        ones2[i, :] = jnp.where(iota == 2, 1.0, 0.0)
        zbuf[i, :] = jnp.zeros((16,), jnp.float32)
        return 0

    lax.fori_loop(0, _CHUNK, fill, 0)
    _zero_acc_stripe(acc, zbuf, s)
    plsc.subcore_barrier()

    def pf(t, carry):
        ge, ga = carry
        ct = cntall[t, pl.ds(0, 16)][0]
        return (ge + ct, ga + ((ct + 127) // 128) * 128)

    ge, ga = lax.fori_loop(0, w, pf, (jnp.int32(0), jnp.int32(0)))
    cnt = cntall[w, pl.ds(0, 16)][0]
    nch = (cnt + 127) // 128

    def chunk(k, _):
        kb = pl.multiple_of(k * 128, 128)
        pltpu.sync_copy(src_hbm.at[w].at[pl.ds(kb, 128)], sv)
        pltpu.sync_copy(col_hbm.at[w].at[pl.ds(kb, 128)], cv)
        rb = ge + k * 128
        al = pl.multiple_of((rb // 16) * 16, 16)
        sh = rb - al
        pltpu.sync_copy(t1_hbm.at[pl.ds(al, 144)], tw1)
        pltpu.sync_copy(t2_hbm.at[pl.ds(al, 144)], tw2)
        for g in range(8):
            lidx = k * 128 + g * 16 + iota
            vld = lidx < cnt
            colg = cv[pl.ds(g * 16, 16)]
            srcg = sv[pl.ds(g * 16, 16)]
            t1g = tw1[pl.ds(sh + g * 16, 16)]
            t2g = tw2[pl.ds(sh + g * 16, 16)]
            d0v[pl.ds(g * 16, 16)] = jnp.where(
                vld & (t1g > 0), colg, jnp.int32(_TRASH))
            d1v[pl.ds(g * 16, 16)] = jnp.where(
                vld & (t2g > 0), colg, jnp.int32(_TRASH))
            d2v[pl.ds(g * 16, 16)] = jnp.where(vld, colg, jnp.int32(_TRASH))
            sv2[pl.ds(g * 16, 16)] = jnp.where(vld, srcg, jnp.int32(0))
        pltpu.sync_copy(ones0, acc.at[d0v], add=True)
        pltpu.sync_copy(ones1, acc.at[d1v], add=True)
        pltpu.sync_copy(ones2, acc.at[d2v], add=True)
        ob = pl.multiple_of(ga + k * 128, 128)
        pltpu.sync_copy(sv2, srcO.at[pl.ds(ob, 128)])
        pltpu.sync_copy(d0v, d0O.at[pl.ds(ob, 128)])
        pltpu.sync_copy(d1v, d1O.at[pl.ds(ob, 128)])
        pltpu.sync_copy(d2v, d2O.at[pl.ds(ob, 128)])
        return 0

    lax.fori_loop(0, nch, chunk, 0)
    plsc.subcore_barrier()
    base = s * _STRIPE
    pltpu.sync_copy(acc.at[pl.ds(base, _STRIPE)],
                    degO.at[c].at[pl.ds(base, _STRIPE)])


def _sc_rank_deg(srcB, colB, cntB, t1, t2, S):
    mesh = plsc.VectorSubcoreMesh(core_axis_name="c", subcore_axis_name="s")
    return pl.kernel(
        _e2_body,
        out_type=(jax.ShapeDtypeStruct((S,), jnp.int32),
                  jax.ShapeDtypeStruct((S,), jnp.int32),
                  jax.ShapeDtypeStruct((S,), jnp.int32),
                  jax.ShapeDtypeStruct((S,), jnp.int32),
                  jax.ShapeDtypeStruct((2, _NROWS, 16), jnp.float32)),
        mesh=mesh,
        scratch_types=[
            pltpu.VMEM((32, 16), jnp.int32),
            pltpu.VMEM((_CHUNK,), jnp.int32),
            pltpu.VMEM((_CHUNK,), jnp.int32),
            pltpu.VMEM((_CHUNK,), jnp.int32),
            pltpu.VMEM((_CHUNK,), jnp.int32),
            pltpu.VMEM((_CHUNK,), jnp.int32),
            pltpu.VMEM((_CHUNK,), jnp.int32),
            pltpu.VMEM((144,), jnp.float32),
            pltpu.VMEM((144,), jnp.float32),
            pltpu.VMEM((_CHUNK, 16), jnp.float32),
            pltpu.VMEM((_CHUNK, 16), jnp.float32),
            pltpu.VMEM((_CHUNK, 16), jnp.float32),
            pltpu.VMEM((_CHUNK, 16), jnp.float32),
            pltpu.VMEM_SHARED((_NROWS, 16), jnp.float32),
        ],
        compiler_params=pltpu.CompilerParams(
            use_tc_tiling_on_sc=False, needs_layout_passes=False),
        interpret=_INTERPRET,
    )(srcB, colB, cntB, t1, t2)


def _make_agg(F):
    def body(y0_hbm, y1_hbm, y2_hbm, s_hbm, d0_hbm, d1_hbm, d2_hbm, cnt_hbm,
             o0_hbm, o1_hbm, o2_hbm,
             cntall, sv, dv0, dv1, dv2, r0, r1, r2, sem,
             acc0, acc1, acc2):
        c = lax.axis_index("c")
        s = lax.axis_index("s")
        w = _wid()
        pltpu.sync_copy(cnt_hbm, cntall)
        _zero_vmem2d(r0, F)
        _zero_acc_stripe(acc0, r0, s)
        _zero_acc_stripe(acc1, r0, s)
        _zero_acc_stripe(acc2, r0, s)
        plsc.subcore_barrier()

        def pf(t, ga):
            ct = cntall[t, pl.ds(0, 16)][0]
            return ga + ((ct + 127) // 128) * 128

        ga = lax.fori_loop(0, w, pf, jnp.int32(0))
        cnt = cntall[w, pl.ds(0, 16)][0]
        nch = (cnt + 127) // 128

        def chunk(k, _):
            base = pl.multiple_of(ga + k * 128, 128)
            pltpu.sync_copy(s_hbm.at[pl.ds(base, _CHUNK)], sv)
            pltpu.sync_copy(d0_hbm.at[pl.ds(base, _CHUNK)], dv0)
            pltpu.sync_copy(d1_hbm.at[pl.ds(base, _CHUNK)], dv1)
            pltpu.sync_copy(d2_hbm.at[pl.ds(base, _CHUNK)], dv2)
            cp0 = pltpu.async_copy(y0_hbm.at[sv], r0, sem)
            cp1 = pltpu.async_copy(y1_hbm.at[sv], r1, sem)
            cp2 = pltpu.async_copy(y2_hbm.at[sv], r2, sem)
            cp0.wait()
            cp1.wait()
            cp2.wait()
            pltpu.sync_copy(r0, acc0.at[dv0], add=True)
            pltpu.sync_copy(r1, acc1.at[dv1], add=True)
            pltpu.sync_copy(r2, acc2.at[dv2], add=True)
            return 0

        lax.fori_loop(0, nch, chunk, 0)
        plsc.subcore_barrier()
        base = s * _STRIPE
        pltpu.sync_copy(acc0.at[pl.ds(base, _STRIPE)],
                        o0_hbm.at[c].at[pl.ds(base, _STRIPE)])
        pltpu.sync_copy(acc1.at[pl.ds(base, _STRIPE)],
                        o1_hbm.at[c].at[pl.ds(base, _STRIPE)])
        pltpu.sync_copy(acc2.at[pl.ds(base, _STRIPE)],
                        o2_hbm.at[c].at[pl.ds(base, _STRIPE)])

    return body


def _sc_agg(F, y0, y1, y2, src, d0, d1, d2, cntB):
    mesh = plsc.VectorSubcoreMesh(core_axis_name="c", subcore_axis_name="s")
    out = jax.ShapeDtypeStruct((2, _NROWS, F), jnp.float32)
    return pl.kernel(
        _make_agg(F),
        out_type=(out, out, out),
        mesh=mesh,
        scratch_types=[
            pltpu.VMEM((32, 16), jnp.int32),
            pltpu.VMEM((_CHUNK,), jnp.int32),
            pltpu.VMEM((_CHUNK,), jnp.int32),
            pltpu.VMEM((_CHUNK,), jnp.int32),
            pltpu.VMEM((_CHUNK,), jnp.int32),
            pltpu.VMEM((_CHUNK, F), jnp.float32),
            pltpu.VMEM((_CHUNK, F), jnp.float32),
            pltpu.VMEM((_CHUNK, F), jnp.float32),
            pltpu.SemaphoreType.DMA,
            pltpu.VMEM_SHARED((_NROWS, F), jnp.float32),
            pltpu.VMEM_SHARED((_NROWS, F), jnp.float32),
            pltpu.VMEM_SHARED((_NROWS, F), jnp.float32),
        ],
        compiler_params=pltpu.CompilerParams(
            use_tc_tiling_on_sc=False, needs_layout_passes=False),
        interpret=_INTERPRET,
    )(y0, y1, y2, src, d0, d1, d2, cntB)


def _ksum_body(a_ref, m_ref, o_ref):
    o_ref[...] = jnp.dot(a_ref[...], m_ref[...],
                         preferred_element_type=jnp.float32)


def _lr(x):
    return jnp.where(x >= 0, x, 0.01 * x)


def _k1_body(gene_ref, w_ref, degp_ref, xw_ref, y0_ref, y1_ref, y2_ref,
             dinv_ref):
    xw = jnp.dot(gene_ref[...], w_ref[...], preferred_element_type=jnp.float32)
    degp = degp_ref[...]
    deg = degp[0, :_N, 0:3] + degp[1, :_N, 0:3] + 1.0
    dinv = jax.lax.rsqrt(deg)
    dinv_ref[...] = dinv
    drep = jnp.repeat(dinv, 64, axis=1)
    xw_ref[...] = xw
    y = xw * drep
    y0_ref[...] = y[:, 0:64]
    y1_ref[...] = y[:, 64:128]
    y2_ref[...] = y[:, 128:192]


def _k2_body(s0_ref, s1_ref, s2_ref, xw_ref, dinv_ref, b_ref, w2_ref,
             xw2_ref, y0_ref, y1_ref, y2_ref):
    dinv = dinv_ref[...]
    drep = jnp.repeat(dinv, 64, axis=1)
    scat = jnp.concatenate(
        [s0_ref[0, :_N, :] + s0_ref[1, :_N, :],
         s1_ref[0, :_N, :] + s1_ref[1, :_N, :],
         s2_ref[0, :_N, :] + s2_ref[1, :_N, :]], axis=1)
    h1 = _lr(drep * (scat + drep * xw_ref[...]) + b_ref[...])
    xw2 = jnp.dot(h1, w2_ref[...], preferred_element_type=jnp.float32)
    xw2_ref[...] = xw2
    drep2 = jnp.repeat(dinv, 32, axis=1)
    y2 = xw2 * drep2
    y0_ref[...] = y2[:, 0:32]
    y1_ref[...] = y2[:, 32:64]
    y2_ref[...] = y2[:, 64:96]


def _k3_body(s0_ref, s1_ref, s2_ref, xw_ref, dinv_ref, b_ref, prot_ref,
             pew_ref, peb_ref, mnw_ref, mnb_ref, vrw_ref, vrb_ref,
             decw_ref, decb_ref, om_ref,
             h2_ref, mu_ref, lv_ref, c0_ref, pr_ref):
    dinv = dinv_ref[...]
    drep = jnp.repeat(dinv, 32, axis=1)
    scat = jnp.concatenate(
        [s0_ref[0, :_N, :] + s0_ref[1, :_N, :],
         s1_ref[0, :_N, :] + s1_ref[1, :_N, :],
         s2_ref[0, :_N, :] + s2_ref[1, :_N, :]], axis=1)
    h2 = _lr(drep * (scat + drep * xw_ref[...]) + b_ref[...])
    h2_ref[...] = h2
    enc = _lr(jnp.dot(prot_ref[...], pew_ref[...],
                      preferred_element_type=jnp.float32) + peb_ref[...])
    mu = _lr(jnp.dot(enc, mnw_ref[...],
                     preferred_element_type=jnp.float32) + mnb_ref[...])
    lv = _lr(jnp.dot(enc, vrw_ref[...],
                     preferred_element_type=jnp.float32) + vrb_ref[...])
    mu_ref[...] = mu
    lv_ref[...] = lv
    w0 = om_ref[0, 0]
    w1 = om_ref[0, 1]
    gex = h2[:, 64:96]
    c0 = (w0 * gex + w1 * mu) / (w0 + w1)
    c0_ref[...] = c0
    pr_ref[...] = _lr(jnp.dot(c0, decw_ref[...],
                              preferred_element_type=jnp.float32) + decb_ref[...])


def _k4_body(a_ref, b_ref, o_ref):
    o_ref[...] = jax.lax.dot_general(
        a_ref[...], b_ref[...], (((1,), (1,)), ((), ())),
        preferred_element_type=jnp.float32)


def kernel(gene_matrix, protein_matrix, adjacency_matrix, W1, b1, W2, b2,
           pe_W, pe_b, mn_W, mn_b, vr_W, vr_b, dec_W, dec_b, omega):
    N, G = gene_matrix.shape
    P = protein_matrix.shape[1]
    L = W2.shape[1]
    F1, F2 = 2 * L, L
    S = 32 * N

    r_ = jnp.arange(N)
    q_ = r_ // 64
    ones_blk = ((q_[:, None] == jnp.arange(96)[None, :]) &
                (q_[:, None] < _NQ)).astype(jnp.float32)
    summary = pl.pallas_call(
        _ksum_body,
        grid=(6,),
        in_specs=[pl.BlockSpec((N // 6, N), lambda i: (i, 0)),
                  pl.BlockSpec((N, 96), lambda i: (0, 0))],
        out_specs=pl.BlockSpec((N // 6, 96), lambda i: (i, 0)),
        out_shape=jax.ShapeDtypeStruct((N, 96), jnp.float32),
        interpret=_INTERPRET,
    )(adjacency_matrix, ones_blk)

    srcB, colB, cntB = _sc_extract(adjacency_matrix, summary)

    mk = jax.random.key(42)
    if jax.config.jax_threefry_partitionable:
        u1 = jax.random.uniform(jax.random.fold_in(mk, 1), (S,))
        u2 = jax.random.uniform(jax.random.fold_in(mk, 2), (S,))
    else:
        E = jnp.sum(cntB[:, 0]).astype(jnp.uint32)
        u1 = _unif_prefix(jax.random.key_data(jax.random.fold_in(mk, 1)), S, E)
        u2 = _unif_prefix(jax.random.key_data(jax.random.fold_in(mk, 2)), S, E)
    t1 = (u1 >= 0.4).astype(jnp.float32)
    t2 = (u2 >= 0.5).astype(jnp.float32)
    f1 = (jax.random.uniform(jax.random.fold_in(mk, 3), (G,)) >= 0.3).astype(jnp.float32)
    f2 = (jax.random.uniform(jax.random.fold_in(mk, 4), (G,)) >= 0.2).astype(jnp.float32)

    src, d0, d1, d2, deg_parts = _sc_rank_deg(srcB, colB, cntB, t1, t2, S)

    W1cat = jnp.concatenate([W1 * f1[:, None], W1 * f2[:, None], W1], axis=1)
    z = jnp.zeros((2 * L, L), jnp.float32)
    W2bd = jnp.block([[W2, z, z], [z, W2, z], [z, z, W2]])
    b1t = jnp.tile(b1, 3)[None, :]
    b2t = jnp.tile(b2, 3)[None, :]

    xw1, ya, yb, yc, dinv = pl.pallas_call(
        _k1_body,
        out_shape=(jax.ShapeDtypeStruct((N, 3 * F1), jnp.float32),
                   jax.ShapeDtypeStruct((N, F1), jnp.float32),
                   jax.ShapeDtypeStruct((N, F1), jnp.float32),
                   jax.ShapeDtypeStruct((N, F1), jnp.float32),
                   jax.ShapeDtypeStruct((N, 3), jnp.float32)),
        interpret=_INTERPRET,
    )(gene_matrix, W1cat, deg_parts)

    sa, sb, sc_ = _sc_agg(F1, ya, yb, yc, src, d0, d1, d2, cntB)

    xw2, ya2, yb2, yc2 = pl.pallas_call(
        _k2_body,
        out_shape=(jax.ShapeDtypeStruct((N, 3 * F2), jnp.float32),
                   jax.ShapeDtypeStruct((N, F2), jnp.float32),
                   jax.ShapeDtypeStruct((N, F2), jnp.float32),
                   jax.ShapeDtypeStruct((N, F2), jnp.float32)),
        interpret=_INTERPRET,
    )(sa, sb, sc_, xw1, dinv, b1t, W2bd)

    sa2, sb2, sc2 = _sc_agg(F2, ya2, yb2, yc2, src, d0, d1, d2, cntB)

    h2, mu, logvar, c0, pex_recons = pl.pallas_call(
        _k3_body,
        out_shape=(jax.ShapeDtypeStruct((N, 3 * F2), jnp.float32),
                   jax.ShapeDtypeStruct((N, L), jnp.float32),
                   jax.ShapeDtypeStruct((N, L), jnp.float32),
                   jax.ShapeDtypeStruct((N, L), jnp.float32),
                   jax.ShapeDtypeStruct((N, P), jnp.float32)),
        interpret=_INTERPRET,
    )(sa2, sb2, sc2, xw2, dinv, b2t, protein_matrix, pe_W, pe_b[None, :],
      mn_W, mn_b[None, :], vr_W, vr_b[None, :], dec_W, dec_b[None, :],
      omega[None, :])

    BM = 600
    adj_recon = pl.pallas_call(
        _k4_body,
        grid=(N // BM,),
        in_specs=[pl.BlockSpec((BM, L), lambda i: (i, 0)),
                  pl.BlockSpec((N, L), lambda i: (0, 0))],
        out_specs=pl.BlockSpec((BM, N), lambda i: (i, 0)),
        out_shape=jax.ShapeDtypeStruct((N, N), jnp.float32),
        interpret=_INTERPRET,
    )(c0, c0)

    z1, z2, gex_z = h2[:, :L], h2[:, L:2 * L], h2[:, 2 * L:]
    return (adj_recon, pex_recons, z1, z2, gex_z, mu, mu, logvar, c0, c0, omega)


def _tf2x32(k0, k1, x0, x1):
    def rotl(x, d):
        return (x << jnp.uint32(d)) | (x >> jnp.uint32(32 - d))
    ks = (k0, k1, k0 ^ k1 ^ jnp.uint32(0x1BD11BDA))
    x0 = x0 + ks[0]
    x1 = x1 + ks[1]
    rotations = ((13, 15, 26, 6), (17, 29, 16, 24))
    for i in range(1, 6):
        for r in rotations[(i - 1) % 2]:
            x0 = x0 + x1
            x1 = rotl(x1, r)
            x1 = x0 ^ x1
        x0 = x0 + ks[i % 3]
        x1 = x1 + ks[(i + 1) % 3] + jnp.uint32(i)
    return x0, x1


def _unif_prefix(kd, S, e):
    k0 = kd[0]
    k1 = kd[1]
    idx = jnp.arange(S, dtype=jnp.uint32)
    half = (e + jnp.uint32(1)) // jnp.uint32(2)
    c1a = jnp.where(half + idx < e, half + idx, jnp.uint32(0))
    a0, _ = _tf2x32(k0, k1, idx, c1a)
    c0b = jnp.where(idx >= half, idx - half, jnp.uint32(0))
    _, b1 = _tf2x32(k0, k1, c0b, idx)
    bits = jnp.where(idx < half, a0, b1)
    f = jax.lax.bitcast_convert_type(
        (bits >> jnp.uint32(9)) | jnp.uint32(0x3F800000), jnp.float32)
    return jnp.maximum(jnp.float32(0.0), f - jnp.float32(1.0))


# speedup vs baseline: 12.0885x; 1.0994x over previous
import jax
import jax.numpy as jnp
from jax import lax
from jax.experimental import pallas as pl
from jax.experimental.pallas import tpu as pltpu
from jax.experimental.pallas import tpu_sc as plsc

_INTERPRET = False

_N = 6000
_NP = 6016
_NROWS = 6016
_STRIPE = _NROWS // 16
_CHUNK = 128
_CAP = 16384
_RPT = 188
_TRASH = 6000


def _wid():
    return lax.axis_index("s") * 2 + lax.axis_index("c")


def _zero_vmem2d(buf, ncols):
    z = jnp.zeros((16,), jnp.float32)

    def body(i, _):
        for k in range(ncols // 16):
            buf[i, pl.ds(k * 16, 16)] = z
        return 0

    lax.fori_loop(0, buf.shape[0], body, 0)


def _zero_acc_stripe(acc, zbuf, s):
    base = s * _STRIPE
    pltpu.sync_copy(zbuf, acc.at[pl.ds(base, _CHUNK)])
    pltpu.sync_copy(zbuf, acc.at[pl.ds(base + _CHUNK, _CHUNK)])
    rem = _STRIPE - 2 * _CHUNK
    pltpu.sync_copy(zbuf.at[pl.ds(0, rem)], acc.at[pl.ds(base + 2 * _CHUNK, rem)])


_RBLK = 6
_NBLK = 32
_NQ = 93


def _e1_process_block(rowbuf, sumbuf, hitq, colb, srcb, r0, nr, bb, blk, off):
    iota = jnp.arange(16, dtype=jnp.int32)

    def do_row(i, off_i):
        rglob = bb + i
        rsplat = jnp.broadcast_to(rglob, (16,)).astype(jnp.int32)

        def emit(o2, colids, m, pc):
            plsc.store_compressed(colb.at[pl.ds(o2, 16)], colids, mask=m)
            plsc.store_compressed(srcb.at[pl.ds(o2, 16)], rsplat, mask=m)
            return o2 + pc

        hoff = jnp.int32(0)
        for sv in range(6):
            f = sumbuf[i, pl.ds(16 * sv, 16)]
            m = f != 0.0
            pc = plsc.all_reduce_population_count(m)[0]
            plsc.store_compressed(
                hitq.at[pl.ds(hoff, 16)], (16 * sv + iota), mask=m)
            hoff = hoff + pc

        def hit_quad(j, o):
            qid = hitq[pl.ds(j, 16)][0]
            base = qid * 64
            for k in range(4):
                x = rowbuf[i, pl.ds(base + 16 * k, 16)]
                mk_ = x != 0.0
                pck = plsc.all_reduce_population_count(mk_)[0]
                o = emit(o, base + k * 16 + iota, mk_, pck)
            return o

        off_i = lax.fori_loop(0, hoff, hit_quad, off_i)
        for t in range(3):
            v = _NQ * 4 + t
            x = rowbuf[i, pl.ds(v * 16, 16)]
            m = x != 0.0
            pc = plsc.all_reduce_population_count(m)[0]
            off_i = lax.cond(
                pc > 0,
                lambda o3, m=m, v=v, pc=pc: emit(
                    o3, (v * 16 + iota).astype(jnp.int32), m, pc),
                lambda o3: o3, off_i)
        return off_i

    def row_iter(i, off_i):
        rglob = bb + i
        valid = (rglob >= r0 + blk * _RBLK) & (rglob < r0 + nr)
        return lax.cond(valid, lambda o: do_row(i, o), lambda o: o, off_i)

    for i in range(_RBLK):
        off = row_iter(i, off)
    return off


def _e1_body(a_hbm, sum_hbm, src_hbm, col_hbm, cnt_hbm,
             rb0, rb1, sb0, sb1, hitq, colb, srcb, cntv, sem0, sem1):
    w = _wid()
    r0 = w * _RPT
    nr = jnp.where(w == 31, _N - 31 * _RPT, _RPT)

    def bbase(b):
        return jnp.minimum(r0 + b * _RBLK, _N - _RBLK)

    def issue(b, rb, sb, sem):
        pltpu.async_copy(a_hbm.at[pl.ds(bbase(b), _RBLK)], rb, sem)
        pltpu.async_copy(sum_hbm.at[pl.ds(bbase(b), _RBLK)], sb, sem)

    def drain(rb, sb, sem):
        pltpu.make_async_copy(a_hbm.at[pl.ds(0, _RBLK)], rb, sem).wait()
        pltpu.make_async_copy(sum_hbm.at[pl.ds(0, _RBLK)], sb, sem).wait()

    issue(0, rb0, sb0, sem0)

    def pair(p, off):
        b0 = 2 * p
        b1 = 2 * p + 1
        issue(b1, rb1, sb1, sem1)
        drain(rb0, sb0, sem0)
        off = _e1_process_block(rb0, sb0, hitq, colb, srcb, r0, nr,
                                bbase(b0), b0, off)

        def prefetch(_):
            issue(b0 + 2, rb0, sb0, sem0)
            return 0

        lax.cond(p < _NBLK // 2 - 1, prefetch, lambda _: 0, 0)
        drain(rb1, sb1, sem1)
        off = _e1_process_block(rb1, sb1, hitq, colb, srcb, r0, nr,
                                bbase(b1), b1, off)
        return off

    off = lax.fori_loop(0, _NBLK // 2, pair, jnp.int32(0))
    cntv[...] = jnp.broadcast_to(off, (16,)).astype(jnp.int32)
    pltpu.sync_copy(cntv, cnt_hbm.at[w])
    pltpu.sync_copy(colb.at[pl.ds(0, _CAP)], col_hbm.at[w])
    pltpu.sync_copy(srcb.at[pl.ds(0, _CAP)], src_hbm.at[w])


def _sc_extract(a_f32, summary):
    mesh = plsc.VectorSubcoreMesh(core_axis_name="c", subcore_axis_name="s")
    return pl.kernel(
        _e1_body,
        out_type=(jax.ShapeDtypeStruct((32, _CAP), jnp.int32),
                  jax.ShapeDtypeStruct((32, _CAP), jnp.int32),
                  jax.ShapeDtypeStruct((32, 16), jnp.int32)),
        mesh=mesh,
        scratch_types=[
            pltpu.VMEM((_RBLK, _N), jnp.float32),
            pltpu.VMEM((_RBLK, _N), jnp.float32),
            pltpu.VMEM((_RBLK, 96), jnp.float32),
            pltpu.VMEM((_RBLK, 96), jnp.float32),
            pltpu.VMEM((112,), jnp.int32),
            pltpu.VMEM((_CAP + 16,), jnp.int32),
            pltpu.VMEM((_CAP + 16,), jnp.int32),
            pltpu.VMEM((16,), jnp.int32),
            pltpu.SemaphoreType.DMA,
            pltpu.SemaphoreType.DMA,
        ],
        compiler_params=pltpu.CompilerParams(
            use_tc_tiling_on_sc=False, needs_layout_passes=False),
        interpret=_INTERPRET,
    )(a_f32, summary)


def _e2_body(src_hbm, col_hbm, cnt_hbm, t1_hbm, t2_hbm,
             srcO, d0O, d1O, d2O, degO,
             cntall, sv, cv, sv2, d0v, d1v, d2v, tw1, tw2,
             ones0, ones1, ones2, zbuf, acc):
    c = lax.axis_index("c")
    s = lax.axis_index("s")
    w = _wid()
    iota = jnp.arange(16, dtype=jnp.int32)

    pltpu.sync_copy(cnt_hbm, cntall)

    def fill(i, _):
        ones0[i, :] = jnp.where(iota == 0, 1.0, 0.0)
        ones1[i, :] = jnp.where(iota == 1, 1.0, 0.0)
        ones2[i, :] = jnp.where(iota == 2, 1.0, 0.0)
        zbuf[i, :] = jnp.zeros((16,), jnp.float32)
        return 0

    lax.fori_loop(0, _CHUNK, fill, 0)
    _zero_acc_stripe(acc, zbuf, s)
    plsc.subcore_barrier()

    def pf(t, carry):
        ge, ga = carry
        ct = cntall[t, pl.ds(0, 16)][0]
        return (ge + ct, ga + ((ct + 127) // 128) * 128)

    ge, ga = lax.fori_loop(0, w, pf, (jnp.int32(0), jnp.int32(0)))
    cnt = cntall[w, pl.ds(0, 16)][0]
    nch = (cnt + 127) // 128

    def chunk(k, _):
        kb = pl.multiple_of(k * 128, 128)
        pltpu.sync_copy(src_hbm.at[w].at[pl.ds(kb, 128)], sv)
        pltpu.sync_copy(col_hbm.at[w].at[pl.ds(kb, 128)], cv)
        rb = ge + k * 128
        al = pl.multiple_of((rb // 16) * 16, 16)
        sh = rb - al
        pltpu.sync_copy(t1_hbm.at[pl.ds(al, 144)], tw1)
        pltpu.sync_copy(t2_hbm.at[pl.ds(al, 144)], tw2)
        for g in range(8):
            lidx = k * 128 + g * 16 + iota
            vld = lidx < cnt
            colg = cv[pl.ds(g * 16, 16)]
            srcg = sv[pl.ds(g * 16, 16)]
            t1g = tw1[pl.ds(sh + g * 16, 16)]
            t2g = tw2[pl.ds(sh + g * 16, 16)]
            d0v[pl.ds(g * 16, 16)] = jnp.where(
                vld & (t1g > 0), colg, jnp.int32(_TRASH))
            d1v[pl.ds(g * 16, 16)] = jnp.where(
                vld & (t2g > 0), colg, jnp.int32(_TRASH))
            d2v[pl.ds(g * 16, 16)] = jnp.where(vld, colg, jnp.int32(_TRASH))
            sv2[pl.ds(g * 16, 16)] = jnp.where(vld, srcg, jnp.int32(0))
        pltpu.sync_copy(ones0, acc.at[d0v], add=True)
        pltpu.sync_copy(ones1, acc.at[d1v], add=True)
        pltpu.sync_copy(ones2, acc.at[d2v], add=True)
        ob = pl.multiple_of(ga + k * 128, 128)
        pltpu.sync_copy(sv2, srcO.at[pl.ds(ob, 128)])
        pltpu.sync_copy(d0v, d0O.at[pl.ds(ob, 128)])
        pltpu.sync_copy(d1v, d1O.at[pl.ds(ob, 128)])
        pltpu.sync_copy(d2v, d2O.at[pl.ds(ob, 128)])
        return 0

    lax.fori_loop(0, nch, chunk, 0)
    plsc.subcore_barrier()
    base = s * _STRIPE
    pltpu.sync_copy(acc.at[pl.ds(base, _STRIPE)],
                    degO.at[c].at[pl.ds(base, _STRIPE)])


def _sc_rank_deg(srcB, colB, cntB, t1, t2, S):
    mesh = plsc.VectorSubcoreMesh(core_axis_name="c", subcore_axis_name="s")
    return pl.kernel(
        _e2_body,
        out_type=(jax.ShapeDtypeStruct((S,), jnp.int32),
                  jax.ShapeDtypeStruct((S,), jnp.int32),
                  jax.ShapeDtypeStruct((S,), jnp.int32),
                  jax.ShapeDtypeStruct((S,), jnp.int32),
                  jax.ShapeDtypeStruct((2, _NROWS, 16), jnp.float32)),
        mesh=mesh,
        scratch_types=[
            pltpu.VMEM((32, 16), jnp.int32),
            pltpu.VMEM((_CHUNK,), jnp.int32),
            pltpu.VMEM((_CHUNK,), jnp.int32),
            pltpu.VMEM((_CHUNK,), jnp.int32),
            pltpu.VMEM((_CHUNK,), jnp.int32),
            pltpu.VMEM((_CHUNK,), jnp.int32),
            pltpu.VMEM((_CHUNK,), jnp.int32),
            pltpu.VMEM((144,), jnp.float32),
            pltpu.VMEM((144,), jnp.float32),
            pltpu.VMEM((_CHUNK, 16), jnp.float32),
            pltpu.VMEM((_CHUNK, 16), jnp.float32),
            pltpu.VMEM((_CHUNK, 16), jnp.float32),
            pltpu.VMEM((_CHUNK, 16), jnp.float32),
            pltpu.VMEM_SHARED((_NROWS, 16), jnp.float32),
        ],
        compiler_params=pltpu.CompilerParams(
            use_tc_tiling_on_sc=False, needs_layout_passes=False),
        interpret=_INTERPRET,
    )(srcB, colB, cntB, t1, t2)


def _make_agg(F):
    def body(y0_hbm, y1_hbm, y2_hbm, s_hbm, d0_hbm, d1_hbm, d2_hbm, cnt_hbm,
             o0_hbm, o1_hbm, o2_hbm,
             cntall, svA, dv0A, dv1A, dv2A, svB, dv0B, dv1B, dv2B,
             r0A, r1A, r2A, r0B, r1B, r2B, semA, semB,
             acc0, acc1, acc2):
        c = lax.axis_index("c")
        s = lax.axis_index("s")
        w = _wid()
        pltpu.sync_copy(cnt_hbm, cntall)
        _zero_vmem2d(r0A, F)
        _zero_acc_stripe(acc0, r0A, s)
        _zero_acc_stripe(acc1, r0A, s)
        _zero_acc_stripe(acc2, r0A, s)
        plsc.subcore_barrier()

        def pf(t, ga):
            ct = cntall[t, pl.ds(0, 16)][0]
            return ga + ((ct + 127) // 128) * 128

        ga = lax.fori_loop(0, w, pf, jnp.int32(0))
        cnt = cntall[w, pl.ds(0, 16)][0]
        nch = (cnt + 127) // 128

        def load_idx(k, sv, dv0, dv1, dv2):
            base = pl.multiple_of(ga + k * 128, 128)
            pltpu.sync_copy(s_hbm.at[pl.ds(base, _CHUNK)], sv)
            pltpu.sync_copy(d0_hbm.at[pl.ds(base, _CHUNK)], dv0)
            pltpu.sync_copy(d1_hbm.at[pl.ds(base, _CHUNK)], dv1)
            pltpu.sync_copy(d2_hbm.at[pl.ds(base, _CHUNK)], dv2)

        def issue_gather(sv, r0, r1, r2, sem):
            pltpu.async_copy(y0_hbm.at[sv], r0, sem)
            pltpu.async_copy(y1_hbm.at[sv], r1, sem)
            pltpu.async_copy(y2_hbm.at[sv], r2, sem)

        def drain_gather(r0, r1, r2, sem):
            pltpu.make_async_copy(y0_hbm.at[pl.ds(0, _CHUNK)], r0, sem).wait()
            pltpu.make_async_copy(y1_hbm.at[pl.ds(0, _CHUNK)], r1, sem).wait()
            pltpu.make_async_copy(y2_hbm.at[pl.ds(0, _CHUNK)], r2, sem).wait()

        def scat(r0, r1, r2, dv0, dv1, dv2):
            pltpu.sync_copy(r0, acc0.at[dv0], add=True)
            pltpu.sync_copy(r1, acc1.at[dv1], add=True)
            pltpu.sync_copy(r2, acc2.at[dv2], add=True)

        def prologue(_):
            load_idx(0, svA, dv0A, dv1A, dv2A)
            issue_gather(svA, r0A, r1A, r2A, semA)
            return 0

        lax.cond(nch > 0, prologue, lambda _: 0, 0)

        def pair(p, _):
            a = 2 * p
            b = 2 * p + 1

            def do_b(_):
                load_idx(b, svB, dv0B, dv1B, dv2B)
                issue_gather(svB, r0B, r1B, r2B, semB)
                return 0

            lax.cond(b < nch, do_b, lambda _: 0, 0)
            drain_gather(r0A, r1A, r2A, semA)
            scat(r0A, r1A, r2A, dv0A, dv1A, dv2A)

            def do_a2(_):
                load_idx(a + 2, svA, dv0A, dv1A, dv2A)
                issue_gather(svA, r0A, r1A, r2A, semA)
                return 0

            lax.cond(a + 2 < nch, do_a2, lambda _: 0, 0)

            def fin_b(_):
                drain_gather(r0B, r1B, r2B, semB)
                scat(r0B, r1B, r2B, dv0B, dv1B, dv2B)
                return 0

            lax.cond(b < nch, fin_b, lambda _: 0, 0)
            return 0

        lax.fori_loop(0, (nch + 1) // 2, pair, 0)
        plsc.subcore_barrier()
        base = s * _STRIPE
        pltpu.sync_copy(acc0.at[pl.ds(base, _STRIPE)],
                        o0_hbm.at[c].at[pl.ds(base, _STRIPE)])
        pltpu.sync_copy(acc1.at[pl.ds(base, _STRIPE)],
                        o1_hbm.at[c].at[pl.ds(base, _STRIPE)])
        pltpu.sync_copy(acc2.at[pl.ds(base, _STRIPE)],
                        o2_hbm.at[c].at[pl.ds(base, _STRIPE)])

    return body


def _sc_agg(F, y0, y1, y2, src, d0, d1, d2, cntB):
    mesh = plsc.VectorSubcoreMesh(core_axis_name="c", subcore_axis_name="s")
    out = jax.ShapeDtypeStruct((2, _NROWS, F), jnp.float32)
    return pl.kernel(
        _make_agg(F),
        out_type=(out, out, out),
        mesh=mesh,
        scratch_types=(
            [pltpu.VMEM((32, 16), jnp.int32)]
            + [pltpu.VMEM((_CHUNK,), jnp.int32)] * 8
            + [pltpu.VMEM((_CHUNK, F), jnp.float32)] * 6
            + [pltpu.SemaphoreType.DMA, pltpu.SemaphoreType.DMA]
            + [pltpu.VMEM_SHARED((_NROWS, F), jnp.float32)] * 3
        ),
        compiler_params=pltpu.CompilerParams(
            use_tc_tiling_on_sc=False, needs_layout_passes=False),
        interpret=_INTERPRET,
    )(y0, y1, y2, src, d0, d1, d2, cntB)


def _ksum_body(a_ref, m_ref, o_ref):
    o_ref[...] = jnp.dot(a_ref[...], m_ref[...],
                         preferred_element_type=jnp.float32)


def _lr(x):
    return jnp.where(x >= 0, x, 0.01 * x)


def _k1_body(gene_ref, w_ref, degp_ref, xw_ref, y0_ref, y1_ref, y2_ref,
             dinv_ref):
    xw = jnp.dot(gene_ref[...], w_ref[...], preferred_element_type=jnp.float32)
    degp = degp_ref[...]
    deg = degp[0, :_N, 0:3] + degp[1, :_N, 0:3] + 1.0
    dinv = jax.lax.rsqrt(deg)
    dinv_ref[...] = dinv
    drep = jnp.repeat(dinv, 64, axis=1)
    xw_ref[...] = xw
    y = xw * drep
    y0_ref[...] = y[:, 0:64]
    y1_ref[...] = y[:, 64:128]
    y2_ref[...] = y[:, 128:192]


def _k2_body(s0_ref, s1_ref, s2_ref, xw_ref, dinv_ref, b_ref, w2_ref,
             xw2_ref, y0_ref, y1_ref, y2_ref):
    dinv = dinv_ref[...]
    drep = jnp.repeat(dinv, 64, axis=1)
    scat = jnp.concatenate(
        [s0_ref[0, :_N, :] + s0_ref[1, :_N, :],
         s1_ref[0, :_N, :] + s1_ref[1, :_N, :],
         s2_ref[0, :_N, :] + s2_ref[1, :_N, :]], axis=1)
    h1 = _lr(drep * (scat + drep * xw_ref[...]) + b_ref[...])
    xw2 = jnp.dot(h1, w2_ref[...], preferred_element_type=jnp.float32)
    xw2_ref[...] = xw2
    drep2 = jnp.repeat(dinv, 32, axis=1)
    y2 = xw2 * drep2
    y0_ref[...] = y2[:, 0:32]
    y1_ref[...] = y2[:, 32:64]
    y2_ref[...] = y2[:, 64:96]


def _k3_body(s0_ref, s1_ref, s2_ref, xw_ref, dinv_ref, b_ref, prot_ref,
             pew_ref, peb_ref, mnw_ref, mnb_ref, vrw_ref, vrb_ref,
             decw_ref, decb_ref, om_ref,
             h2_ref, mu_ref, lv_ref, c0_ref, pr_ref):
    dinv = dinv_ref[...]
    drep = jnp.repeat(dinv, 32, axis=1)
    scat = jnp.concatenate(
        [s0_ref[0, :_N, :] + s0_ref[1, :_N, :],
         s1_ref[0, :_N, :] + s1_ref[1, :_N, :],
         s2_ref[0, :_N, :] + s2_ref[1, :_N, :]], axis=1)
    h2 = _lr(drep * (scat + drep * xw_ref[...]) + b_ref[...])
    h2_ref[...] = h2
    enc = _lr(jnp.dot(prot_ref[...], pew_ref[...],
                      preferred_element_type=jnp.float32) + peb_ref[...])
    mu = _lr(jnp.dot(enc, mnw_ref[...],
                     preferred_element_type=jnp.float32) + mnb_ref[...])
    lv = _lr(jnp.dot(enc, vrw_ref[...],
                     preferred_element_type=jnp.float32) + vrb_ref[...])
    mu_ref[...] = mu
    lv_ref[...] = lv
    w0 = om_ref[0, 0]
    w1 = om_ref[0, 1]
    gex = h2[:, 64:96]
    c0 = (w0 * gex + w1 * mu) / (w0 + w1)
    c0_ref[...] = c0
    pr_ref[...] = _lr(jnp.dot(c0, decw_ref[...],
                              preferred_element_type=jnp.float32) + decb_ref[...])


def _k4_body(a_ref, b_ref, o_ref):
    o_ref[...] = jax.lax.dot_general(
        a_ref[...], b_ref[...], (((1,), (1,)), ((), ())),
        preferred_element_type=jnp.float32)


def kernel(gene_matrix, protein_matrix, adjacency_matrix, W1, b1, W2, b2,
           pe_W, pe_b, mn_W, mn_b, vr_W, vr_b, dec_W, dec_b, omega):
    N, G = gene_matrix.shape
    P = protein_matrix.shape[1]
    L = W2.shape[1]
    F1, F2 = 2 * L, L
    S = 32 * N

    r_ = jnp.arange(N)
    q_ = r_ // 64
    ones_blk = ((q_[:, None] == jnp.arange(96)[None, :]) &
                (q_[:, None] < _NQ)).astype(jnp.float32)
    summary = pl.pallas_call(
        _ksum_body,
        grid=(6,),
        in_specs=[pl.BlockSpec((N // 6, N), lambda i: (i, 0)),
                  pl.BlockSpec((N, 96), lambda i: (0, 0))],
        out_specs=pl.BlockSpec((N // 6, 96), lambda i: (i, 0)),
        out_shape=jax.ShapeDtypeStruct((N, 96), jnp.float32),
        interpret=_INTERPRET,
    )(adjacency_matrix, ones_blk)

    srcB, colB, cntB = _sc_extract(adjacency_matrix, summary)

    mk = jax.random.key(42)
    if jax.config.jax_threefry_partitionable:
        u1 = jax.random.uniform(jax.random.fold_in(mk, 1), (S,))
        u2 = jax.random.uniform(jax.random.fold_in(mk, 2), (S,))
    else:
        E = jnp.sum(cntB[:, 0]).astype(jnp.uint32)
        u1 = _unif_prefix(jax.random.key_data(jax.random.fold_in(mk, 1)), S, E)
        u2 = _unif_prefix(jax.random.key_data(jax.random.fold_in(mk, 2)), S, E)
    t1 = (u1 >= 0.4).astype(jnp.float32)
    t2 = (u2 >= 0.5).astype(jnp.float32)
    f1 = (jax.random.uniform(jax.random.fold_in(mk, 3), (G,)) >= 0.3).astype(jnp.float32)
    f2 = (jax.random.uniform(jax.random.fold_in(mk, 4), (G,)) >= 0.2).astype(jnp.float32)

    src, d0, d1, d2, deg_parts = _sc_rank_deg(srcB, colB, cntB, t1, t2, S)

    W1cat = jnp.concatenate([W1 * f1[:, None], W1 * f2[:, None], W1], axis=1)
    z = jnp.zeros((2 * L, L), jnp.float32)
    W2bd = jnp.block([[W2, z, z], [z, W2, z], [z, z, W2]])
    b1t = jnp.tile(b1, 3)[None, :]
    b2t = jnp.tile(b2, 3)[None, :]

    xw1, ya, yb, yc, dinv = pl.pallas_call(
        _k1_body,
        out_shape=(jax.ShapeDtypeStruct((N, 3 * F1), jnp.float32),
                   jax.ShapeDtypeStruct((N, F1), jnp.float32),
                   jax.ShapeDtypeStruct((N, F1), jnp.float32),
                   jax.ShapeDtypeStruct((N, F1), jnp.float32),
                   jax.ShapeDtypeStruct((N, 3), jnp.float32)),
        interpret=_INTERPRET,
    )(gene_matrix, W1cat, deg_parts)

    sa, sb, sc_ = _sc_agg(F1, ya, yb, yc, src, d0, d1, d2, cntB)

    xw2, ya2, yb2, yc2 = pl.pallas_call(
        _k2_body,
        out_shape=(jax.ShapeDtypeStruct((N, 3 * F2), jnp.float32),
                   jax.ShapeDtypeStruct((N, F2), jnp.float32),
                   jax.ShapeDtypeStruct((N, F2), jnp.float32),
                   jax.ShapeDtypeStruct((N, F2), jnp.float32)),
        interpret=_INTERPRET,
    )(sa, sb, sc_, xw1, dinv, b1t, W2bd)

    sa2, sb2, sc2 = _sc_agg(F2, ya2, yb2, yc2, src, d0, d1, d2, cntB)

    h2, mu, logvar, c0, pex_recons = pl.pallas_call(
        _k3_body,
        out_shape=(jax.ShapeDtypeStruct((N, 3 * F2), jnp.float32),
                   jax.ShapeDtypeStruct((N, L), jnp.float32),
                   jax.ShapeDtypeStruct((N, L), jnp.float32),
                   jax.ShapeDtypeStruct((N, L), jnp.float32),
                   jax.ShapeDtypeStruct((N, P), jnp.float32)),
        interpret=_INTERPRET,
    )(sa2, sb2, sc2, xw2, dinv, b2t, protein_matrix, pe_W, pe_b[None, :],
      mn_W, mn_b[None, :], vr_W, vr_b[None, :], dec_W, dec_b[None, :],
      omega[None, :])

    BM = 600
    adj_recon = pl.pallas_call(
        _k4_body,
        grid=(N // BM,),
        in_specs=[pl.BlockSpec((BM, L), lambda i: (i, 0)),
                  pl.BlockSpec((N, L), lambda i: (0, 0))],
        out_specs=pl.BlockSpec((BM, N), lambda i: (i, 0)),
        out_shape=jax.ShapeDtypeStruct((N, N), jnp.float32),
        interpret=_INTERPRET,
    )(c0, c0)

    z1, z2, gex_z = h2[:, :L], h2[:, L:2 * L], h2[:, 2 * L:]
    return (adj_recon, pex_recons, z1, z2, gex_z, mu, mu, logvar, c0, c0, omega)


def _tf2x32(k0, k1, x0, x1):
    def rotl(x, d):
        return (x << jnp.uint32(d)) | (x >> jnp.uint32(32 - d))
    ks = (k0, k1, k0 ^ k1 ^ jnp.uint32(0x1BD11BDA))
    x0 = x0 + ks[0]
    x1 = x1 + ks[1]
    rotations = ((13, 15, 26, 6), (17, 29, 16, 24))
    for i in range(1, 6):
        for r in rotations[(i - 1) % 2]:
            x0 = x0 + x1
            x1 = rotl(x1, r)
            x1 = x0 ^ x1
        x0 = x0 + ks[i % 3]
        x1 = x1 + ks[(i + 1) % 3] + jnp.uint32(i)
    return x0, x1


def _unif_prefix(kd, S, e):
    k0 = kd[0]
    k1 = kd[1]
    idx = jnp.arange(S, dtype=jnp.uint32)
    half = (e + jnp.uint32(1)) // jnp.uint32(2)
    c1a = jnp.where(half + idx < e, half + idx, jnp.uint32(0))
    a0, _ = _tf2x32(k0, k1, idx, c1a)
    c0b = jnp.where(idx >= half, idx - half, jnp.uint32(0))
    _, b1 = _tf2x32(k0, k1, c0b, idx)
    bits = jnp.where(idx < half, a0, b1)
    f = jax.lax.bitcast_convert_type(
        (bits >> jnp.uint32(9)) | jnp.uint32(0x3F800000), jnp.float32)
    return jnp.maximum(jnp.float32(0.0), f - jnp.float32(1.0))


# speedup vs baseline: 12.1626x; 1.0061x over previous
import jax
import jax.numpy as jnp
from jax import lax
from jax.experimental import pallas as pl
from jax.experimental.pallas import tpu as pltpu
from jax.experimental.pallas import tpu_sc as plsc

_INTERPRET = False

_N = 6000
_NP = 6016
_NROWS = 6016
_STRIPE = _NROWS // 16
_CHUNK = 128
_CAP = 16384
_RPT = 188
_TRASH = 6000


def _wid():
    return lax.axis_index("s") * 2 + lax.axis_index("c")


def _zero_vmem2d(buf, ncols):
    z = jnp.zeros((16,), jnp.float32)

    def body(i, _):
        for k in range(ncols // 16):
            buf[i, pl.ds(k * 16, 16)] = z
        return 0

    lax.fori_loop(0, buf.shape[0], body, 0)


def _zero_acc_stripe(acc, zbuf, s):
    base = s * _STRIPE
    pltpu.sync_copy(zbuf, acc.at[pl.ds(base, _CHUNK)])
    pltpu.sync_copy(zbuf, acc.at[pl.ds(base + _CHUNK, _CHUNK)])
    rem = _STRIPE - 2 * _CHUNK
    pltpu.sync_copy(zbuf.at[pl.ds(0, rem)], acc.at[pl.ds(base + 2 * _CHUNK, rem)])


_RBLK = 6
_NBLK = 32
_NQ = 93


def _e1_process_block(rowbuf, sumbuf, hitq, colb, srcb, r0, nr, bb, blk, off):
    iota = jnp.arange(16, dtype=jnp.int32)

    def do_row(i, off_i):
        rglob = bb + i
        rsplat = jnp.broadcast_to(rglob, (16,)).astype(jnp.int32)

        def emit(o2, colids, m, pc):
            plsc.store_compressed(colb.at[pl.ds(o2, 16)], colids, mask=m)
            plsc.store_compressed(srcb.at[pl.ds(o2, 16)], rsplat, mask=m)
            return o2 + pc

        hoff = jnp.int32(0)
        for sv in range(6):
            f = sumbuf[i, pl.ds(16 * sv, 16)]
            m = f != 0.0
            pc = plsc.all_reduce_population_count(m)[0]
            plsc.store_compressed(
                hitq.at[pl.ds(hoff, 16)], (16 * sv + iota), mask=m)
            hoff = hoff + pc

        def hit_quad(j, o):
            qid = hitq[pl.ds(j, 16)][0]
            base = qid * 64
            for k in range(4):
                x = rowbuf[i, pl.ds(base + 16 * k, 16)]
                mk_ = x != 0.0
                pck = plsc.all_reduce_population_count(mk_)[0]
                o = emit(o, base + k * 16 + iota, mk_, pck)
            return o

        off_i = lax.fori_loop(0, hoff, hit_quad, off_i)
        for t in range(3):
            v = _NQ * 4 + t
            x = rowbuf[i, pl.ds(v * 16, 16)]
            m = x != 0.0
            pc = plsc.all_reduce_population_count(m)[0]
            off_i = lax.cond(
                pc > 0,
                lambda o3, m=m, v=v, pc=pc: emit(
                    o3, (v * 16 + iota).astype(jnp.int32), m, pc),
                lambda o3: o3, off_i)
        return off_i

    def row_iter(i, off_i):
        rglob = bb + i
        valid = (rglob >= r0 + blk * _RBLK) & (rglob < r0 + nr)
        return lax.cond(valid, lambda o: do_row(i, o), lambda o: o, off_i)

    for i in range(_RBLK):
        off = row_iter(i, off)
    return off


def _e1_body(a_hbm, sum_hbm, src_hbm, col_hbm, cnt_hbm,
             rb0, rb1, sb0, sb1, hitq, colb, srcb, cntv, sem0, sem1):
    w = _wid()
    r0 = w * _RPT
    nr = jnp.where(w == 31, _N - 31 * _RPT, _RPT)

    def bbase(b):
        return jnp.minimum(r0 + b * _RBLK, _N - _RBLK)

    def issue(b, rb, sb, sem):
        pltpu.async_copy(a_hbm.at[pl.ds(bbase(b), _RBLK)], rb, sem)
        pltpu.async_copy(sum_hbm.at[pl.ds(bbase(b), _RBLK)], sb, sem)

    def drain(rb, sb, sem):
        pltpu.make_async_copy(a_hbm.at[pl.ds(0, _RBLK)], rb, sem).wait()
        pltpu.make_async_copy(sum_hbm.at[pl.ds(0, _RBLK)], sb, sem).wait()

    issue(0, rb0, sb0, sem0)

    def pair(p, off):
        b0 = 2 * p
        b1 = 2 * p + 1
        issue(b1, rb1, sb1, sem1)
        drain(rb0, sb0, sem0)
        off = _e1_process_block(rb0, sb0, hitq, colb, srcb, r0, nr,
                                bbase(b0), b0, off)

        def prefetch(_):
            issue(b0 + 2, rb0, sb0, sem0)
            return 0

        lax.cond(p < _NBLK // 2 - 1, prefetch, lambda _: 0, 0)
        drain(rb1, sb1, sem1)
        off = _e1_process_block(rb1, sb1, hitq, colb, srcb, r0, nr,
                                bbase(b1), b1, off)
        return off

    off = lax.fori_loop(0, _NBLK // 2, pair, jnp.int32(0))
    cntv[...] = jnp.broadcast_to(off, (16,)).astype(jnp.int32)
    pltpu.sync_copy(cntv, cnt_hbm.at[w])
    pltpu.sync_copy(colb.at[pl.ds(0, _CAP)], col_hbm.at[w])
    pltpu.sync_copy(srcb.at[pl.ds(0, _CAP)], src_hbm.at[w])


def _sc_extract(a_f32, summary):
    mesh = plsc.VectorSubcoreMesh(core_axis_name="c", subcore_axis_name="s")
    return pl.kernel(
        _e1_body,
        out_type=(jax.ShapeDtypeStruct((32, _CAP), jnp.int32),
                  jax.ShapeDtypeStruct((32, _CAP), jnp.int32),
                  jax.ShapeDtypeStruct((32, 16), jnp.int32)),
        mesh=mesh,
        scratch_types=[
            pltpu.VMEM((_RBLK, _N), jnp.float32),
            pltpu.VMEM((_RBLK, _N), jnp.float32),
            pltpu.VMEM((_RBLK, 96), jnp.float32),
            pltpu.VMEM((_RBLK, 96), jnp.float32),
            pltpu.VMEM((112,), jnp.int32),
            pltpu.VMEM((_CAP + 16,), jnp.int32),
            pltpu.VMEM((_CAP + 16,), jnp.int32),
            pltpu.VMEM((16,), jnp.int32),
            pltpu.SemaphoreType.DMA,
            pltpu.SemaphoreType.DMA,
        ],
        compiler_params=pltpu.CompilerParams(
            use_tc_tiling_on_sc=False, needs_layout_passes=False),
        interpret=_INTERPRET,
    )(a_f32, summary)


def _e2_body(src_hbm, col_hbm, cnt_hbm, t1_hbm, t2_hbm,
             srcO, d0O, d1O, d2O, degO,
             cntall, sv, cv, sv2, d0v, d1v, d2v, tw1, tw2,
             ones0, ones1, ones2, zbuf, acc):
    c = lax.axis_index("c")
    s = lax.axis_index("s")
    w = _wid()
    iota = jnp.arange(16, dtype=jnp.int32)

    pltpu.sync_copy(cnt_hbm, cntall)

    def fill(i, _):
        ones0[i, :] = jnp.where(iota == 0, 1.0, 0.0)
        ones1[i, :] = jnp.where(iota == 1, 1.0, 0.0)
        ones2[i, :] = jnp.where(iota == 2, 1.0, 0.0)
        zbuf[i, :] = jnp.zeros((16,), jnp.float32)
        return 0

    lax.fori_loop(0, _CHUNK, fill, 0)
    _zero_acc_stripe(acc, zbuf, s)
    plsc.subcore_barrier()

    def pf(t, carry):
        ge, ga = carry
        ct = cntall[t, pl.ds(0, 16)][0]
        return (ge + ct, ga + ((ct + 127) // 128) * 128)

    ge, ga = lax.fori_loop(0, w, pf, (jnp.int32(0), jnp.int32(0)))
    cnt = cntall[w, pl.ds(0, 16)][0]
    nch = (cnt + 127) // 128

    def chunk(k, _):
        kb = pl.multiple_of(k * 128, 128)
        pltpu.sync_copy(src_hbm.at[w].at[pl.ds(kb, 128)], sv)
        pltpu.sync_copy(col_hbm.at[w].at[pl.ds(kb, 128)], cv)
        rb = ge + k * 128
        al = pl.multiple_of((rb // 16) * 16, 16)
        sh = rb - al
        pltpu.sync_copy(t1_hbm.at[pl.ds(al, 144)], tw1)
        pltpu.sync_copy(t2_hbm.at[pl.ds(al, 144)], tw2)
        for g in range(8):
            lidx = k * 128 + g * 16 + iota
            vld = lidx < cnt
            colg = cv[pl.ds(g * 16, 16)]
            srcg = sv[pl.ds(g * 16, 16)]
            t1g = tw1[pl.ds(sh + g * 16, 16)]
            t2g = tw2[pl.ds(sh + g * 16, 16)]
            d0v[pl.ds(g * 16, 16)] = jnp.where(
                vld & (t1g > 0), colg, jnp.int32(_TRASH))
            d1v[pl.ds(g * 16, 16)] = jnp.where(
                vld & (t2g > 0), colg, jnp.int32(_TRASH))
            d2v[pl.ds(g * 16, 16)] = jnp.where(vld, colg, jnp.int32(_TRASH))
            sv2[pl.ds(g * 16, 16)] = jnp.where(vld, srcg, jnp.int32(0))
        pltpu.sync_copy(ones0, acc.at[d0v], add=True)
        pltpu.sync_copy(ones1, acc.at[d1v], add=True)
        pltpu.sync_copy(ones2, acc.at[d2v], add=True)
        ob = pl.multiple_of(ga + k * 128, 128)
        pltpu.sync_copy(sv2, srcO.at[pl.ds(ob, 128)])
        pltpu.sync_copy(d0v, d0O.at[pl.ds(ob, 128)])
        pltpu.sync_copy(d1v, d1O.at[pl.ds(ob, 128)])
        pltpu.sync_copy(d2v, d2O.at[pl.ds(ob, 128)])
        return 0

    lax.fori_loop(0, nch, chunk, 0)
    plsc.subcore_barrier()
    base = s * _STRIPE
    pltpu.sync_copy(acc.at[pl.ds(base, _STRIPE)],
                    degO.at[c].at[pl.ds(base, _STRIPE)])


def _sc_rank_deg(srcB, colB, cntB, t1, t2, S):
    mesh = plsc.VectorSubcoreMesh(core_axis_name="c", subcore_axis_name="s")
    return pl.kernel(
        _e2_body,
        out_type=(jax.ShapeDtypeStruct((S,), jnp.int32),
                  jax.ShapeDtypeStruct((S,), jnp.int32),
                  jax.ShapeDtypeStruct((S,), jnp.int32),
                  jax.ShapeDtypeStruct((S,), jnp.int32),
                  jax.ShapeDtypeStruct((2, _NROWS, 16), jnp.float32)),
        mesh=mesh,
        scratch_types=[
            pltpu.VMEM((32, 16), jnp.int32),
            pltpu.VMEM((_CHUNK,), jnp.int32),
            pltpu.VMEM((_CHUNK,), jnp.int32),
            pltpu.VMEM((_CHUNK,), jnp.int32),
            pltpu.VMEM((_CHUNK,), jnp.int32),
            pltpu.VMEM((_CHUNK,), jnp.int32),
            pltpu.VMEM((_CHUNK,), jnp.int32),
            pltpu.VMEM((144,), jnp.float32),
            pltpu.VMEM((144,), jnp.float32),
            pltpu.VMEM((_CHUNK, 16), jnp.float32),
            pltpu.VMEM((_CHUNK, 16), jnp.float32),
            pltpu.VMEM((_CHUNK, 16), jnp.float32),
            pltpu.VMEM((_CHUNK, 16), jnp.float32),
            pltpu.VMEM_SHARED((_NROWS, 16), jnp.float32),
        ],
        compiler_params=pltpu.CompilerParams(
            use_tc_tiling_on_sc=False, needs_layout_passes=False),
        interpret=_INTERPRET,
    )(srcB, colB, cntB, t1, t2)


def _make_agg(F):
    def body(y0_hbm, y1_hbm, y2_hbm, s_hbm, d0_hbm, d1_hbm, d2_hbm, cnt_hbm,
             o0_hbm, o1_hbm, o2_hbm,
             cntall, svA, dv0A, dv1A, dv2A, svB, dv0B, dv1B, dv2B,
             r0A, r1A, r2A, r0B, r1B, r2B, semA, semB,
             acc0, acc1, acc2):
        c = lax.axis_index("c")
        s = lax.axis_index("s")
        w = _wid()
        pltpu.sync_copy(cnt_hbm, cntall)
        _zero_vmem2d(r0A, F)
        _zero_acc_stripe(acc0, r0A, s)
        _zero_acc_stripe(acc1, r0A, s)
        _zero_acc_stripe(acc2, r0A, s)
        plsc.subcore_barrier()

        def pf(t, ga):
            ct = cntall[t, pl.ds(0, 16)][0]
            return ga + ((ct + 127) // 128) * 128

        ga = lax.fori_loop(0, w, pf, jnp.int32(0))
        cnt = cntall[w, pl.ds(0, 16)][0]
        nch = (cnt + 127) // 128

        def load_idx(k, sv, dv0, dv1, dv2):
            base = pl.multiple_of(ga + k * 128, 128)
            pltpu.sync_copy(s_hbm.at[pl.ds(base, _CHUNK)], sv)
            pltpu.sync_copy(d0_hbm.at[pl.ds(base, _CHUNK)], dv0)
            pltpu.sync_copy(d1_hbm.at[pl.ds(base, _CHUNK)], dv1)
            pltpu.sync_copy(d2_hbm.at[pl.ds(base, _CHUNK)], dv2)

        def issue_gather(sv, r0, r1, r2, sem):
            pltpu.async_copy(y0_hbm.at[sv], r0, sem)
            pltpu.async_copy(y1_hbm.at[sv], r1, sem)
            pltpu.async_copy(y2_hbm.at[sv], r2, sem)

        def drain_gather(r0, r1, r2, sem):
            pltpu.make_async_copy(y0_hbm.at[pl.ds(0, _CHUNK)], r0, sem).wait()
            pltpu.make_async_copy(y1_hbm.at[pl.ds(0, _CHUNK)], r1, sem).wait()
            pltpu.make_async_copy(y2_hbm.at[pl.ds(0, _CHUNK)], r2, sem).wait()

        def scat(r0, r1, r2, dv0, dv1, dv2):
            pltpu.sync_copy(r0, acc0.at[dv0], add=True)
            pltpu.sync_copy(r1, acc1.at[dv1], add=True)
            pltpu.sync_copy(r2, acc2.at[dv2], add=True)

        def prologue(_):
            load_idx(0, svA, dv0A, dv1A, dv2A)
            issue_gather(svA, r0A, r1A, r2A, semA)
            return 0

        lax.cond(nch > 0, prologue, lambda _: 0, 0)

        def pair(p, _):
            a = 2 * p
            b = 2 * p + 1

            def do_b(_):
                load_idx(b, svB, dv0B, dv1B, dv2B)
                issue_gather(svB, r0B, r1B, r2B, semB)
                return 0

            lax.cond(b < nch, do_b, lambda _: 0, 0)
            drain_gather(r0A, r1A, r2A, semA)
            scat(r0A, r1A, r2A, dv0A, dv1A, dv2A)

            def do_a2(_):
                load_idx(a + 2, svA, dv0A, dv1A, dv2A)
                issue_gather(svA, r0A, r1A, r2A, semA)
                return 0

            lax.cond(a + 2 < nch, do_a2, lambda _: 0, 0)

            def fin_b(_):
                drain_gather(r0B, r1B, r2B, semB)
                scat(r0B, r1B, r2B, dv0B, dv1B, dv2B)
                return 0

            lax.cond(b < nch, fin_b, lambda _: 0, 0)
            return 0

        lax.fori_loop(0, (nch + 1) // 2, pair, 0)
        plsc.subcore_barrier()
        base = s * _STRIPE
        pltpu.sync_copy(acc0.at[pl.ds(base, _STRIPE)],
                        o0_hbm.at[c].at[pl.ds(base, _STRIPE)])
        pltpu.sync_copy(acc1.at[pl.ds(base, _STRIPE)],
                        o1_hbm.at[c].at[pl.ds(base, _STRIPE)])
        pltpu.sync_copy(acc2.at[pl.ds(base, _STRIPE)],
                        o2_hbm.at[c].at[pl.ds(base, _STRIPE)])

    return body


def _sc_agg(F, y0, y1, y2, src, d0, d1, d2, cntB):
    mesh = plsc.VectorSubcoreMesh(core_axis_name="c", subcore_axis_name="s")
    out = jax.ShapeDtypeStruct((2, _NROWS, F), jnp.float32)
    return pl.kernel(
        _make_agg(F),
        out_type=(out, out, out),
        mesh=mesh,
        scratch_types=(
            [pltpu.VMEM((32, 16), jnp.int32)]
            + [pltpu.VMEM((_CHUNK,), jnp.int32)] * 8
            + [pltpu.VMEM((_CHUNK, F), jnp.float32)] * 6
            + [pltpu.SemaphoreType.DMA, pltpu.SemaphoreType.DMA]
            + [pltpu.VMEM_SHARED((_NROWS, F), jnp.float32)] * 3
        ),
        compiler_params=pltpu.CompilerParams(
            use_tc_tiling_on_sc=False, needs_layout_passes=False),
        interpret=_INTERPRET,
    )(y0, y1, y2, src, d0, d1, d2, cntB)


def _ksum_body(a_ref, m_ref, o_ref):
    o_ref[...] = jnp.dot(a_ref[...], m_ref[...],
                         preferred_element_type=jnp.float32)


def _lr(x):
    return jnp.where(x >= 0, x, 0.01 * x)


def _k1_body(xwin_ref, degp_ref, y0_ref, y1_ref, y2_ref, dinv_ref):
    xw = xwin_ref[...]
    degp = degp_ref[...]
    deg = degp[0, :_N, 0:3] + degp[1, :_N, 0:3] + 1.0
    dinv = jax.lax.rsqrt(deg)
    dinv_ref[...] = dinv
    drep = jnp.repeat(dinv, 64, axis=1)
    y = xw * drep
    y0_ref[...] = y[:, 0:64]
    y1_ref[...] = y[:, 64:128]
    y2_ref[...] = y[:, 128:192]


def _k2_body(s0_ref, s1_ref, s2_ref, xw_ref, dinv_ref, b_ref, w2_ref,
             xw2_ref, y0_ref, y1_ref, y2_ref):
    dinv = dinv_ref[...]
    drep = jnp.repeat(dinv, 64, axis=1)
    scat = jnp.concatenate(
        [s0_ref[0, :_N, :] + s0_ref[1, :_N, :],
         s1_ref[0, :_N, :] + s1_ref[1, :_N, :],
         s2_ref[0, :_N, :] + s2_ref[1, :_N, :]], axis=1)
    h1 = _lr(drep * (scat + drep * xw_ref[...]) + b_ref[...])
    xw2 = jnp.dot(h1, w2_ref[...], preferred_element_type=jnp.float32)
    xw2_ref[...] = xw2
    drep2 = jnp.repeat(dinv, 32, axis=1)
    y2 = xw2 * drep2
    y0_ref[...] = y2[:, 0:32]
    y1_ref[...] = y2[:, 32:64]
    y2_ref[...] = y2[:, 64:96]


def _k3_body(s0_ref, s1_ref, s2_ref, xw_ref, dinv_ref, b_ref, mu_ref,
             decw_ref, decb_ref, om_ref, h2_ref, c0_ref, pr_ref):
    dinv = dinv_ref[...]
    drep = jnp.repeat(dinv, 32, axis=1)
    scat = jnp.concatenate(
        [s0_ref[0, :_N, :] + s0_ref[1, :_N, :],
         s1_ref[0, :_N, :] + s1_ref[1, :_N, :],
         s2_ref[0, :_N, :] + s2_ref[1, :_N, :]], axis=1)
    h2 = _lr(drep * (scat + drep * xw_ref[...]) + b_ref[...])
    h2_ref[...] = h2
    mu = mu_ref[...]
    w0 = om_ref[0, 0]
    w1 = om_ref[0, 1]
    gex = h2[:, 64:96]
    c0 = (w0 * gex + w1 * mu) / (w0 + w1)
    c0_ref[...] = c0
    pr_ref[...] = _lr(jnp.dot(c0, decw_ref[...],
                              preferred_element_type=jnp.float32) + decb_ref[...])


def _kheads_body(prot_ref, pew_ref, peb_ref, mnw_ref, mnb_ref, vrw_ref,
                 vrb_ref, mu_ref, lv_ref):
    enc = _lr(jnp.dot(prot_ref[...], pew_ref[...],
                      preferred_element_type=jnp.float32) + peb_ref[...])
    mu_ref[...] = _lr(jnp.dot(enc, mnw_ref[...],
                              preferred_element_type=jnp.float32) + mnb_ref[...])
    lv_ref[...] = _lr(jnp.dot(enc, vrw_ref[...],
                              preferred_element_type=jnp.float32) + vrb_ref[...])


def _k4_body(a_ref, b_ref, o_ref):
    o_ref[...] = jax.lax.dot_general(
        a_ref[...], b_ref[...], (((1,), (1,)), ((), ())),
        preferred_element_type=jnp.float32)


def kernel(gene_matrix, protein_matrix, adjacency_matrix, W1, b1, W2, b2,
           pe_W, pe_b, mn_W, mn_b, vr_W, vr_b, dec_W, dec_b, omega):
    N, G = gene_matrix.shape
    P = protein_matrix.shape[1]
    L = W2.shape[1]
    F1, F2 = 2 * L, L
    S = 32 * N

    r_ = jnp.arange(N)
    q_ = r_ // 64
    ones_blk = ((q_[:, None] == jnp.arange(96)[None, :]) &
                (q_[:, None] < _NQ)).astype(jnp.float32)
    summary = pl.pallas_call(
        _ksum_body,
        grid=(6,),
        in_specs=[pl.BlockSpec((N // 6, N), lambda i: (i, 0)),
                  pl.BlockSpec((N, 96), lambda i: (0, 0))],
        out_specs=pl.BlockSpec((N // 6, 96), lambda i: (i, 0)),
        out_shape=jax.ShapeDtypeStruct((N, 96), jnp.float32),
        interpret=_INTERPRET,
    )(adjacency_matrix, ones_blk)

    f1 = (jax.random.uniform(jax.random.fold_in(jax.random.key(42), 3),
                             (G,)) >= 0.3).astype(jnp.float32)
    f2 = (jax.random.uniform(jax.random.fold_in(jax.random.key(42), 4),
                             (G,)) >= 0.2).astype(jnp.float32)
    W1cat = jnp.concatenate([W1 * f1[:, None], W1 * f2[:, None], W1], axis=1)
    xw1 = pl.pallas_call(
        _ksum_body,
        grid=(6,),
        in_specs=[pl.BlockSpec((N // 6, G), lambda i: (i, 0)),
                  pl.BlockSpec((G, 3 * F1), lambda i: (0, 0))],
        out_specs=pl.BlockSpec((N // 6, 3 * F1), lambda i: (i, 0)),
        out_shape=jax.ShapeDtypeStruct((N, 3 * F1), jnp.float32),
        interpret=_INTERPRET,
    )(gene_matrix, W1cat)

    mu, logvar = pl.pallas_call(
        _kheads_body,
        out_shape=(jax.ShapeDtypeStruct((N, L), jnp.float32),
                   jax.ShapeDtypeStruct((N, L), jnp.float32)),
        interpret=_INTERPRET,
    )(protein_matrix, pe_W, pe_b[None, :], mn_W, mn_b[None, :],
      vr_W, vr_b[None, :])

    srcB, colB, cntB = _sc_extract(adjacency_matrix, summary)

    mk = jax.random.key(42)
    if jax.config.jax_threefry_partitionable:
        u1 = jax.random.uniform(jax.random.fold_in(mk, 1), (S,))
        u2 = jax.random.uniform(jax.random.fold_in(mk, 2), (S,))
    else:
        E = jnp.sum(cntB[:, 0]).astype(jnp.uint32)
        u1 = _unif_prefix(jax.random.key_data(jax.random.fold_in(mk, 1)), S, E)
        u2 = _unif_prefix(jax.random.key_data(jax.random.fold_in(mk, 2)), S, E)
    t1 = (u1 >= 0.4).astype(jnp.float32)
    t2 = (u2 >= 0.5).astype(jnp.float32)
    src, d0, d1, d2, deg_parts = _sc_rank_deg(srcB, colB, cntB, t1, t2, S)

    z = jnp.zeros((2 * L, L), jnp.float32)
    W2bd = jnp.block([[W2, z, z], [z, W2, z], [z, z, W2]])
    b1t = jnp.tile(b1, 3)[None, :]
    b2t = jnp.tile(b2, 3)[None, :]

    ya, yb, yc, dinv = pl.pallas_call(
        _k1_body,
        out_shape=(jax.ShapeDtypeStruct((N, F1), jnp.float32),
                   jax.ShapeDtypeStruct((N, F1), jnp.float32),
                   jax.ShapeDtypeStruct((N, F1), jnp.float32),
                   jax.ShapeDtypeStruct((N, 3), jnp.float32)),
        interpret=_INTERPRET,
    )(xw1, deg_parts)

    sa, sb, sc_ = _sc_agg(F1, ya, yb, yc, src, d0, d1, d2, cntB)

    xw2, ya2, yb2, yc2 = pl.pallas_call(
        _k2_body,
        out_shape=(jax.ShapeDtypeStruct((N, 3 * F2), jnp.float32),
                   jax.ShapeDtypeStruct((N, F2), jnp.float32),
                   jax.ShapeDtypeStruct((N, F2), jnp.float32),
                   jax.ShapeDtypeStruct((N, F2), jnp.float32)),
        interpret=_INTERPRET,
    )(sa, sb, sc_, xw1, dinv, b1t, W2bd)

    sa2, sb2, sc2 = _sc_agg(F2, ya2, yb2, yc2, src, d0, d1, d2, cntB)

    h2, c0, pex_recons = pl.pallas_call(
        _k3_body,
        out_shape=(jax.ShapeDtypeStruct((N, 3 * F2), jnp.float32),
                   jax.ShapeDtypeStruct((N, L), jnp.float32),
                   jax.ShapeDtypeStruct((N, P), jnp.float32)),
        interpret=_INTERPRET,
    )(sa2, sb2, sc2, xw2, dinv, b2t, mu, dec_W, dec_b[None, :],
      omega[None, :])

    BM = 600
    adj_recon = pl.pallas_call(
        _k4_body,
        grid=(N // BM,),
        in_specs=[pl.BlockSpec((BM, L), lambda i: (i, 0)),
                  pl.BlockSpec((N, L), lambda i: (0, 0))],
        out_specs=pl.BlockSpec((BM, N), lambda i: (i, 0)),
        out_shape=jax.ShapeDtypeStruct((N, N), jnp.float32),
        interpret=_INTERPRET,
    )(c0, c0)

    z1, z2, gex_z = h2[:, :L], h2[:, L:2 * L], h2[:, 2 * L:]
    return (adj_recon, pex_recons, z1, z2, gex_z, mu, mu, logvar, c0, c0, omega)


def _tf2x32(k0, k1, x0, x1):
    def rotl(x, d):
        return (x << jnp.uint32(d)) | (x >> jnp.uint32(32 - d))
    ks = (k0, k1, k0 ^ k1 ^ jnp.uint32(0x1BD11BDA))
    x0 = x0 + ks[0]
    x1 = x1 + ks[1]
    rotations = ((13, 15, 26, 6), (17, 29, 16, 24))
    for i in range(1, 6):
        for r in rotations[(i - 1) % 2]:
            x0 = x0 + x1
            x1 = rotl(x1, r)
            x1 = x0 ^ x1
        x0 = x0 + ks[i % 3]
        x1 = x1 + ks[(i + 1) % 3] + jnp.uint32(i)
    return x0, x1


def _unif_prefix(kd, S, e):
    k0 = kd[0]
    k1 = kd[1]
    idx = jnp.arange(S, dtype=jnp.uint32)
    half = (e + jnp.uint32(1)) // jnp.uint32(2)
    c1a = jnp.where(half + idx < e, half + idx, jnp.uint32(0))
    a0, _ = _tf2x32(k0, k1, idx, c1a)
    c0b = jnp.where(idx >= half, idx - half, jnp.uint32(0))
    _, b1 = _tf2x32(k0, k1, c0b, idx)
    bits = jnp.where(idx < half, a0, b1)
    f = jax.lax.bitcast_convert_type(
        (bits >> jnp.uint32(9)) | jnp.uint32(0x3F800000), jnp.float32)
    return jnp.maximum(jnp.float32(0.0), f - jnp.float32(1.0))


# speedup vs baseline: 13.0362x; 1.0718x over previous
import jax
import jax.numpy as jnp
from jax import lax
from jax.experimental import pallas as pl
from jax.experimental.pallas import tpu as pltpu
from jax.experimental.pallas import tpu_sc as plsc

_INTERPRET = False

_N = 6000
_NP = 6016
_NROWS = 6016
_STRIPE = _NROWS // 16
_CHUNK = 128
_CAP = 16384
_RPT = 188
_TRASH = 6000


def _wid():
    return lax.axis_index("s") * 2 + lax.axis_index("c")


def _zero_vmem2d(buf, ncols):
    z = jnp.zeros((16,), jnp.float32)

    def body(i, _):
        for k in range(ncols // 16):
            buf[i, pl.ds(k * 16, 16)] = z
        return 0

    lax.fori_loop(0, buf.shape[0], body, 0)


def _zero_acc_stripe(acc, zbuf, s):
    base = s * _STRIPE
    pltpu.sync_copy(zbuf, acc.at[pl.ds(base, _CHUNK)])
    pltpu.sync_copy(zbuf, acc.at[pl.ds(base + _CHUNK, _CHUNK)])
    rem = _STRIPE - 2 * _CHUNK
    pltpu.sync_copy(zbuf.at[pl.ds(0, rem)], acc.at[pl.ds(base + 2 * _CHUNK, rem)])


_RBLK = 6
_NBLK = 32
_NQ = 93


def _e1_process_block(rowbuf, sumbuf, hitq, colb, srcb, r0, nr, bb, blk, off):
    iota = jnp.arange(16, dtype=jnp.int32)

    def do_row(i, off_i):
        rglob = bb + i
        rsplat = jnp.broadcast_to(rglob, (16,)).astype(jnp.int32)

        def emit(o2, colids, m, pc):
            plsc.store_compressed(colb.at[pl.ds(o2, 16)], colids, mask=m)
            plsc.store_compressed(srcb.at[pl.ds(o2, 16)], rsplat, mask=m)
            return o2 + pc

        hoff = jnp.int32(0)
        for sv in range(6):
            f = sumbuf[i, pl.ds(16 * sv, 16)]
            m = f != 0.0
            pc = plsc.all_reduce_population_count(m)[0]
            plsc.store_compressed(
                hitq.at[pl.ds(hoff, 16)], (16 * sv + iota), mask=m)
            hoff = hoff + pc

        def hit_quad(j, o):
            qid = hitq[pl.ds(j, 16)][0]
            base = qid * 64
            xs = [rowbuf[i, pl.ds(base + 16 * k, 16)] for k in range(4)]
            ms = [x != 0.0 for x in xs]
            pcs = [plsc.all_reduce_population_count(m)[0] for m in ms]
            for k in range(4):
                o = emit(o, base + k * 16 + iota, ms[k], pcs[k])
            return o

        off_i = lax.fori_loop(0, hoff, hit_quad, off_i)
        for t in range(3):
            v = _NQ * 4 + t
            x = rowbuf[i, pl.ds(v * 16, 16)]
            m = x != 0.0
            pc = plsc.all_reduce_population_count(m)[0]
            off_i = lax.cond(
                pc > 0,
                lambda o3, m=m, v=v, pc=pc: emit(
                    o3, (v * 16 + iota).astype(jnp.int32), m, pc),
                lambda o3: o3, off_i)
        return off_i

    def row_iter(i, off_i):
        rglob = bb + i
        valid = (rglob >= r0 + blk * _RBLK) & (rglob < r0 + nr)
        return lax.cond(valid, lambda o: do_row(i, o), lambda o: o, off_i)

    for i in range(_RBLK):
        off = row_iter(i, off)
    return off


def _e1_body(a_hbm, sum_hbm, src_hbm, col_hbm, cnt_hbm,
             rb0, rb1, sb0, sb1, hitq, colb, srcb, cntv, sem0, sem1):
    w = _wid()
    r0 = w * _RPT
    nr = jnp.where(w == 31, _N - 31 * _RPT, _RPT)

    def bbase(b):
        return jnp.minimum(r0 + b * _RBLK, _N - _RBLK)

    def issue(b, rb, sb, sem):
        pltpu.async_copy(a_hbm.at[pl.ds(bbase(b), _RBLK)], rb, sem)
        pltpu.async_copy(sum_hbm.at[pl.ds(bbase(b), _RBLK)], sb, sem)

    def drain(rb, sb, sem):
        pltpu.make_async_copy(a_hbm.at[pl.ds(0, _RBLK)], rb, sem).wait()
        pltpu.make_async_copy(sum_hbm.at[pl.ds(0, _RBLK)], sb, sem).wait()

    issue(0, rb0, sb0, sem0)

    def pair(p, off):
        b0 = 2 * p
        b1 = 2 * p + 1
        issue(b1, rb1, sb1, sem1)
        drain(rb0, sb0, sem0)
        off = _e1_process_block(rb0, sb0, hitq, colb, srcb, r0, nr,
                                bbase(b0), b0, off)

        def prefetch(_):
            issue(b0 + 2, rb0, sb0, sem0)
            return 0

        lax.cond(p < _NBLK // 2 - 1, prefetch, lambda _: 0, 0)
        drain(rb1, sb1, sem1)
        off = _e1_process_block(rb1, sb1, hitq, colb, srcb, r0, nr,
                                bbase(b1), b1, off)
        return off

    off = lax.fori_loop(0, _NBLK // 2, pair, jnp.int32(0))
    cntv[...] = jnp.broadcast_to(off, (16,)).astype(jnp.int32)
    pltpu.sync_copy(cntv, cnt_hbm.at[w])
    pltpu.sync_copy(colb.at[pl.ds(0, _CAP)], col_hbm.at[w])
    pltpu.sync_copy(srcb.at[pl.ds(0, _CAP)], src_hbm.at[w])


def _sc_extract(a_f32, summary):
    mesh = plsc.VectorSubcoreMesh(core_axis_name="c", subcore_axis_name="s")
    return pl.kernel(
        _e1_body,
        out_type=(jax.ShapeDtypeStruct((32, _CAP), jnp.int32),
                  jax.ShapeDtypeStruct((32, _CAP), jnp.int32),
                  jax.ShapeDtypeStruct((32, 16), jnp.int32)),
        mesh=mesh,
        scratch_types=[
            pltpu.VMEM((_RBLK, _N), jnp.float32),
            pltpu.VMEM((_RBLK, _N), jnp.float32),
            pltpu.VMEM((_RBLK, 96), jnp.float32),
            pltpu.VMEM((_RBLK, 96), jnp.float32),
            pltpu.VMEM((112,), jnp.int32),
            pltpu.VMEM((_CAP + 16,), jnp.int32),
            pltpu.VMEM((_CAP + 16,), jnp.int32),
            pltpu.VMEM((16,), jnp.int32),
            pltpu.SemaphoreType.DMA,
            pltpu.SemaphoreType.DMA,
        ],
        compiler_params=pltpu.CompilerParams(
            use_tc_tiling_on_sc=False, needs_layout_passes=False),
        interpret=_INTERPRET,
    )(a_f32, summary)


def _e2_body(src_hbm, col_hbm, cnt_hbm, t1_hbm, t2_hbm,
             srcO, d0O, d1O, d2O, degO,
             cntall, sv, cv, sv2, d0v, d1v, d2v, tw1, tw2,
             ones0, ones1, ones2, zbuf, acc):
    c = lax.axis_index("c")
    s = lax.axis_index("s")
    w = _wid()
    iota = jnp.arange(16, dtype=jnp.int32)

    pltpu.sync_copy(cnt_hbm, cntall)

    def fill(i, _):
        ones0[i, :] = jnp.where(iota == 0, 1.0, 0.0)
        ones1[i, :] = jnp.where(iota == 1, 1.0, 0.0)
        ones2[i, :] = jnp.where(iota == 2, 1.0, 0.0)
        zbuf[i, :] = jnp.zeros((16,), jnp.float32)
        return 0

    lax.fori_loop(0, _CHUNK, fill, 0)
    _zero_acc_stripe(acc, zbuf, s)
    plsc.subcore_barrier()

    def pf(t, carry):
        ge, ga = carry
        ct = cntall[t, pl.ds(0, 16)][0]
        return (ge + ct, ga + ((ct + 127) // 128) * 128)

    ge, ga = lax.fori_loop(0, w, pf, (jnp.int32(0), jnp.int32(0)))
    cnt = cntall[w, pl.ds(0, 16)][0]
    nch = (cnt + 127) // 128

    def chunk(k, _):
        kb = pl.multiple_of(k * 128, 128)
        pltpu.sync_copy(src_hbm.at[w].at[pl.ds(kb, 128)], sv)
        pltpu.sync_copy(col_hbm.at[w].at[pl.ds(kb, 128)], cv)
        rb = ge + k * 128
        al = pl.multiple_of((rb // 16) * 16, 16)
        sh = rb - al
        pltpu.sync_copy(t1_hbm.at[pl.ds(al, 144)], tw1)
        pltpu.sync_copy(t2_hbm.at[pl.ds(al, 144)], tw2)
        for g in range(8):
            lidx = k * 128 + g * 16 + iota
            vld = lidx < cnt
            colg = cv[pl.ds(g * 16, 16)]
            srcg = sv[pl.ds(g * 16, 16)]
            t1g = tw1[pl.ds(sh + g * 16, 16)]
            t2g = tw2[pl.ds(sh + g * 16, 16)]
            d0v[pl.ds(g * 16, 16)] = jnp.where(
                vld & (t1g > 0), colg, jnp.int32(_TRASH))
            d1v[pl.ds(g * 16, 16)] = jnp.where(
                vld & (t2g > 0), colg, jnp.int32(_TRASH))
            d2v[pl.ds(g * 16, 16)] = jnp.where(vld, colg, jnp.int32(_TRASH))
            sv2[pl.ds(g * 16, 16)] = jnp.where(vld, srcg, jnp.int32(0))
        pltpu.sync_copy(ones0, acc.at[d0v], add=True)
        pltpu.sync_copy(ones1, acc.at[d1v], add=True)
        pltpu.sync_copy(ones2, acc.at[d2v], add=True)
        ob = pl.multiple_of(ga + k * 128, 128)
        pltpu.sync_copy(sv2, srcO.at[pl.ds(ob, 128)])
        pltpu.sync_copy(d0v, d0O.at[pl.ds(ob, 128)])
        pltpu.sync_copy(d1v, d1O.at[pl.ds(ob, 128)])
        pltpu.sync_copy(d2v, d2O.at[pl.ds(ob, 128)])
        return 0

    lax.fori_loop(0, nch, chunk, 0)
    plsc.subcore_barrier()
    base = s * _STRIPE
    pltpu.sync_copy(acc.at[pl.ds(base, _STRIPE)],
                    degO.at[c].at[pl.ds(base, _STRIPE)])


def _sc_rank_deg(srcB, colB, cntB, t1, t2, S):
    mesh = plsc.VectorSubcoreMesh(core_axis_name="c", subcore_axis_name="s")
    return pl.kernel(
        _e2_body,
        out_type=(jax.ShapeDtypeStruct((S,), jnp.int32),
                  jax.ShapeDtypeStruct((S,), jnp.int32),
                  jax.ShapeDtypeStruct((S,), jnp.int32),
                  jax.ShapeDtypeStruct((S,), jnp.int32),
                  jax.ShapeDtypeStruct((2, _NROWS, 16), jnp.float32)),
        mesh=mesh,
        scratch_types=[
            pltpu.VMEM((32, 16), jnp.int32),
            pltpu.VMEM((_CHUNK,), jnp.int32),
            pltpu.VMEM((_CHUNK,), jnp.int32),
            pltpu.VMEM((_CHUNK,), jnp.int32),
            pltpu.VMEM((_CHUNK,), jnp.int32),
            pltpu.VMEM((_CHUNK,), jnp.int32),
            pltpu.VMEM((_CHUNK,), jnp.int32),
            pltpu.VMEM((144,), jnp.float32),
            pltpu.VMEM((144,), jnp.float32),
            pltpu.VMEM((_CHUNK, 16), jnp.float32),
            pltpu.VMEM((_CHUNK, 16), jnp.float32),
            pltpu.VMEM((_CHUNK, 16), jnp.float32),
            pltpu.VMEM((_CHUNK, 16), jnp.float32),
            pltpu.VMEM_SHARED((_NROWS, 16), jnp.float32),
        ],
        compiler_params=pltpu.CompilerParams(
            use_tc_tiling_on_sc=False, needs_layout_passes=False),
        interpret=_INTERPRET,
    )(srcB, colB, cntB, t1, t2)


def _make_agg(F):
    def body(y0_hbm, y1_hbm, y2_hbm, s_hbm, d0_hbm, d1_hbm, d2_hbm, cnt_hbm,
             o0_hbm, o1_hbm, o2_hbm,
             cntall, svA, dv0A, dv1A, dv2A, svB, dv0B, dv1B, dv2B,
             r0A, r1A, r2A, r0B, r1B, r2B, semA, semB,
             acc0, acc1, acc2):
        c = lax.axis_index("c")
        s = lax.axis_index("s")
        w = _wid()
        pltpu.sync_copy(cnt_hbm, cntall)
        _zero_vmem2d(r0A, F)
        _zero_acc_stripe(acc0, r0A, s)
        _zero_acc_stripe(acc1, r0A, s)
        _zero_acc_stripe(acc2, r0A, s)
        plsc.subcore_barrier()

        def pf(t, ga):
            ct = cntall[t, pl.ds(0, 16)][0]
            return ga + ((ct + 127) // 128) * 128

        ga = lax.fori_loop(0, w, pf, jnp.int32(0))
        cnt = cntall[w, pl.ds(0, 16)][0]
        nch = (cnt + 127) // 128

        def load_idx(k, sv, dv0, dv1, dv2):
            base = pl.multiple_of(ga + k * 128, 128)
            pltpu.sync_copy(s_hbm.at[pl.ds(base, _CHUNK)], sv)
            pltpu.sync_copy(d0_hbm.at[pl.ds(base, _CHUNK)], dv0)
            pltpu.sync_copy(d1_hbm.at[pl.ds(base, _CHUNK)], dv1)
            pltpu.sync_copy(d2_hbm.at[pl.ds(base, _CHUNK)], dv2)

        def issue_gather(sv, r0, r1, r2, sem):
            pltpu.async_copy(y0_hbm.at[sv], r0, sem)
            pltpu.async_copy(y1_hbm.at[sv], r1, sem)
            pltpu.async_copy(y2_hbm.at[sv], r2, sem)

        def drain_gather(r0, r1, r2, sem):
            pltpu.make_async_copy(y0_hbm.at[pl.ds(0, _CHUNK)], r0, sem).wait()
            pltpu.make_async_copy(y1_hbm.at[pl.ds(0, _CHUNK)], r1, sem).wait()
            pltpu.make_async_copy(y2_hbm.at[pl.ds(0, _CHUNK)], r2, sem).wait()

        def scat(r0, r1, r2, dv0, dv1, dv2):
            pltpu.sync_copy(r0, acc0.at[dv0], add=True)
            pltpu.sync_copy(r1, acc1.at[dv1], add=True)
            pltpu.sync_copy(r2, acc2.at[dv2], add=True)

        def prologue(_):
            load_idx(0, svA, dv0A, dv1A, dv2A)
            issue_gather(svA, r0A, r1A, r2A, semA)
            return 0

        lax.cond(nch > 0, prologue, lambda _: 0, 0)

        def pair(p, _):
            a = 2 * p
            b = 2 * p + 1

            def do_b(_):
                load_idx(b, svB, dv0B, dv1B, dv2B)
                issue_gather(svB, r0B, r1B, r2B, semB)
                return 0

            lax.cond(b < nch, do_b, lambda _: 0, 0)
            drain_gather(r0A, r1A, r2A, semA)
            scat(r0A, r1A, r2A, dv0A, dv1A, dv2A)

            def do_a2(_):
                load_idx(a + 2, svA, dv0A, dv1A, dv2A)
                issue_gather(svA, r0A, r1A, r2A, semA)
                return 0

            lax.cond(a + 2 < nch, do_a2, lambda _: 0, 0)

            def fin_b(_):
                drain_gather(r0B, r1B, r2B, semB)
                scat(r0B, r1B, r2B, dv0B, dv1B, dv2B)
                return 0

            lax.cond(b < nch, fin_b, lambda _: 0, 0)
            return 0

        lax.fori_loop(0, (nch + 1) // 2, pair, 0)
        plsc.subcore_barrier()
        base = s * _STRIPE
        pltpu.sync_copy(acc0.at[pl.ds(base, _STRIPE)],
                        o0_hbm.at[c].at[pl.ds(base, _STRIPE)])
        pltpu.sync_copy(acc1.at[pl.ds(base, _STRIPE)],
                        o1_hbm.at[c].at[pl.ds(base, _STRIPE)])
        pltpu.sync_copy(acc2.at[pl.ds(base, _STRIPE)],
                        o2_hbm.at[c].at[pl.ds(base, _STRIPE)])

    return body


def _sc_agg(F, y0, y1, y2, src, d0, d1, d2, cntB):
    mesh = plsc.VectorSubcoreMesh(core_axis_name="c", subcore_axis_name="s")
    out = jax.ShapeDtypeStruct((2, _NROWS, F), jnp.float32)
    return pl.kernel(
        _make_agg(F),
        out_type=(out, out, out),
        mesh=mesh,
        scratch_types=(
            [pltpu.VMEM((32, 16), jnp.int32)]
            + [pltpu.VMEM((_CHUNK,), jnp.int32)] * 8
            + [pltpu.VMEM((_CHUNK, F), jnp.float32)] * 6
            + [pltpu.SemaphoreType.DMA, pltpu.SemaphoreType.DMA]
            + [pltpu.VMEM_SHARED((_NROWS, F), jnp.float32)] * 3
        ),
        compiler_params=pltpu.CompilerParams(
            use_tc_tiling_on_sc=False, needs_layout_passes=False),
        interpret=_INTERPRET,
    )(y0, y1, y2, src, d0, d1, d2, cntB)


def _ksum_body(a_ref, m_ref, o_ref):
    o_ref[...] = jnp.dot(a_ref[...], m_ref[...],
                         preferred_element_type=jnp.float32)


def _lr(x):
    return jnp.where(x >= 0, x, 0.01 * x)


def _k1_body(xwin_ref, degp_ref, y0_ref, y1_ref, y2_ref, dinv_ref):
    xw = xwin_ref[...]
    degp = degp_ref[...]
    deg = degp[0, :_N, 0:3] + degp[1, :_N, 0:3] + 1.0
    dinv = jax.lax.rsqrt(deg)
    dinv_ref[...] = dinv
    drep = jnp.repeat(dinv, 64, axis=1)
    y = xw * drep
    y0_ref[...] = y[:, 0:64]
    y1_ref[...] = y[:, 64:128]
    y2_ref[...] = y[:, 128:192]


def _k2_body(s0_ref, s1_ref, s2_ref, xw_ref, dinv_ref, b_ref, w2_ref,
             xw2_ref, y0_ref, y1_ref, y2_ref):
    dinv = dinv_ref[...]
    drep = jnp.repeat(dinv, 64, axis=1)
    scat = jnp.concatenate(
        [s0_ref[0, :_N, :] + s0_ref[1, :_N, :],
         s1_ref[0, :_N, :] + s1_ref[1, :_N, :],
         s2_ref[0, :_N, :] + s2_ref[1, :_N, :]], axis=1)
    h1 = _lr(drep * (scat + drep * xw_ref[...]) + b_ref[...])
    xw2 = jnp.dot(h1, w2_ref[...], preferred_element_type=jnp.float32)
    xw2_ref[...] = xw2
    drep2 = jnp.repeat(dinv, 32, axis=1)
    y2 = xw2 * drep2
    y0_ref[...] = y2[:, 0:32]
    y1_ref[...] = y2[:, 32:64]
    y2_ref[...] = y2[:, 64:96]


def _k3_body(s0_ref, s1_ref, s2_ref, xw_ref, dinv_ref, b_ref, mu_ref,
             decw_ref, decb_ref, om_ref, h2_ref, c0_ref, pr_ref):
    dinv = dinv_ref[...]
    drep = jnp.repeat(dinv, 32, axis=1)
    scat = jnp.concatenate(
        [s0_ref[0, :_N, :] + s0_ref[1, :_N, :],
         s1_ref[0, :_N, :] + s1_ref[1, :_N, :],
         s2_ref[0, :_N, :] + s2_ref[1, :_N, :]], axis=1)
    h2 = _lr(drep * (scat + drep * xw_ref[...]) + b_ref[...])
    h2_ref[...] = h2
    mu = mu_ref[...]
    w0 = om_ref[0, 0]
    w1 = om_ref[0, 1]
    gex = h2[:, 64:96]
    c0 = (w0 * gex + w1 * mu) / (w0 + w1)
    c0_ref[...] = c0
    pr_ref[...] = _lr(jnp.dot(c0, decw_ref[...],
                              preferred_element_type=jnp.float32) + decb_ref[...])


def _kheads_body(prot_ref, pew_ref, peb_ref, mnw_ref, mnb_ref, vrw_ref,
                 vrb_ref, mu_ref, lv_ref):
    enc = _lr(jnp.dot(prot_ref[...], pew_ref[...],
                      preferred_element_type=jnp.float32) + peb_ref[...])
    mu_ref[...] = _lr(jnp.dot(enc, mnw_ref[...],
                              preferred_element_type=jnp.float32) + mnb_ref[...])
    lv_ref[...] = _lr(jnp.dot(enc, vrw_ref[...],
                              preferred_element_type=jnp.float32) + vrb_ref[...])


def _k4_body(a_ref, b_ref, o_ref):
    o_ref[...] = jax.lax.dot_general(
        a_ref[...], b_ref[...], (((1,), (1,)), ((), ())),
        preferred_element_type=jnp.float32)


def kernel(gene_matrix, protein_matrix, adjacency_matrix, W1, b1, W2, b2,
           pe_W, pe_b, mn_W, mn_b, vr_W, vr_b, dec_W, dec_b, omega):
    N, G = gene_matrix.shape
    P = protein_matrix.shape[1]
    L = W2.shape[1]
    F1, F2 = 2 * L, L
    S = 32 * N

    r_ = jnp.arange(N)
    q_ = r_ // 64
    ones_blk = ((q_[:, None] == jnp.arange(96)[None, :]) &
                (q_[:, None] < _NQ)).astype(jnp.float32)
    summary = pl.pallas_call(
        _ksum_body,
        grid=(6,),
        in_specs=[pl.BlockSpec((N // 6, N), lambda i: (i, 0)),
                  pl.BlockSpec((N, 96), lambda i: (0, 0))],
        out_specs=pl.BlockSpec((N // 6, 96), lambda i: (i, 0)),
        out_shape=jax.ShapeDtypeStruct((N, 96), jnp.float32),
        interpret=_INTERPRET,
    )(adjacency_matrix, ones_blk)

    f1 = (jax.random.uniform(jax.random.fold_in(jax.random.key(42), 3),
                             (G,)) >= 0.3).astype(jnp.float32)
    f2 = (jax.random.uniform(jax.random.fold_in(jax.random.key(42), 4),
                             (G,)) >= 0.2).astype(jnp.float32)
    W1cat = jnp.concatenate([W1 * f1[:, None], W1 * f2[:, None], W1], axis=1)
    xw1 = pl.pallas_call(
        _ksum_body,
        grid=(6,),
        in_specs=[pl.BlockSpec((N // 6, G), lambda i: (i, 0)),
                  pl.BlockSpec((G, 3 * F1), lambda i: (0, 0))],
        out_specs=pl.BlockSpec((N // 6, 3 * F1), lambda i: (i, 0)),
        out_shape=jax.ShapeDtypeStruct((N, 3 * F1), jnp.float32),
        interpret=_INTERPRET,
    )(gene_matrix, W1cat)

    mu, logvar = pl.pallas_call(
        _kheads_body,
        out_shape=(jax.ShapeDtypeStruct((N, L), jnp.float32),
                   jax.ShapeDtypeStruct((N, L), jnp.float32)),
        interpret=_INTERPRET,
    )(protein_matrix, pe_W, pe_b[None, :], mn_W, mn_b[None, :],
      vr_W, vr_b[None, :])

    srcB, colB, cntB = _sc_extract(adjacency_matrix, summary)

    mk = jax.random.key(42)
    if jax.config.jax_threefry_partitionable:
        u1 = jax.random.uniform(jax.random.fold_in(mk, 1), (S,))
        u2 = jax.random.uniform(jax.random.fold_in(mk, 2), (S,))
    else:
        E = jnp.sum(cntB[:, 0]).astype(jnp.uint32)
        u1 = _unif_prefix(jax.random.key_data(jax.random.fold_in(mk, 1)), S, E)
        u2 = _unif_prefix(jax.random.key_data(jax.random.fold_in(mk, 2)), S, E)
    t1 = (u1 >= 0.4).astype(jnp.float32)
    t2 = (u2 >= 0.5).astype(jnp.float32)
    src, d0, d1, d2, deg_parts = _sc_rank_deg(srcB, colB, cntB, t1, t2, S)

    z = jnp.zeros((2 * L, L), jnp.float32)
    W2bd = jnp.block([[W2, z, z], [z, W2, z], [z, z, W2]])
    b1t = jnp.tile(b1, 3)[None, :]
    b2t = jnp.tile(b2, 3)[None, :]

    ya, yb, yc, dinv = pl.pallas_call(
        _k1_body,
        out_shape=(jax.ShapeDtypeStruct((N, F1), jnp.float32),
                   jax.ShapeDtypeStruct((N, F1), jnp.float32),
                   jax.ShapeDtypeStruct((N, F1), jnp.float32),
                   jax.ShapeDtypeStruct((N, 3), jnp.float32)),
        interpret=_INTERPRET,
    )(xw1, deg_parts)

    sa, sb, sc_ = _sc_agg(F1, ya, yb, yc, src, d0, d1, d2, cntB)

    xw2, ya2, yb2, yc2 = pl.pallas_call(
        _k2_body,
        out_shape=(jax.ShapeDtypeStruct((N, 3 * F2), jnp.float32),
                   jax.ShapeDtypeStruct((N, F2), jnp.float32),
                   jax.ShapeDtypeStruct((N, F2), jnp.float32),
                   jax.ShapeDtypeStruct((N, F2), jnp.float32)),
        interpret=_INTERPRET,
    )(sa, sb, sc_, xw1, dinv, b1t, W2bd)

    sa2, sb2, sc2 = _sc_agg(F2, ya2, yb2, yc2, src, d0, d1, d2, cntB)

    h2, c0, pex_recons = pl.pallas_call(
        _k3_body,
        out_shape=(jax.ShapeDtypeStruct((N, 3 * F2), jnp.float32),
                   jax.ShapeDtypeStruct((N, L), jnp.float32),
                   jax.ShapeDtypeStruct((N, P), jnp.float32)),
        interpret=_INTERPRET,
    )(sa2, sb2, sc2, xw2, dinv, b2t, mu, dec_W, dec_b[None, :],
      omega[None, :])

    BM = 600
    adj_recon = pl.pallas_call(
        _k4_body,
        grid=(N // BM,),
        in_specs=[pl.BlockSpec((BM, L), lambda i: (i, 0)),
                  pl.BlockSpec((N, L), lambda i: (0, 0))],
        out_specs=pl.BlockSpec((BM, N), lambda i: (i, 0)),
        out_shape=jax.ShapeDtypeStruct((N, N), jnp.float32),
        interpret=_INTERPRET,
    )(c0, c0)

    z1, z2, gex_z = h2[:, :L], h2[:, L:2 * L], h2[:, 2 * L:]
    return (adj_recon, pex_recons, z1, z2, gex_z, mu, mu, logvar, c0, c0, omega)


def _tf2x32(k0, k1, x0, x1):
    def rotl(x, d):
        return (x << jnp.uint32(d)) | (x >> jnp.uint32(32 - d))
    ks = (k0, k1, k0 ^ k1 ^ jnp.uint32(0x1BD11BDA))
    x0 = x0 + ks[0]
    x1 = x1 + ks[1]
    rotations = ((13, 15, 26, 6), (17, 29, 16, 24))
    for i in range(1, 6):
        for r in rotations[(i - 1) % 2]:
            x0 = x0 + x1
            x1 = rotl(x1, r)
            x1 = x0 ^ x1
        x0 = x0 + ks[i % 3]
        x1 = x1 + ks[(i + 1) % 3] + jnp.uint32(i)
    return x0, x1


def _unif_prefix(kd, S, e):
    k0 = kd[0]
    k1 = kd[1]
    idx = jnp.arange(S, dtype=jnp.uint32)
    half = (e + jnp.uint32(1)) // jnp.uint32(2)
    c1a = jnp.where(half + idx < e, half + idx, jnp.uint32(0))
    a0, _ = _tf2x32(k0, k1, idx, c1a)
    c0b = jnp.where(idx >= half, idx - half, jnp.uint32(0))
    _, b1 = _tf2x32(k0, k1, c0b, idx)
    bits = jnp.where(idx < half, a0, b1)
    f = jax.lax.bitcast_convert_type(
        (bits >> jnp.uint32(9)) | jnp.uint32(0x3F800000), jnp.float32)
    return jnp.maximum(jnp.float32(0.0), f - jnp.float32(1.0))


# speedup vs baseline: 13.2468x; 1.0162x over previous
import jax
import jax.numpy as jnp
from jax import lax
from jax.experimental import pallas as pl
from jax.experimental.pallas import tpu as pltpu
from jax.experimental.pallas import tpu_sc as plsc

_INTERPRET = False

_N = 6000
_NP = 6016
_NROWS = 6016
_STRIPE = _NROWS // 16
_CHUNK = 128
_CAP = 16384
_RPT = 188
_TRASH = 6000


def _wid():
    return lax.axis_index("s") * 2 + lax.axis_index("c")


def _zero_vmem2d(buf, ncols):
    z = jnp.zeros((16,), jnp.float32)

    def body(i, _):
        for k in range(ncols // 16):
            buf[i, pl.ds(k * 16, 16)] = z
        return 0

    lax.fori_loop(0, buf.shape[0], body, 0)


def _zero_acc_stripe(acc, zbuf, s):
    base = s * _STRIPE
    pltpu.sync_copy(zbuf, acc.at[pl.ds(base, _CHUNK)])
    pltpu.sync_copy(zbuf, acc.at[pl.ds(base + _CHUNK, _CHUNK)])
    rem = _STRIPE - 2 * _CHUNK
    pltpu.sync_copy(zbuf.at[pl.ds(0, rem)], acc.at[pl.ds(base + 2 * _CHUNK, rem)])


_RBLK = 6
_NBLK = 32
_NQ = 93


def _e1_process_block(rowbuf, sumbuf, hitq, colb, srcb, r0, nr, bb, blk, off):
    iota = jnp.arange(16, dtype=jnp.int32)

    def do_row(i, off_i):
        rglob = bb + i
        rsplat = jnp.broadcast_to(rglob, (16,)).astype(jnp.int32)

        def emit(o2, colids, m, pc):
            plsc.store_compressed(colb.at[pl.ds(o2, 16)], colids, mask=m)
            plsc.store_compressed(srcb.at[pl.ds(o2, 16)], rsplat, mask=m)
            return o2 + pc

        hoff = jnp.int32(0)
        for sv in range(6):
            f = sumbuf[i, pl.ds(16 * sv, 16)]
            m = f != 0.0
            pc = plsc.all_reduce_population_count(m)[0]
            plsc.store_compressed(
                hitq.at[pl.ds(hoff, 16)], (16 * sv + iota), mask=m)
            hoff = hoff + pc

        def hit_quad(j, o):
            qid = hitq[pl.ds(j, 16)][0]
            base = qid * 64
            xs = [rowbuf[i, pl.ds(base + 16 * k, 16)] for k in range(4)]
            ms = [x != 0.0 for x in xs]
            pcs = [plsc.all_reduce_population_count(m)[0] for m in ms]
            for k in range(4):
                o = emit(o, base + k * 16 + iota, ms[k], pcs[k])
            return o

        off_i = lax.fori_loop(0, hoff, hit_quad, off_i)
        for t in range(3):
            v = _NQ * 4 + t
            x = rowbuf[i, pl.ds(v * 16, 16)]
            m = x != 0.0
            pc = plsc.all_reduce_population_count(m)[0]
            off_i = lax.cond(
                pc > 0,
                lambda o3, m=m, v=v, pc=pc: emit(
                    o3, (v * 16 + iota).astype(jnp.int32), m, pc),
                lambda o3: o3, off_i)
        return off_i

    def row_iter(i, off_i):
        rglob = bb + i
        valid = (rglob >= r0 + blk * _RBLK) & (rglob < r0 + nr)
        return lax.cond(valid, lambda o: do_row(i, o), lambda o: o, off_i)

    for i in range(_RBLK):
        off = row_iter(i, off)
    return off


def _e1_body(a_hbm, sum_hbm, src_hbm, col_hbm, cnt_hbm,
             rb0, rb1, sb0, sb1, hitq, colb, srcb, cntv, sem0, sem1):
    w = _wid()
    r0 = w * _RPT
    nr = jnp.where(w == 31, _N - 31 * _RPT, _RPT)

    def bbase(b):
        return jnp.minimum(r0 + b * _RBLK, _N - _RBLK)

    def issue(b, rb, sb, sem):
        pltpu.async_copy(a_hbm.at[pl.ds(bbase(b), _RBLK)], rb, sem)
        pltpu.async_copy(sum_hbm.at[pl.ds(bbase(b), _RBLK)], sb, sem)

    def drain(rb, sb, sem):
        pltpu.make_async_copy(a_hbm.at[pl.ds(0, _RBLK)], rb, sem).wait()
        pltpu.make_async_copy(sum_hbm.at[pl.ds(0, _RBLK)], sb, sem).wait()

    issue(0, rb0, sb0, sem0)

    def pair(p, off):
        b0 = 2 * p
        b1 = 2 * p + 1
        issue(b1, rb1, sb1, sem1)
        drain(rb0, sb0, sem0)
        off = _e1_process_block(rb0, sb0, hitq, colb, srcb, r0, nr,
                                bbase(b0), b0, off)

        def prefetch(_):
            issue(b0 + 2, rb0, sb0, sem0)
            return 0

        lax.cond(p < _NBLK // 2 - 1, prefetch, lambda _: 0, 0)
        drain(rb1, sb1, sem1)
        off = _e1_process_block(rb1, sb1, hitq, colb, srcb, r0, nr,
                                bbase(b1), b1, off)
        return off

    off = lax.fori_loop(0, _NBLK // 2, pair, jnp.int32(0))
    cntv[...] = jnp.broadcast_to(off, (16,)).astype(jnp.int32)
    pltpu.sync_copy(cntv, cnt_hbm.at[w])
    pltpu.sync_copy(colb.at[pl.ds(0, _CAP)], col_hbm.at[w])
    pltpu.sync_copy(srcb.at[pl.ds(0, _CAP)], src_hbm.at[w])


def _sc_extract(a_f32, summary):
    mesh = plsc.VectorSubcoreMesh(core_axis_name="c", subcore_axis_name="s")
    return pl.kernel(
        _e1_body,
        out_type=(jax.ShapeDtypeStruct((32, _CAP), jnp.int32),
                  jax.ShapeDtypeStruct((32, _CAP), jnp.int32),
                  jax.ShapeDtypeStruct((32, 16), jnp.int32)),
        mesh=mesh,
        scratch_types=[
            pltpu.VMEM((_RBLK, _N), jnp.float32),
            pltpu.VMEM((_RBLK, _N), jnp.float32),
            pltpu.VMEM((_RBLK, 96), jnp.float32),
            pltpu.VMEM((_RBLK, 96), jnp.float32),
            pltpu.VMEM((112,), jnp.int32),
            pltpu.VMEM((_CAP + 16,), jnp.int32),
            pltpu.VMEM((_CAP + 16,), jnp.int32),
            pltpu.VMEM((16,), jnp.int32),
            pltpu.SemaphoreType.DMA,
            pltpu.SemaphoreType.DMA,
        ],
        compiler_params=pltpu.CompilerParams(
            use_tc_tiling_on_sc=False, needs_layout_passes=False),
        interpret=_INTERPRET,
    )(a_f32, summary)


def _e2_body(src_hbm, col_hbm, cnt_hbm, t1_hbm, t2_hbm,
             srcO, d0O, d1O, d2O, degO,
             cntall, sv, cv, sv2, d0v, d1v, d2v, tw1, tw2,
             ones0, ones1, ones2, zbuf, lsem, acc):
    c = lax.axis_index("c")
    s = lax.axis_index("s")
    w = _wid()
    iota = jnp.arange(16, dtype=jnp.int32)

    pltpu.sync_copy(cnt_hbm, cntall)

    def fill(i, _):
        ones0[i, :] = jnp.where(iota == 0, 1.0, 0.0)
        ones1[i, :] = jnp.where(iota == 1, 1.0, 0.0)
        ones2[i, :] = jnp.where(iota == 2, 1.0, 0.0)
        zbuf[i, :] = jnp.zeros((16,), jnp.float32)
        return 0

    lax.fori_loop(0, _CHUNK, fill, 0)
    _zero_acc_stripe(acc, zbuf, s)
    plsc.subcore_barrier()

    def pf(t, carry):
        ge, ga = carry
        ct = cntall[t, pl.ds(0, 16)][0]
        return (ge + ct, ga + ((ct + 127) // 128) * 128)

    ge, ga = lax.fori_loop(0, w, pf, (jnp.int32(0), jnp.int32(0)))
    cnt = cntall[w, pl.ds(0, 16)][0]
    nch = (cnt + 127) // 128

    def chunk(k, _):
        kb = pl.multiple_of(k * 128, 128)
        rb = ge + k * 128
        al = pl.multiple_of((rb // 16) * 16, 16)
        sh = rb - al
        pltpu.async_copy(src_hbm.at[w].at[pl.ds(kb, 128)], sv, lsem)
        pltpu.async_copy(col_hbm.at[w].at[pl.ds(kb, 128)], cv, lsem)
        pltpu.async_copy(t1_hbm.at[pl.ds(al, 144)], tw1, lsem)
        pltpu.async_copy(t2_hbm.at[pl.ds(al, 144)], tw2, lsem)
        pltpu.make_async_copy(src_hbm.at[w].at[pl.ds(kb, 128)], sv, lsem).wait()
        pltpu.make_async_copy(col_hbm.at[w].at[pl.ds(kb, 128)], cv, lsem).wait()
        pltpu.make_async_copy(t1_hbm.at[pl.ds(al, 144)], tw1, lsem).wait()
        pltpu.make_async_copy(t2_hbm.at[pl.ds(al, 144)], tw2, lsem).wait()
        for g in range(8):
            lidx = k * 128 + g * 16 + iota
            vld = lidx < cnt
            colg = cv[pl.ds(g * 16, 16)]
            srcg = sv[pl.ds(g * 16, 16)]
            t1g = tw1[pl.ds(sh + g * 16, 16)]
            t2g = tw2[pl.ds(sh + g * 16, 16)]
            d0v[pl.ds(g * 16, 16)] = jnp.where(
                vld & (t1g > 0), colg, jnp.int32(_TRASH))
            d1v[pl.ds(g * 16, 16)] = jnp.where(
                vld & (t2g > 0), colg, jnp.int32(_TRASH))
            d2v[pl.ds(g * 16, 16)] = jnp.where(vld, colg, jnp.int32(_TRASH))
            sv2[pl.ds(g * 16, 16)] = jnp.where(vld, srcg, jnp.int32(0))
        pltpu.sync_copy(ones0, acc.at[d0v], add=True)
        pltpu.sync_copy(ones1, acc.at[d1v], add=True)
        pltpu.sync_copy(ones2, acc.at[d2v], add=True)
        ob = pl.multiple_of(ga + k * 128, 128)
        pltpu.sync_copy(sv2, srcO.at[pl.ds(ob, 128)])
        pltpu.sync_copy(d0v, d0O.at[pl.ds(ob, 128)])
        pltpu.sync_copy(d1v, d1O.at[pl.ds(ob, 128)])
        pltpu.sync_copy(d2v, d2O.at[pl.ds(ob, 128)])
        return 0

    lax.fori_loop(0, nch, chunk, 0)
    plsc.subcore_barrier()
    base = s * _STRIPE
    pltpu.sync_copy(acc.at[pl.ds(base, _STRIPE)],
                    degO.at[c].at[pl.ds(base, _STRIPE)])


def _sc_rank_deg(srcB, colB, cntB, t1, t2, S):
    mesh = plsc.VectorSubcoreMesh(core_axis_name="c", subcore_axis_name="s")
    return pl.kernel(
        _e2_body,
        out_type=(jax.ShapeDtypeStruct((S,), jnp.int32),
                  jax.ShapeDtypeStruct((S,), jnp.int32),
                  jax.ShapeDtypeStruct((S,), jnp.int32),
                  jax.ShapeDtypeStruct((S,), jnp.int32),
                  jax.ShapeDtypeStruct((2, _NROWS, 16), jnp.float32)),
        mesh=mesh,
        scratch_types=[
            pltpu.VMEM((32, 16), jnp.int32),
            pltpu.VMEM((_CHUNK,), jnp.int32),
            pltpu.VMEM((_CHUNK,), jnp.int32),
            pltpu.VMEM((_CHUNK,), jnp.int32),
            pltpu.VMEM((_CHUNK,), jnp.int32),
            pltpu.VMEM((_CHUNK,), jnp.int32),
            pltpu.VMEM((_CHUNK,), jnp.int32),
            pltpu.VMEM((144,), jnp.float32),
            pltpu.VMEM((144,), jnp.float32),
            pltpu.VMEM((_CHUNK, 16), jnp.float32),
            pltpu.VMEM((_CHUNK, 16), jnp.float32),
            pltpu.VMEM((_CHUNK, 16), jnp.float32),
            pltpu.VMEM((_CHUNK, 16), jnp.float32),
            pltpu.SemaphoreType.DMA,
            pltpu.VMEM_SHARED((_NROWS, 16), jnp.float32),
        ],
        compiler_params=pltpu.CompilerParams(
            use_tc_tiling_on_sc=False, needs_layout_passes=False),
        interpret=_INTERPRET,
    )(srcB, colB, cntB, t1, t2)


def _make_agg(F):
    def body(y0_hbm, y1_hbm, y2_hbm, s_hbm, d0_hbm, d1_hbm, d2_hbm, cnt_hbm,
             o0_hbm, o1_hbm, o2_hbm,
             cntall, svA, dv0A, dv1A, dv2A, svB, dv0B, dv1B, dv2B,
             r0A, r1A, r2A, r0B, r1B, r2B, semA, semB,
             acc0, acc1, acc2):
        c = lax.axis_index("c")
        s = lax.axis_index("s")
        w = _wid()
        pltpu.sync_copy(cnt_hbm, cntall)
        _zero_vmem2d(r0A, F)
        _zero_acc_stripe(acc0, r0A, s)
        _zero_acc_stripe(acc1, r0A, s)
        _zero_acc_stripe(acc2, r0A, s)
        plsc.subcore_barrier()

        def pf(t, ga):
            ct = cntall[t, pl.ds(0, 16)][0]
            return ga + ((ct + 127) // 128) * 128

        ga = lax.fori_loop(0, w, pf, jnp.int32(0))
        cnt = cntall[w, pl.ds(0, 16)][0]
        nch = (cnt + 127) // 128

        def load_idx(k, sv, dv0, dv1, dv2):
            base = pl.multiple_of(ga + k * 128, 128)
            pltpu.sync_copy(s_hbm.at[pl.ds(base, _CHUNK)], sv)
            pltpu.sync_copy(d0_hbm.at[pl.ds(base, _CHUNK)], dv0)
            pltpu.sync_copy(d1_hbm.at[pl.ds(base, _CHUNK)], dv1)
            pltpu.sync_copy(d2_hbm.at[pl.ds(base, _CHUNK)], dv2)

        def issue_gather(sv, r0, r1, r2, sem):
            pltpu.async_copy(y0_hbm.at[sv], r0, sem)
            pltpu.async_copy(y1_hbm.at[sv], r1, sem)
            pltpu.async_copy(y2_hbm.at[sv], r2, sem)

        def drain_gather(r0, r1, r2, sem):
            pltpu.make_async_copy(y0_hbm.at[pl.ds(0, _CHUNK)], r0, sem).wait()
            pltpu.make_async_copy(y1_hbm.at[pl.ds(0, _CHUNK)], r1, sem).wait()
            pltpu.make_async_copy(y2_hbm.at[pl.ds(0, _CHUNK)], r2, sem).wait()

        def scat(r0, r1, r2, dv0, dv1, dv2):
            pltpu.sync_copy(r0, acc0.at[dv0], add=True)
            pltpu.sync_copy(r1, acc1.at[dv1], add=True)
            pltpu.sync_copy(r2, acc2.at[dv2], add=True)

        def prologue(_):
            load_idx(0, svA, dv0A, dv1A, dv2A)
            issue_gather(svA, r0A, r1A, r2A, semA)
            return 0

        lax.cond(nch > 0, prologue, lambda _: 0, 0)

        def pair(p, _):
            a = 2 * p
            b = 2 * p + 1

            def do_b(_):
                load_idx(b, svB, dv0B, dv1B, dv2B)
                issue_gather(svB, r0B, r1B, r2B, semB)
                return 0

            lax.cond(b < nch, do_b, lambda _: 0, 0)
            drain_gather(r0A, r1A, r2A, semA)
            scat(r0A, r1A, r2A, dv0A, dv1A, dv2A)

            def do_a2(_):
                load_idx(a + 2, svA, dv0A, dv1A, dv2A)
                issue_gather(svA, r0A, r1A, r2A, semA)
                return 0

            lax.cond(a + 2 < nch, do_a2, lambda _: 0, 0)

            def fin_b(_):
                drain_gather(r0B, r1B, r2B, semB)
                scat(r0B, r1B, r2B, dv0B, dv1B, dv2B)
                return 0

            lax.cond(b < nch, fin_b, lambda _: 0, 0)
            return 0

        lax.fori_loop(0, (nch + 1) // 2, pair, 0)
        plsc.subcore_barrier()
        base = s * _STRIPE
        pltpu.sync_copy(acc0.at[pl.ds(base, _STRIPE)],
                        o0_hbm.at[c].at[pl.ds(base, _STRIPE)])
        pltpu.sync_copy(acc1.at[pl.ds(base, _STRIPE)],
                        o1_hbm.at[c].at[pl.ds(base, _STRIPE)])
        pltpu.sync_copy(acc2.at[pl.ds(base, _STRIPE)],
                        o2_hbm.at[c].at[pl.ds(base, _STRIPE)])

    return body


def _sc_agg(F, y0, y1, y2, src, d0, d1, d2, cntB):
    mesh = plsc.VectorSubcoreMesh(core_axis_name="c", subcore_axis_name="s")
    out = jax.ShapeDtypeStruct((2, _NROWS, F), jnp.float32)
    return pl.kernel(
        _make_agg(F),
        out_type=(out, out, out),
        mesh=mesh,
        scratch_types=(
            [pltpu.VMEM((32, 16), jnp.int32)]
            + [pltpu.VMEM((_CHUNK,), jnp.int32)] * 8
            + [pltpu.VMEM((_CHUNK, F), jnp.float32)] * 6
            + [pltpu.SemaphoreType.DMA, pltpu.SemaphoreType.DMA]
            + [pltpu.VMEM_SHARED((_NROWS, F), jnp.float32)] * 3
        ),
        compiler_params=pltpu.CompilerParams(
            use_tc_tiling_on_sc=False, needs_layout_passes=False),
        interpret=_INTERPRET,
    )(y0, y1, y2, src, d0, d1, d2, cntB)


def _ksum_body(a_ref, m_ref, o_ref):
    o_ref[...] = jnp.dot(a_ref[...], m_ref[...],
                         preferred_element_type=jnp.float32)


def _lr(x):
    return jnp.where(x >= 0, x, 0.01 * x)


def _k1_body(xwin_ref, degp_ref, y0_ref, y1_ref, y2_ref, dinv_ref):
    xw = xwin_ref[...]
    degp = degp_ref[...]
    deg = degp[0, :_N, 0:3] + degp[1, :_N, 0:3] + 1.0
    dinv = jax.lax.rsqrt(deg)
    dinv_ref[...] = dinv
    drep = jnp.repeat(dinv, 64, axis=1)
    y = xw * drep
    y0_ref[...] = y[:, 0:64]
    y1_ref[...] = y[:, 64:128]
    y2_ref[...] = y[:, 128:192]


def _k2_body(s0_ref, s1_ref, s2_ref, xw_ref, dinv_ref, b_ref, w2_ref,
             xw2_ref, y0_ref, y1_ref, y2_ref):
    dinv = dinv_ref[...]
    drep = jnp.repeat(dinv, 64, axis=1)
    scat = jnp.concatenate(
        [s0_ref[0, :_N, :] + s0_ref[1, :_N, :],
         s1_ref[0, :_N, :] + s1_ref[1, :_N, :],
         s2_ref[0, :_N, :] + s2_ref[1, :_N, :]], axis=1)
    h1 = _lr(drep * (scat + drep * xw_ref[...]) + b_ref[...])
    xw2 = jnp.dot(h1, w2_ref[...], preferred_element_type=jnp.float32)
    xw2_ref[...] = xw2
    drep2 = jnp.repeat(dinv, 32, axis=1)
    y2 = xw2 * drep2
    y0_ref[...] = y2[:, 0:32]
    y1_ref[...] = y2[:, 32:64]
    y2_ref[...] = y2[:, 64:96]


def _k3_body(s0_ref, s1_ref, s2_ref, xw_ref, dinv_ref, b_ref, mu_ref,
             decw_ref, decb_ref, om_ref, h2_ref, c0_ref, pr_ref):
    dinv = dinv_ref[...]
    drep = jnp.repeat(dinv, 32, axis=1)
    scat = jnp.concatenate(
        [s0_ref[0, :_N, :] + s0_ref[1, :_N, :],
         s1_ref[0, :_N, :] + s1_ref[1, :_N, :],
         s2_ref[0, :_N, :] + s2_ref[1, :_N, :]], axis=1)
    h2 = _lr(drep * (scat + drep * xw_ref[...]) + b_ref[...])
    h2_ref[...] = h2
    mu = mu_ref[...]
    w0 = om_ref[0, 0]
    w1 = om_ref[0, 1]
    gex = h2[:, 64:96]
    c0 = (w0 * gex + w1 * mu) / (w0 + w1)
    c0_ref[...] = c0
    pr_ref[...] = _lr(jnp.dot(c0, decw_ref[...],
                              preferred_element_type=jnp.float32) + decb_ref[...])


def _kheads_body(prot_ref, pew_ref, peb_ref, mnw_ref, mnb_ref, vrw_ref,
                 vrb_ref, mu_ref, lv_ref):
    enc = _lr(jnp.dot(prot_ref[...], pew_ref[...],
                      preferred_element_type=jnp.float32) + peb_ref[...])
    mu_ref[...] = _lr(jnp.dot(enc, mnw_ref[...],
                              preferred_element_type=jnp.float32) + mnb_ref[...])
    lv_ref[...] = _lr(jnp.dot(enc, vrw_ref[...],
                              preferred_element_type=jnp.float32) + vrb_ref[...])


def _k4_body(a_ref, b_ref, o_ref):
    o_ref[...] = jax.lax.dot_general(
        a_ref[...], b_ref[...], (((1,), (1,)), ((), ())),
        preferred_element_type=jnp.float32)


def kernel(gene_matrix, protein_matrix, adjacency_matrix, W1, b1, W2, b2,
           pe_W, pe_b, mn_W, mn_b, vr_W, vr_b, dec_W, dec_b, omega):
    N, G = gene_matrix.shape
    P = protein_matrix.shape[1]
    L = W2.shape[1]
    F1, F2 = 2 * L, L
    S = 32 * N

    r_ = jnp.arange(N)
    q_ = r_ // 64
    ones_blk = ((q_[:, None] == jnp.arange(96)[None, :]) &
                (q_[:, None] < _NQ)).astype(jnp.float32)
    summary = pl.pallas_call(
        _ksum_body,
        grid=(6,),
        in_specs=[pl.BlockSpec((N // 6, N), lambda i: (i, 0)),
                  pl.BlockSpec((N, 96), lambda i: (0, 0))],
        out_specs=pl.BlockSpec((N // 6, 96), lambda i: (i, 0)),
        out_shape=jax.ShapeDtypeStruct((N, 96), jnp.float32),
        interpret=_INTERPRET,
    )(adjacency_matrix, ones_blk)

    f1 = (jax.random.uniform(jax.random.fold_in(jax.random.key(42), 3),
                             (G,)) >= 0.3).astype(jnp.float32)
    f2 = (jax.random.uniform(jax.random.fold_in(jax.random.key(42), 4),
                             (G,)) >= 0.2).astype(jnp.float32)
    W1cat = jnp.concatenate([W1 * f1[:, None], W1 * f2[:, None], W1], axis=1)
    xw1 = pl.pallas_call(
        _ksum_body,
        grid=(6,),
        in_specs=[pl.BlockSpec((N // 6, G), lambda i: (i, 0)),
                  pl.BlockSpec((G, 3 * F1), lambda i: (0, 0))],
        out_specs=pl.BlockSpec((N // 6, 3 * F1), lambda i: (i, 0)),
        out_shape=jax.ShapeDtypeStruct((N, 3 * F1), jnp.float32),
        interpret=_INTERPRET,
    )(gene_matrix, W1cat)

    mu, logvar = pl.pallas_call(
        _kheads_body,
        out_shape=(jax.ShapeDtypeStruct((N, L), jnp.float32),
                   jax.ShapeDtypeStruct((N, L), jnp.float32)),
        interpret=_INTERPRET,
    )(protein_matrix, pe_W, pe_b[None, :], mn_W, mn_b[None, :],
      vr_W, vr_b[None, :])

    srcB, colB, cntB = _sc_extract(adjacency_matrix, summary)

    mk = jax.random.key(42)
    if jax.config.jax_threefry_partitionable:
        u1 = jax.random.uniform(jax.random.fold_in(mk, 1), (S,))
        u2 = jax.random.uniform(jax.random.fold_in(mk, 2), (S,))
    else:
        E = jnp.sum(cntB[:, 0]).astype(jnp.uint32)
        u1 = _unif_prefix(jax.random.key_data(jax.random.fold_in(mk, 1)), S, E)
        u2 = _unif_prefix(jax.random.key_data(jax.random.fold_in(mk, 2)), S, E)
    t1 = (u1 >= 0.4).astype(jnp.float32)
    t2 = (u2 >= 0.5).astype(jnp.float32)
    src, d0, d1, d2, deg_parts = _sc_rank_deg(srcB, colB, cntB, t1, t2, S)

    z = jnp.zeros((2 * L, L), jnp.float32)
    W2bd = jnp.block([[W2, z, z], [z, W2, z], [z, z, W2]])
    b1t = jnp.tile(b1, 3)[None, :]
    b2t = jnp.tile(b2, 3)[None, :]

    ya, yb, yc, dinv = pl.pallas_call(
        _k1_body,
        out_shape=(jax.ShapeDtypeStruct((N, F1), jnp.float32),
                   jax.ShapeDtypeStruct((N, F1), jnp.float32),
                   jax.ShapeDtypeStruct((N, F1), jnp.float32),
                   jax.ShapeDtypeStruct((N, 3), jnp.float32)),
        interpret=_INTERPRET,
    )(xw1, deg_parts)

    sa, sb, sc_ = _sc_agg(F1, ya, yb, yc, src, d0, d1, d2, cntB)

    xw2, ya2, yb2, yc2 = pl.pallas_call(
        _k2_body,
        out_shape=(jax.ShapeDtypeStruct((N, 3 * F2), jnp.float32),
                   jax.ShapeDtypeStruct((N, F2), jnp.float32),
                   jax.ShapeDtypeStruct((N, F2), jnp.float32),
                   jax.ShapeDtypeStruct((N, F2), jnp.float32)),
        interpret=_INTERPRET,
    )(sa, sb, sc_, xw1, dinv, b1t, W2bd)

    sa2, sb2, sc2 = _sc_agg(F2, ya2, yb2, yc2, src, d0, d1, d2, cntB)

    h2, c0, pex_recons = pl.pallas_call(
        _k3_body,
        out_shape=(jax.ShapeDtypeStruct((N, 3 * F2), jnp.float32),
                   jax.ShapeDtypeStruct((N, L), jnp.float32),
                   jax.ShapeDtypeStruct((N, P), jnp.float32)),
        interpret=_INTERPRET,
    )(sa2, sb2, sc2, xw2, dinv, b2t, mu, dec_W, dec_b[None, :],
      omega[None, :])

    BM = 600
    adj_recon = pl.pallas_call(
        _k4_body,
        grid=(N // BM,),
        in_specs=[pl.BlockSpec((BM, L), lambda i: (i, 0)),
                  pl.BlockSpec((N, L), lambda i: (0, 0))],
        out_specs=pl.BlockSpec((BM, N), lambda i: (i, 0)),
        out_shape=jax.ShapeDtypeStruct((N, N), jnp.float32),
        interpret=_INTERPRET,
    )(c0, c0)

    z1, z2, gex_z = h2[:, :L], h2[:, L:2 * L], h2[:, 2 * L:]
    return (adj_recon, pex_recons, z1, z2, gex_z, mu, mu, logvar, c0, c0, omega)


def _tf2x32(k0, k1, x0, x1):
    def rotl(x, d):
        return (x << jnp.uint32(d)) | (x >> jnp.uint32(32 - d))
    ks = (k0, k1, k0 ^ k1 ^ jnp.uint32(0x1BD11BDA))
    x0 = x0 + ks[0]
    x1 = x1 + ks[1]
    rotations = ((13, 15, 26, 6), (17, 29, 16, 24))
    for i in range(1, 6):
        for r in rotations[(i - 1) % 2]:
            x0 = x0 + x1
            x1 = rotl(x1, r)
            x1 = x0 ^ x1
        x0 = x0 + ks[i % 3]
        x1 = x1 + ks[(i + 1) % 3] + jnp.uint32(i)
    return x0, x1


def _unif_prefix(kd, S, e):
    k0 = kd[0]
    k1 = kd[1]
    idx = jnp.arange(S, dtype=jnp.uint32)
    half = (e + jnp.uint32(1)) // jnp.uint32(2)
    c1a = jnp.where(half + idx < e, half + idx, jnp.uint32(0))
    a0, _ = _tf2x32(k0, k1, idx, c1a)
    c0b = jnp.where(idx >= half, idx - half, jnp.uint32(0))
    _, b1 = _tf2x32(k0, k1, c0b, idx)
    bits = jnp.where(idx < half, a0, b1)
    f = jax.lax.bitcast_convert_type(
        (bits >> jnp.uint32(9)) | jnp.uint32(0x3F800000), jnp.float32)
    return jnp.maximum(jnp.float32(0.0), f - jnp.float32(1.0))
